# baseline (speedup 1.0000x reference)
import functools
import numpy as np
import jax
import jax.numpy as jnp
from jax.experimental import pallas as pl
from jax.experimental.pallas import tpu as pltpu

_CUTOFF = 5.0
_NRAD = 6
_NSPH = 7
_NB = 2
_NDO = 3

_BE = 1280
_BT = 3200
_BN = 2000

_F32 = jnp.float32


def _swish(x):
    return x * jax.nn.sigmoid(x)


def _envelope(d):
    a = -28.0
    b = 48.0
    c = -21.0
    d2 = d * d
    d4 = d2 * d2
    d5 = d4 * d
    env = 1.0 / d + a * d5 + b * d5 * d + c * d5 * d2
    return jnp.where(d < 1.0, env, 0.0)


def _rbf_from_d(d):
    k = jax.lax.broadcasted_iota(jnp.int32, (1, _NRAD), 1).astype(_F32)
    freq = (k + 1.0) * np.float32(np.pi)
    return _envelope(d) * jnp.sin(freq * d)


def _iota4(et):
    return jax.lax.broadcasted_iota(jnp.int32, (1, 4), 1)


def _dot(a, b):
    return jnp.dot(a, b, preferred_element_type=_F32)


def _node_body(na, w, b, h_out):
    h_out[...] = _dot(na[...], w[...]) + b[...]


def _node_embed(node_attr, W_node, b_node):
    n = node_attr.shape[0]
    grid = n // _BN
    return pl.pallas_call(
        _node_body,
        grid=(grid,),
        in_specs=[
            pl.BlockSpec((_BN, node_attr.shape[1]), lambda i: (i, 0)),
            pl.BlockSpec(W_node.shape, lambda i: (0, 0)),
            pl.BlockSpec((1, b_node.shape[0]), lambda i: (0, 0)),
        ],
        out_specs=pl.BlockSpec((_BN, 64), lambda i: (i, 0)),
        out_shape=jax.ShapeDtypeStruct((n, 64), _F32),
    )(node_attr, W_node, b_node.reshape(1, -1))


def _edge_body(dij, et, hi, hj, w1, w2, wr, we4, bemb, x_out):
    d = dij[...] / _CUTOFF
    rbf = _rbf_from_d(d)
    oh = (et[...] == _iota4(et)).astype(_F32)
    acc = (_dot(hi[...], w1[...]) + _dot(hj[...], w2[...])
           + _dot(rbf, wr[...]) + _dot(oh, we4[...]) + bemb[...])
    x_out[...] = _swish(acc)


def _edge_embed(dij2, et2, hi, hj, w1, w2, wr, we4, bemb):
    e = dij2.shape[0]
    grid = e // _BE
    wspec = lambda a: pl.BlockSpec(a.shape, lambda i: (0, 0))
    return pl.pallas_call(
        _edge_body,
        grid=(grid,),
        in_specs=[
            pl.BlockSpec((_BE, 1), lambda i: (i, 0)),
            pl.BlockSpec((_BE, 1), lambda i: (i, 0)),
            pl.BlockSpec((_BE, 64), lambda i: (i, 0)),
            pl.BlockSpec((_BE, 64), lambda i: (i, 0)),
            wspec(w1), wspec(w2), wspec(wr), wspec(we4), wspec(bemb),
        ],
        out_specs=pl.BlockSpec((_BE, 64), lambda i: (i, 0)),
        out_shape=jax.ShapeDtypeStruct((e, 64), _F32),
    )(dij2, et2, hi, hj, w1, w2, wr, we4, bemb)


def _int_pre_body(x, et, dij, we4, wji, bji, wkj, bkj, wrbf, wdown,
                  xji_out, xdown_out):
    oh = (et[...] == _iota4(et)).astype(_F32)
    m = x[...] + _dot(oh, we4[...])
    xji_out[...] = _swish(_dot(m, wji[...]) + bji[...])
    d = dij[...] / _CUTOFF
    rbf_p = _dot(_rbf_from_d(d), wrbf[...])
    xkj = _swish(_dot(m, wkj[...]) + bkj[...]) * rbf_p
    xdown_out[...] = _dot(xkj, wdown[...])


def _int_pre(x, et2, dij2, we4, wji, bji, wkj, bkj, wrbf, wdown):
    e = x.shape[0]
    grid = e // _BE
    wspec = lambda a: pl.BlockSpec(a.shape, lambda i: (0, 0))
    return pl.pallas_call(
        _int_pre_body,
        grid=(grid,),
        in_specs=[
            pl.BlockSpec((_BE, 64), lambda i: (i, 0)),
            pl.BlockSpec((_BE, 1), lambda i: (i, 0)),
            pl.BlockSpec((_BE, 1), lambda i: (i, 0)),
            wspec(we4), wspec(wji), wspec(bji), wspec(wkj), wspec(bkj),
            wspec(wrbf), wspec(wdown),
        ],
        out_specs=[
            pl.BlockSpec((_BE, 64), lambda i: (i, 0)),
            pl.BlockSpec((_BE, 32), lambda i: (i, 0)),
        ],
        out_shape=[
            jax.ShapeDtypeStruct((e, 64), _F32),
            jax.ShapeDtypeStruct((e, 32), _F32),
        ],
    )(x, et2, dij2, we4, wji, bji, wkj, bkj, wrbf, wdown)


def _int_post_body(xji, seg, xold, wup, wb0, bb0, wb1, bb1, wskip, bskip,
                   wa00, ba00, wa01, ba01, wa10, ba10, wa11, ba11, x_out):
    hh = xji[...] + _dot(seg[...], wup[...])
    h2 = _swish(_dot(hh, wb0[...]) + bb0[...])
    h2 = _swish(_dot(h2, wb1[...]) + bb1[...])
    hh = hh + h2
    hh = _swish(_dot(hh, wskip[...]) + bskip[...]) + xold[...]
    h2 = _swish(_dot(hh, wa00[...]) + ba00[...])
    h2 = _swish(_dot(h2, wa01[...]) + ba01[...])
    hh = hh + h2
    h2 = _swish(_dot(hh, wa10[...]) + ba10[...])
    h2 = _swish(_dot(h2, wa11[...]) + ba11[...])
    x_out[...] = hh + h2


def _int_post(xji, seg, xold, *ws):
    e = xji.shape[0]
    grid = e // _BE
    wspec = lambda a: pl.BlockSpec(a.shape, lambda i: (0, 0))
    return pl.pallas_call(
        _int_post_body,
        grid=(grid,),
        in_specs=[
            pl.BlockSpec((_BE, 64), lambda i: (i, 0)),
            pl.BlockSpec((_BE, 32), lambda i: (i, 0)),
            pl.BlockSpec((_BE, 64), lambda i: (i, 0)),
        ] + [wspec(w) for w in ws],
        out_specs=pl.BlockSpec((_BE, 64), lambda i: (i, 0)),
        out_shape=jax.ShapeDtypeStruct((e, 64), _F32),
    )(xji, seg, xold, *ws)


def _sbf_body(ang, dt, wsb0, wsb1, sp0_out, sp1_out):
    ncols = _NSPH * _NRAD
    k = jax.lax.broadcasted_iota(jnp.int32, (1, ncols), 1)
    lcol = k // _NRAD
    ncol = k - lcol * _NRAD + 1
    zs = np.float32(np.pi) * (ncol.astype(_F32)
                              + 0.5 * lcol.astype(_F32))
    d = dt[...] / _CUTOFF + 1e-9
    env = _envelope(d)
    x = zs * d
    sx = jnp.sin(x)
    cx = jnp.cos(x)
    j0 = sx / x
    j1 = sx / (x * x) - cx / x
    res = jnp.where(lcol == 0, j0, 0.0)
    res = jnp.where(lcol == 1, j1, res)
    jm2, jm1 = j0, j1
    for ll in range(2, _NSPH):
        jl = (2.0 * ll - 1.0) / x * jm1 - jm2
        res = jnp.where(lcol == ll, jl, res)
        jm2, jm1 = jm1, jl
    c = jnp.cos(ang[...])
    ones42 = jnp.zeros_like(x) + 1.0
    p = jnp.where(lcol == 0, 1.0, 0.0)
    p = jnp.where(lcol == 1, c, p)
    pm2 = ones42
    pm1 = c * ones42
    for ll in range(2, _NSPH):
        pc = ((2.0 * ll - 1.0) * c * pm1 - (ll - 1.0) * pm2) / ll
        p = jnp.where(lcol == ll, pc, p)
        pm2, pm1 = pm1, pc
    sbf = env * res * p
    sp0_out[...] = _dot(sbf, wsb0[...])
    sp1_out[...] = _dot(sbf, wsb1[...])


def _sbf_project(ang2, dt2, wsb0, wsb1):
    t = ang2.shape[0]
    grid = t // _BT
    wspec = lambda a: pl.BlockSpec(a.shape, lambda i: (0, 0))
    return pl.pallas_call(
        _sbf_body,
        grid=(grid,),
        in_specs=[
            pl.BlockSpec((_BT, 1), lambda i: (i, 0)),
            pl.BlockSpec((_BT, 1), lambda i: (i, 0)),
            wspec(wsb0), wspec(wsb1),
        ],
        out_specs=[
            pl.BlockSpec((_BT, 32), lambda i: (i, 0)),
            pl.BlockSpec((_BT, 32), lambda i: (i, 0)),
        ],
        out_shape=[
            jax.ShapeDtypeStruct((t, 32), _F32),
            jax.ShapeDtypeStruct((t, 32), _F32),
        ],
    )(ang2, dt2, wsb0, wsb1)


def _out_body(t_in, wup, d0, b0, d1, b1, d2, b2, wf, p_out):
    t = _dot(t_in[...], wup[...])
    t = _swish(_dot(t, d0[...]) + b0[...])
    t = _swish(_dot(t, d1[...]) + b1[...])
    t = _swish(_dot(t, d2[...]) + b2[...])
    p_out[...] = _dot(t, wf[...])


def _out_block(t_nodes, wup, dw, db, wf_pad):
    n = t_nodes.shape[0]
    grid = n // _BN
    wspec = lambda a: pl.BlockSpec(a.shape, lambda i: (0, 0))
    args = [t_nodes, wup,
            dw[0], db[0].reshape(1, -1), dw[1], db[1].reshape(1, -1),
            dw[2], db[2].reshape(1, -1), wf_pad]
    return pl.pallas_call(
        _out_body,
        grid=(grid,),
        in_specs=[pl.BlockSpec((_BN, 64), lambda i: (i, 0))]
        + [wspec(a) for a in args[1:]],
        out_specs=pl.BlockSpec((_BN, 128), lambda i: (i, 0)),
        out_shape=jax.ShapeDtypeStruct((n, 128), _F32),
    )(*args)


def kernel(node_attr, edge_type, Dij, Anglesijk, batch_seg, idnb_i, idnb_j,
           id_expand_kj, id_reduce_ji, emb_table, W_rbf_emb, W_node, b_node,
           W_emb, b_emb, int_W_edge, int_W_rbf1, int_W_rbf2, int_W_sbf1,
           int_W_sbf2, int_W_ji, int_b_ji, int_W_kj, int_b_kj, int_W_down,
           int_W_up, int_res_bef_W, int_res_bef_b, int_W_skip, int_b_skip,
           int_res_aft_W, int_res_aft_b, out_W_up, out_dense_W, out_dense_b,
           out_W_final):
    n = node_attr.shape[0]
    e = Dij.shape[0]
    nmol = 512

    dij2 = Dij.reshape(e, 1)
    et2 = edge_type.astype(jnp.int32).reshape(e, 1)
    ang2 = Anglesijk.reshape(-1, 1)

    w1 = W_emb[0:64]
    w2 = W_emb[64:128]
    wr = W_rbf_emb @ W_emb[128:192]
    we4 = emb_table @ W_emb[192:256]
    bemb = b_emb.reshape(1, -1)

    h = _node_embed(node_attr, W_node, b_node)
    hi = jnp.take(h, idnb_i, axis=0)
    hj = jnp.take(h, idnb_j, axis=0)
    x = _edge_embed(dij2, et2, hi, hj, w1, w2, wr, we4, bemb)

    dt2 = jnp.take(Dij, id_reduce_ji, axis=0).reshape(-1, 1)
    wsb0 = int_W_sbf1[0] @ int_W_sbf2[0]
    wsb1 = int_W_sbf1[1] @ int_W_sbf2[1]
    sp = _sbf_project(ang2, dt2, wsb0, wsb1)

    wf_pad = [jnp.pad(out_W_final[i], ((0, 0), (0, 128 - out_W_final.shape[2])))
              for i in range(_NB + 1)]

    t0 = jax.ops.segment_sum(x, idnb_i, num_segments=n)
    P = _out_block(t0, out_W_up[0], out_dense_W[0], out_dense_b[0], wf_pad[0])

    for i in range(_NB):
        we4_i = emb_table @ int_W_edge[i]
        wrbf_i = int_W_rbf1[i] @ int_W_rbf2[i]
        xji, xdown = _int_pre(
            x, et2, dij2, we4_i,
            int_W_ji[i], int_b_ji[i].reshape(1, -1),
            int_W_kj[i], int_b_kj[i].reshape(1, -1),
            wrbf_i, int_W_down[i])
        msg = jnp.take(xdown, id_expand_kj, axis=0) * sp[i]
        seg = jax.ops.segment_sum(msg, id_reduce_ji, num_segments=e)
        x = _int_post(
            xji, seg, x, int_W_up[i],
            int_res_bef_W[i, 0, 0], int_res_bef_b[i, 0, 0].reshape(1, -1),
            int_res_bef_W[i, 0, 1], int_res_bef_b[i, 0, 1].reshape(1, -1),
            int_W_skip[i], int_b_skip[i].reshape(1, -1),
            int_res_aft_W[i, 0, 0], int_res_aft_b[i, 0, 0].reshape(1, -1),
            int_res_aft_W[i, 0, 1], int_res_aft_b[i, 0, 1].reshape(1, -1),
            int_res_aft_W[i, 1, 0], int_res_aft_b[i, 1, 0].reshape(1, -1),
            int_res_aft_W[i, 1, 1], int_res_aft_b[i, 1, 1].reshape(1, -1))
        ti = jax.ops.segment_sum(x, idnb_i, num_segments=n)
        P = P + _out_block(ti, out_W_up[i + 1], out_dense_W[i + 1],
                           out_dense_b[i + 1], wf_pad[i + 1])

    out = jax.ops.segment_sum(P, batch_seg, num_segments=nmol)
    return out[:, :12]


# speedup vs baseline: 1.0698x; 1.0698x over previous
import functools
import numpy as np
import jax
import jax.numpy as jnp
from jax.experimental import pallas as pl
from jax.experimental.pallas import tpu as pltpu

_CUTOFF = 5.0
_NRAD = 6
_NSPH = 7
_NB = 2
_NDO = 3

_BE = 1280
_BT = 3200
_BN = 2000

_F32 = jnp.float32


def _swish(x):
    return x * jax.nn.sigmoid(x)


def _envelope(d):
    a = -28.0
    b = 48.0
    c = -21.0
    d2 = d * d
    d4 = d2 * d2
    d5 = d4 * d
    env = 1.0 / d + a * d5 + b * d5 * d + c * d5 * d2
    return jnp.where(d < 1.0, env, 0.0)


def _rbf_from_d(d):
    k = jax.lax.broadcasted_iota(jnp.int32, (1, _NRAD), 1).astype(_F32)
    freq = (k + 1.0) * np.float32(np.pi)
    return _envelope(d) * jnp.sin(freq * d)


def _iota4(et):
    return jax.lax.broadcasted_iota(jnp.int32, (1, 4), 1)


def _dot(a, b):
    return jnp.dot(a, b, preferred_element_type=_F32)


def _node_body(na, w, b, h_out):
    h_out[...] = _dot(na[...], w[...]) + b[...]


def _node_embed(node_attr, W_node, b_node):
    n = node_attr.shape[0]
    grid = n // _BN
    return pl.pallas_call(
        _node_body,
        grid=(grid,),
        in_specs=[
            pl.BlockSpec((_BN, node_attr.shape[1]), lambda i: (i, 0)),
            pl.BlockSpec(W_node.shape, lambda i: (0, 0)),
            pl.BlockSpec((1, b_node.shape[0]), lambda i: (0, 0)),
        ],
        out_specs=pl.BlockSpec((_BN, 64), lambda i: (i, 0)),
        out_shape=jax.ShapeDtypeStruct((n, 64), _F32),
    )(node_attr, W_node, b_node.reshape(1, -1))


def _edge_body(dij, et, hi, hj, w1, w2, wr, we4, bemb, x_out, rbf_out):
    d = dij[...] / _CUTOFF
    rbf = _rbf_from_d(d)
    oh = (et[...] == _iota4(et)).astype(_F32)
    acc = (_dot(hi[...], w1[...]) + _dot(hj[...], w2[...])
           + _dot(rbf, wr[...]) + _dot(oh, we4[...]) + bemb[...])
    x_out[...] = _swish(acc)
    rbf_out[...] = jnp.concatenate(
        [rbf, jnp.zeros_like(rbf[:, 0:2])], axis=1)


def _edge_embed(dij2, et2, hi, hj, w1, w2, wr, we4, bemb):
    e = dij2.shape[0]
    grid = e // _BE
    wspec = lambda a: pl.BlockSpec(a.shape, lambda i: (0, 0))
    return pl.pallas_call(
        _edge_body,
        grid=(grid,),
        in_specs=[
            pl.BlockSpec((_BE, 1), lambda i: (i, 0)),
            pl.BlockSpec((_BE, 1), lambda i: (i, 0)),
            pl.BlockSpec((_BE, 64), lambda i: (i, 0)),
            pl.BlockSpec((_BE, 64), lambda i: (i, 0)),
            wspec(w1), wspec(w2), wspec(wr), wspec(we4), wspec(bemb),
        ],
        out_specs=[
            pl.BlockSpec((_BE, 64), lambda i: (i, 0)),
            pl.BlockSpec((_BE, 8), lambda i: (i, 0)),
        ],
        out_shape=[
            jax.ShapeDtypeStruct((e, 64), _F32),
            jax.ShapeDtypeStruct((e, 8), _F32),
        ],
    )(dij2, et2, hi, hj, w1, w2, wr, we4, bemb)


def _int_pre_body(x, et, rbf8, we4, wji, bji, wkj, bkj, wrbf, wdown,
                  xji_out, xdown_out):
    oh = (et[...] == _iota4(et)).astype(_F32)
    m = x[...] + _dot(oh, we4[...])
    xji_out[...] = _swish(_dot(m, wji[...]) + bji[...])
    rbf_p = _dot(rbf8[:, 0:_NRAD], wrbf[...])
    xkj = _swish(_dot(m, wkj[...]) + bkj[...]) * rbf_p
    xdown_out[...] = _dot(xkj, wdown[...])


def _int_pre(x, et2, rbf8, we4, wji, bji, wkj, bkj, wrbf, wdown):
    e = x.shape[0]
    grid = e // _BE
    wspec = lambda a: pl.BlockSpec(a.shape, lambda i: (0, 0))
    return pl.pallas_call(
        _int_pre_body,
        grid=(grid,),
        in_specs=[
            pl.BlockSpec((_BE, 64), lambda i: (i, 0)),
            pl.BlockSpec((_BE, 1), lambda i: (i, 0)),
            pl.BlockSpec((_BE, 8), lambda i: (i, 0)),
            wspec(we4), wspec(wji), wspec(bji), wspec(wkj), wspec(bkj),
            wspec(wrbf), wspec(wdown),
        ],
        out_specs=[
            pl.BlockSpec((_BE, 64), lambda i: (i, 0)),
            pl.BlockSpec((_BE, 32), lambda i: (i, 0)),
        ],
        out_shape=[
            jax.ShapeDtypeStruct((e, 64), _F32),
            jax.ShapeDtypeStruct((e, 32), _F32),
        ],
    )(x, et2, rbf8, we4, wji, bji, wkj, bkj, wrbf, wdown)


def _int_post_body(xji, seg, xold, wup, wb0, bb0, wb1, bb1, wskip, bskip,
                   wa00, ba00, wa01, ba01, wa10, ba10, wa11, ba11, x_out):
    hh = xji[...] + _dot(seg[...], wup[...])
    h2 = _swish(_dot(hh, wb0[...]) + bb0[...])
    h2 = _swish(_dot(h2, wb1[...]) + bb1[...])
    hh = hh + h2
    hh = _swish(_dot(hh, wskip[...]) + bskip[...]) + xold[...]
    h2 = _swish(_dot(hh, wa00[...]) + ba00[...])
    h2 = _swish(_dot(h2, wa01[...]) + ba01[...])
    hh = hh + h2
    h2 = _swish(_dot(hh, wa10[...]) + ba10[...])
    h2 = _swish(_dot(h2, wa11[...]) + ba11[...])
    x_out[...] = hh + h2


def _int_post(xji, seg, xold, *ws):
    e = xji.shape[0]
    grid = e // _BE
    wspec = lambda a: pl.BlockSpec(a.shape, lambda i: (0, 0))
    return pl.pallas_call(
        _int_post_body,
        grid=(grid,),
        in_specs=[
            pl.BlockSpec((_BE, 64), lambda i: (i, 0)),
            pl.BlockSpec((_BE, 32), lambda i: (i, 0)),
            pl.BlockSpec((_BE, 64), lambda i: (i, 0)),
        ] + [wspec(w) for w in ws],
        out_specs=pl.BlockSpec((_BE, 64), lambda i: (i, 0)),
        out_shape=jax.ShapeDtypeStruct((e, 64), _F32),
    )(xji, seg, xold, *ws)


def _sbf_body(cang, dt, wsb0, wsb1, sp0_out, sp1_out):
    ncols = _NSPH * _NRAD
    k = jax.lax.broadcasted_iota(jnp.int32, (1, ncols), 1)
    lcol = k // _NRAD
    ncol = k - lcol * _NRAD + 1
    zs = np.float32(np.pi) * (ncol.astype(_F32)
                              + 0.5 * lcol.astype(_F32))
    d = dt[...] / _CUTOFF + 1e-9
    env = _envelope(d)
    x = zs * d
    sx = jnp.sin(x)
    cx = jnp.cos(x)
    j0 = sx / x
    j1 = sx / (x * x) - cx / x
    res = jnp.where(lcol == 0, j0, 0.0)
    res = jnp.where(lcol == 1, j1, res)
    jm2, jm1 = j0, j1
    for ll in range(2, _NSPH):
        jl = (2.0 * ll - 1.0) / x * jm1 - jm2
        res = jnp.where(lcol == ll, jl, res)
        jm2, jm1 = jm1, jl
    c = cang[...]
    ones42 = jnp.zeros_like(x) + 1.0
    p = jnp.where(lcol == 0, 1.0, 0.0)
    p = jnp.where(lcol == 1, c, p)
    pm2 = ones42
    pm1 = c * ones42
    for ll in range(2, _NSPH):
        pc = ((2.0 * ll - 1.0) * c * pm1 - (ll - 1.0) * pm2) / ll
        p = jnp.where(lcol == ll, pc, p)
        pm2, pm1 = pm1, pc
    sbf = env * res * p
    sp0_out[...] = _dot(sbf, wsb0[...])
    sp1_out[...] = _dot(sbf, wsb1[...])


def _sbf_project(cang2, dt2, wsb0, wsb1):
    t = cang2.shape[0]
    grid = t // _BT
    wspec = lambda a: pl.BlockSpec(a.shape, lambda i: (0, 0))
    return pl.pallas_call(
        _sbf_body,
        grid=(grid,),
        in_specs=[
            pl.BlockSpec((_BT, 1), lambda i: (i, 0)),
            pl.BlockSpec((_BT, 1), lambda i: (i, 0)),
            wspec(wsb0), wspec(wsb1),
        ],
        out_specs=[
            pl.BlockSpec((_BT, 32), lambda i: (i, 0)),
            pl.BlockSpec((_BT, 32), lambda i: (i, 0)),
        ],
        out_shape=[
            jax.ShapeDtypeStruct((t, 32), _F32),
            jax.ShapeDtypeStruct((t, 32), _F32),
        ],
    )(cang2, dt2, wsb0, wsb1)


def _out_body(t_in, wup, d0, b0, d1, b1, d2, b2, wf, p_out):
    t = _dot(t_in[...], wup[...])
    t = _swish(_dot(t, d0[...]) + b0[...])
    t = _swish(_dot(t, d1[...]) + b1[...])
    t = _swish(_dot(t, d2[...]) + b2[...])
    p_out[...] = _dot(t, wf[...])


def _out_block(t_nodes, wup, dw, db, wf_pad):
    n = t_nodes.shape[0]
    grid = n // _BN
    wspec = lambda a: pl.BlockSpec(a.shape, lambda i: (0, 0))
    args = [t_nodes, wup,
            dw[0], db[0].reshape(1, -1), dw[1], db[1].reshape(1, -1),
            dw[2], db[2].reshape(1, -1), wf_pad]
    return pl.pallas_call(
        _out_body,
        grid=(grid,),
        in_specs=[pl.BlockSpec((_BN, 64), lambda i: (i, 0))]
        + [wspec(a) for a in args[1:]],
        out_specs=pl.BlockSpec((_BN, 128), lambda i: (i, 0)),
        out_shape=jax.ShapeDtypeStruct((n, 128), _F32),
    )(*args)


def kernel(node_attr, edge_type, Dij, Anglesijk, batch_seg, idnb_i, idnb_j,
           id_expand_kj, id_reduce_ji, emb_table, W_rbf_emb, W_node, b_node,
           W_emb, b_emb, int_W_edge, int_W_rbf1, int_W_rbf2, int_W_sbf1,
           int_W_sbf2, int_W_ji, int_b_ji, int_W_kj, int_b_kj, int_W_down,
           int_W_up, int_res_bef_W, int_res_bef_b, int_W_skip, int_b_skip,
           int_res_aft_W, int_res_aft_b, out_W_up, out_dense_W, out_dense_b,
           out_W_final):
    n = node_attr.shape[0]
    e = Dij.shape[0]
    nmol = 512

    dij2 = Dij.reshape(e, 1)
    et2 = edge_type.astype(jnp.int32).reshape(e, 1)
    cang2 = jnp.cos(Anglesijk).reshape(-1, 1)

    w1 = W_emb[0:64]
    w2 = W_emb[64:128]
    wr = W_rbf_emb @ W_emb[128:192]
    we4 = emb_table @ W_emb[192:256]
    bemb = b_emb.reshape(1, -1)

    h = _node_embed(node_attr, W_node, b_node)
    hi = jnp.take(h, idnb_i, axis=0)
    hj = jnp.take(h, idnb_j, axis=0)
    x, rbf8 = _edge_embed(dij2, et2, hi, hj, w1, w2, wr, we4, bemb)

    dt2 = jnp.take(Dij, id_reduce_ji, axis=0).reshape(-1, 1)
    wsb0 = int_W_sbf1[0] @ int_W_sbf2[0]
    wsb1 = int_W_sbf1[1] @ int_W_sbf2[1]
    sp = _sbf_project(cang2, dt2, wsb0, wsb1)

    wf_pad = [jnp.pad(out_W_final[i], ((0, 0), (0, 128 - out_W_final.shape[2])))
              for i in range(_NB + 1)]

    t0 = jax.ops.segment_sum(x, idnb_i, num_segments=n)
    P = _out_block(t0, out_W_up[0], out_dense_W[0], out_dense_b[0], wf_pad[0])

    for i in range(_NB):
        we4_i = emb_table @ int_W_edge[i]
        wrbf_i = int_W_rbf1[i] @ int_W_rbf2[i]
        xji, xdown = _int_pre(
            x, et2, rbf8, we4_i,
            int_W_ji[i], int_b_ji[i].reshape(1, -1),
            int_W_kj[i], int_b_kj[i].reshape(1, -1),
            wrbf_i, int_W_down[i])
        msg = jnp.take(xdown, id_expand_kj, axis=0) * sp[i]
        seg = jax.ops.segment_sum(msg, id_reduce_ji, num_segments=e)
        x = _int_post(
            xji, seg, x, int_W_up[i],
            int_res_bef_W[i, 0, 0], int_res_bef_b[i, 0, 0].reshape(1, -1),
            int_res_bef_W[i, 0, 1], int_res_bef_b[i, 0, 1].reshape(1, -1),
            int_W_skip[i], int_b_skip[i].reshape(1, -1),
            int_res_aft_W[i, 0, 0], int_res_aft_b[i, 0, 0].reshape(1, -1),
            int_res_aft_W[i, 0, 1], int_res_aft_b[i, 0, 1].reshape(1, -1),
            int_res_aft_W[i, 1, 0], int_res_aft_b[i, 1, 0].reshape(1, -1),
            int_res_aft_W[i, 1, 1], int_res_aft_b[i, 1, 1].reshape(1, -1))
        ti = jax.ops.segment_sum(x, idnb_i, num_segments=n)
        P = P + _out_block(ti, out_W_up[i + 1], out_dense_W[i + 1],
                           out_dense_b[i + 1], wf_pad[i + 1])

    out = jax.ops.segment_sum(P, batch_seg, num_segments=nmol)
    return out[:, :12]


# speedup vs baseline: 1.4583x; 1.3632x over previous
import functools
import numpy as np
import jax
import jax.numpy as jnp
from jax import lax
from jax.experimental import pallas as pl
from jax.experimental.pallas import tpu as pltpu
from jax.experimental.pallas import tpu_sc as plsc

_CUTOFF = 5.0
_NRAD = 6
_NSPH = 7
_NB = 2
_NDO = 3

_BE = 1280
_BT = 3200
_BN = 2000

_F32 = jnp.float32


def _swish(x):
    return x * jax.nn.sigmoid(x)


def _envelope(d):
    a = -28.0
    b = 48.0
    c = -21.0
    d2 = d * d
    d4 = d2 * d2
    d5 = d4 * d
    env = 1.0 / d + a * d5 + b * d5 * d + c * d5 * d2
    return jnp.where(d < 1.0, env, 0.0)


def _rbf_from_d(d):
    k = jax.lax.broadcasted_iota(jnp.int32, (1, _NRAD), 1).astype(_F32)
    freq = (k + 1.0) * np.float32(np.pi)
    return _envelope(d) * jnp.sin(freq * d)


def _iota4(et):
    return jax.lax.broadcasted_iota(jnp.int32, (1, 4), 1)


def _dot(a, b):
    return jnp.dot(a, b, preferred_element_type=_F32)


def _node_body(na, w, b, h_out):
    h_out[...] = _dot(na[...], w[...]) + b[...]


def _node_embed(node_attr, W_node, b_node):
    n = node_attr.shape[0]
    grid = n // _BN
    return pl.pallas_call(
        _node_body,
        grid=(grid,),
        in_specs=[
            pl.BlockSpec((_BN, node_attr.shape[1]), lambda i: (i, 0)),
            pl.BlockSpec(W_node.shape, lambda i: (0, 0)),
            pl.BlockSpec((1, b_node.shape[0]), lambda i: (0, 0)),
        ],
        out_specs=pl.BlockSpec((_BN, 64), lambda i: (i, 0)),
        out_shape=jax.ShapeDtypeStruct((n, 64), _F32),
    )(node_attr, W_node, b_node.reshape(1, -1))


def _edge_body(dij, et, hi, hj, w1, w2, wr, we4, bemb, x_out, rbf_out):
    d = dij[...] / _CUTOFF
    rbf = _rbf_from_d(d)
    oh = (et[...] == _iota4(et)).astype(_F32)
    acc = (_dot(hi[...], w1[...]) + _dot(hj[...], w2[...])
           + _dot(rbf, wr[...]) + _dot(oh, we4[...]) + bemb[...])
    x_out[...] = _swish(acc)
    rbf_out[...] = jnp.concatenate(
        [rbf, jnp.zeros_like(rbf[:, 0:2])], axis=1)


def _edge_embed(dij2, et2, hi, hj, w1, w2, wr, we4, bemb):
    e = dij2.shape[0]
    grid = e // _BE
    wspec = lambda a: pl.BlockSpec(a.shape, lambda i: (0, 0))
    return pl.pallas_call(
        _edge_body,
        grid=(grid,),
        in_specs=[
            pl.BlockSpec((_BE, 1), lambda i: (i, 0)),
            pl.BlockSpec((_BE, 1), lambda i: (i, 0)),
            pl.BlockSpec((_BE, 64), lambda i: (i, 0)),
            pl.BlockSpec((_BE, 64), lambda i: (i, 0)),
            wspec(w1), wspec(w2), wspec(wr), wspec(we4), wspec(bemb),
        ],
        out_specs=[
            pl.BlockSpec((_BE, 64), lambda i: (i, 0)),
            pl.BlockSpec((_BE, 8), lambda i: (i, 0)),
        ],
        out_shape=[
            jax.ShapeDtypeStruct((e, 64), _F32),
            jax.ShapeDtypeStruct((e, 8), _F32),
        ],
    )(dij2, et2, hi, hj, w1, w2, wr, we4, bemb)


def _int_pre_body(x, et, rbf8, we4, wji, bji, wkj, bkj, wrbf, wdown,
                  xji_out, xdown_out):
    oh = (et[...] == _iota4(et)).astype(_F32)
    m = x[...] + _dot(oh, we4[...])
    xji_out[...] = _swish(_dot(m, wji[...]) + bji[...])
    rbf_p = _dot(rbf8[:, 0:_NRAD], wrbf[...])
    xkj = _swish(_dot(m, wkj[...]) + bkj[...]) * rbf_p
    xdown_out[...] = _dot(xkj, wdown[...])


def _int_pre(x, et2, rbf8, we4, wji, bji, wkj, bkj, wrbf, wdown):
    e = x.shape[0]
    grid = e // _BE
    wspec = lambda a: pl.BlockSpec(a.shape, lambda i: (0, 0))
    return pl.pallas_call(
        _int_pre_body,
        grid=(grid,),
        in_specs=[
            pl.BlockSpec((_BE, 64), lambda i: (i, 0)),
            pl.BlockSpec((_BE, 1), lambda i: (i, 0)),
            pl.BlockSpec((_BE, 8), lambda i: (i, 0)),
            wspec(we4), wspec(wji), wspec(bji), wspec(wkj), wspec(bkj),
            wspec(wrbf), wspec(wdown),
        ],
        out_specs=[
            pl.BlockSpec((_BE, 64), lambda i: (i, 0)),
            pl.BlockSpec((_BE, 32), lambda i: (i, 0)),
        ],
        out_shape=[
            jax.ShapeDtypeStruct((e, 64), _F32),
            jax.ShapeDtypeStruct((e, 32), _F32),
        ],
    )(x, et2, rbf8, we4, wji, bji, wkj, bkj, wrbf, wdown)


def _int_post_body(xji, seg, xold, wup, wb0, bb0, wb1, bb1, wskip, bskip,
                   wa00, ba00, wa01, ba01, wa10, ba10, wa11, ba11, x_out):
    hh = xji[...] + _dot(seg[...], wup[...])
    h2 = _swish(_dot(hh, wb0[...]) + bb0[...])
    h2 = _swish(_dot(h2, wb1[...]) + bb1[...])
    hh = hh + h2
    hh = _swish(_dot(hh, wskip[...]) + bskip[...]) + xold[...]
    h2 = _swish(_dot(hh, wa00[...]) + ba00[...])
    h2 = _swish(_dot(h2, wa01[...]) + ba01[...])
    hh = hh + h2
    h2 = _swish(_dot(hh, wa10[...]) + ba10[...])
    h2 = _swish(_dot(h2, wa11[...]) + ba11[...])
    x_out[...] = hh + h2


def _int_post(xji, seg, xold, *ws):
    e = xji.shape[0]
    grid = e // _BE
    wspec = lambda a: pl.BlockSpec(a.shape, lambda i: (0, 0))
    return pl.pallas_call(
        _int_post_body,
        grid=(grid,),
        in_specs=[
            pl.BlockSpec((_BE, 64), lambda i: (i, 0)),
            pl.BlockSpec((_BE, 32), lambda i: (i, 0)),
            pl.BlockSpec((_BE, 64), lambda i: (i, 0)),
        ] + [wspec(w) for w in ws],
        out_specs=pl.BlockSpec((_BE, 64), lambda i: (i, 0)),
        out_shape=jax.ShapeDtypeStruct((e, 64), _F32),
    )(xji, seg, xold, *ws)


def _sbf_body(cang, dt, wsb0, wsb1, sp0_out, sp1_out):
    ncols = _NSPH * _NRAD
    k = jax.lax.broadcasted_iota(jnp.int32, (1, ncols), 1)
    lcol = k // _NRAD
    ncol = k - lcol * _NRAD + 1
    zs = np.float32(np.pi) * (ncol.astype(_F32)
                              + 0.5 * lcol.astype(_F32))
    d = dt[...] / _CUTOFF + 1e-9
    env = _envelope(d)
    x = zs * d
    sx = jnp.sin(x)
    cx = jnp.cos(x)
    j0 = sx / x
    j1 = sx / (x * x) - cx / x
    res = jnp.where(lcol == 0, j0, 0.0)
    res = jnp.where(lcol == 1, j1, res)
    jm2, jm1 = j0, j1
    for ll in range(2, _NSPH):
        jl = (2.0 * ll - 1.0) / x * jm1 - jm2
        res = jnp.where(lcol == ll, jl, res)
        jm2, jm1 = jm1, jl
    c = cang[...]
    ones42 = jnp.zeros_like(x) + 1.0
    p = jnp.where(lcol == 0, 1.0, 0.0)
    p = jnp.where(lcol == 1, c, p)
    pm2 = ones42
    pm1 = c * ones42
    for ll in range(2, _NSPH):
        pc = ((2.0 * ll - 1.0) * c * pm1 - (ll - 1.0) * pm2) / ll
        p = jnp.where(lcol == ll, pc, p)
        pm2, pm1 = pm1, pc
    sbf = env * res * p
    sp0_out[...] = _dot(sbf, wsb0[...])
    sp1_out[...] = _dot(sbf, wsb1[...])


def _sbf_project(cang2, dt2, wsb0, wsb1):
    t = cang2.shape[0]
    grid = t // _BT
    wspec = lambda a: pl.BlockSpec(a.shape, lambda i: (0, 0))
    return pl.pallas_call(
        _sbf_body,
        grid=(grid,),
        in_specs=[
            pl.BlockSpec((_BT, 1), lambda i: (i, 0)),
            pl.BlockSpec((_BT, 1), lambda i: (i, 0)),
            wspec(wsb0), wspec(wsb1),
        ],
        out_specs=[
            pl.BlockSpec((_BT, 32), lambda i: (i, 0)),
            pl.BlockSpec((_BT, 32), lambda i: (i, 0)),
        ],
        out_shape=[
            jax.ShapeDtypeStruct((t, 32), _F32),
            jax.ShapeDtypeStruct((t, 32), _F32),
        ],
    )(cang2, dt2, wsb0, wsb1)


def _out_body(t_in, wup, d0, b0, d1, b1, d2, b2, wf, p_out):
    t = _dot(t_in[...], wup[...])
    t = _swish(_dot(t, d0[...]) + b0[...])
    t = _swish(_dot(t, d1[...]) + b1[...])
    t = _swish(_dot(t, d2[...]) + b2[...])
    p_out[...] = _dot(t, wf[...])


def _out_block(t_nodes, wup, dw, db, wf_pad):
    n = t_nodes.shape[0]
    grid = n // _BN
    wspec = lambda a: pl.BlockSpec(a.shape, lambda i: (0, 0))
    args = [t_nodes, wup,
            dw[0], db[0].reshape(1, -1), dw[1], db[1].reshape(1, -1),
            dw[2], db[2].reshape(1, -1), wf_pad]
    return pl.pallas_call(
        _out_body,
        grid=(grid,),
        in_specs=[pl.BlockSpec((_BN, 64), lambda i: (i, 0))]
        + [wspec(a) for a in args[1:]],
        out_specs=pl.BlockSpec((_BN, 128), lambda i: (i, 0)),
        out_shape=jax.ShapeDtypeStruct((n, 128), _F32),
    )(*args)


_SC_NC = 2
_SC_NS = 16
_SC_NW = _SC_NC * _SC_NS
_SC_IW = 125
_SC_CH = 1000


def _sc_gmul_body(table_hbm, idx_hbm, sp_hbm, out_hbm, idx_v, rows_v, sp_v,
                  sem):
    t_total = out_hbm.shape[0]
    n_chunks = t_total // (_SC_NW * _SC_CH)
    wid = lax.axis_index("s") * _SC_NC + lax.axis_index("c")
    base_row = wid * (n_chunks * (_SC_CH // _SC_IW))

    def chunk_body(k, carry):
        row0 = base_row + k * (_SC_CH // _SC_IW)
        t0 = row0 * _SC_IW
        pltpu.sync_copy(idx_hbm.at[pl.ds(row0, _SC_CH // _SC_IW)], idx_v)
        copies = []
        for j in range(_SC_CH // _SC_IW):
            copies.append(pltpu.async_copy(
                table_hbm.at[idx_v.at[j]],
                rows_v.at[pl.ds(j * _SC_IW, _SC_IW)], sem))
        for cp in copies:
            cp.wait()
        pltpu.sync_copy(sp_hbm.at[pl.ds(t0 * 32, _SC_CH * 32)], sp_v)

        def mul_body(r, c2):
            rr = r * 4
            for u in range(4):
                a0 = rows_v[rr + u, pl.ds(0, 16)]
                a1 = rows_v[rr + u, pl.ds(16, 16)]
                b0 = sp_v[pl.ds((rr + u) * 32, 16)]
                b1 = sp_v[pl.ds((rr + u) * 32 + 16, 16)]
                rows_v[rr + u, pl.ds(0, 16)] = a0 * b0
                rows_v[rr + u, pl.ds(16, 16)] = a1 * b1
            return c2

        lax.fori_loop(0, _SC_CH // 4, mul_body, 0)
        pltpu.sync_copy(rows_v, out_hbm.at[pl.ds(t0, _SC_CH)])
        return carry

    lax.fori_loop(0, n_chunks, chunk_body, 0)


def _sc_gather_mul(table, idx_rows, sp_flat):
    t_total = idx_rows.shape[0] * idx_rows.shape[1]
    mesh = plsc.VectorSubcoreMesh(core_axis_name="c", subcore_axis_name="s",
                                  num_cores=_SC_NC, num_subcores=_SC_NS)
    f = pl.kernel(
        _sc_gmul_body,
        out_type=jax.ShapeDtypeStruct((t_total, 32), _F32),
        mesh=mesh,
        scratch_types=[
            pltpu.VMEM((_SC_CH // _SC_IW, _SC_IW), jnp.int32),
            pltpu.VMEM((_SC_CH, 32), _F32),
            pltpu.VMEM((_SC_CH * 32,), _F32),
            pltpu.SemaphoreType.DMA,
        ],
        compiler_params=pltpu.CompilerParams(use_tc_tiling_on_sc=False),
    )
    return f(table, idx_rows, sp_flat)


def kernel(node_attr, edge_type, Dij, Anglesijk, batch_seg, idnb_i, idnb_j,
           id_expand_kj, id_reduce_ji, emb_table, W_rbf_emb, W_node, b_node,
           W_emb, b_emb, int_W_edge, int_W_rbf1, int_W_rbf2, int_W_sbf1,
           int_W_sbf2, int_W_ji, int_b_ji, int_W_kj, int_b_kj, int_W_down,
           int_W_up, int_res_bef_W, int_res_bef_b, int_W_skip, int_b_skip,
           int_res_aft_W, int_res_aft_b, out_W_up, out_dense_W, out_dense_b,
           out_W_final):
    n = node_attr.shape[0]
    e = Dij.shape[0]
    nmol = 512

    dij2 = Dij.reshape(e, 1)
    et2 = edge_type.astype(jnp.int32).reshape(e, 1)
    cang2 = jnp.cos(Anglesijk).reshape(-1, 1)

    w1 = W_emb[0:64]
    w2 = W_emb[64:128]
    wr = W_rbf_emb @ W_emb[128:192]
    we4 = emb_table @ W_emb[192:256]
    bemb = b_emb.reshape(1, -1)

    h = _node_embed(node_attr, W_node, b_node)
    hi = jnp.take(h, idnb_i, axis=0)
    hj = jnp.take(h, idnb_j, axis=0)
    x, rbf8 = _edge_embed(dij2, et2, hi, hj, w1, w2, wr, we4, bemb)

    dt2 = jnp.take(Dij, id_reduce_ji, axis=0).reshape(-1, 1)
    idx_rows = id_expand_kj.astype(jnp.int32).reshape(-1, _SC_IW)
    wsb0 = int_W_sbf1[0] @ int_W_sbf2[0]
    wsb1 = int_W_sbf1[1] @ int_W_sbf2[1]
    sp = _sbf_project(cang2, dt2, wsb0, wsb1)

    wf_pad = [jnp.pad(out_W_final[i], ((0, 0), (0, 128 - out_W_final.shape[2])))
              for i in range(_NB + 1)]

    t0 = jax.ops.segment_sum(x, idnb_i, num_segments=n)
    P = _out_block(t0, out_W_up[0], out_dense_W[0], out_dense_b[0], wf_pad[0])

    for i in range(_NB):
        we4_i = emb_table @ int_W_edge[i]
        wrbf_i = int_W_rbf1[i] @ int_W_rbf2[i]
        xji, xdown = _int_pre(
            x, et2, rbf8, we4_i,
            int_W_ji[i], int_b_ji[i].reshape(1, -1),
            int_W_kj[i], int_b_kj[i].reshape(1, -1),
            wrbf_i, int_W_down[i])
        msg = _sc_gather_mul(xdown, idx_rows, sp[i].reshape(-1))
        seg = jax.ops.segment_sum(msg, id_reduce_ji, num_segments=e)
        x = _int_post(
            xji, seg, x, int_W_up[i],
            int_res_bef_W[i, 0, 0], int_res_bef_b[i, 0, 0].reshape(1, -1),
            int_res_bef_W[i, 0, 1], int_res_bef_b[i, 0, 1].reshape(1, -1),
            int_W_skip[i], int_b_skip[i].reshape(1, -1),
            int_res_aft_W[i, 0, 0], int_res_aft_b[i, 0, 0].reshape(1, -1),
            int_res_aft_W[i, 0, 1], int_res_aft_b[i, 0, 1].reshape(1, -1),
            int_res_aft_W[i, 1, 0], int_res_aft_b[i, 1, 0].reshape(1, -1),
            int_res_aft_W[i, 1, 1], int_res_aft_b[i, 1, 1].reshape(1, -1))
        ti = jax.ops.segment_sum(x, idnb_i, num_segments=n)
        P = P + _out_block(ti, out_W_up[i + 1], out_dense_W[i + 1],
                           out_dense_b[i + 1], wf_pad[i + 1])

    out = jax.ops.segment_sum(P, batch_seg, num_segments=nmol)
    return out[:, :12]


# speedup vs baseline: 1.5044x; 1.0316x over previous
import functools
import numpy as np
import jax
import jax.numpy as jnp
from jax import lax
from jax.experimental import pallas as pl
from jax.experimental.pallas import tpu as pltpu
from jax.experimental.pallas import tpu_sc as plsc

_CUTOFF = 5.0
_NRAD = 6
_NSPH = 7
_NB = 2
_NDO = 3

_BE = 1280
_BT = 3200
_BN = 2000

_F32 = jnp.float32


def _swish(x):
    return x * jax.nn.sigmoid(x)


def _envelope(d):
    a = -28.0
    b = 48.0
    c = -21.0
    d2 = d * d
    d4 = d2 * d2
    d5 = d4 * d
    env = 1.0 / d + a * d5 + b * d5 * d + c * d5 * d2
    return jnp.where(d < 1.0, env, 0.0)


def _rbf_from_d(d):
    k = jax.lax.broadcasted_iota(jnp.int32, (1, _NRAD), 1).astype(_F32)
    freq = (k + 1.0) * np.float32(np.pi)
    return _envelope(d) * jnp.sin(freq * d)


def _iota4(et):
    return jax.lax.broadcasted_iota(jnp.int32, (1, 4), 1)


def _dot(a, b):
    return jnp.dot(a, b, preferred_element_type=_F32)


def _node_body(na, w, b, h_out):
    h_out[...] = _dot(na[...], w[...]) + b[...]


def _node_embed(node_attr, W_node, b_node):
    n = node_attr.shape[0]
    grid = n // _BN
    return pl.pallas_call(
        _node_body,
        grid=(grid,),
        in_specs=[
            pl.BlockSpec((_BN, node_attr.shape[1]), lambda i: (i, 0)),
            pl.BlockSpec(W_node.shape, lambda i: (0, 0)),
            pl.BlockSpec((1, b_node.shape[0]), lambda i: (0, 0)),
        ],
        out_specs=pl.BlockSpec((_BN, 64), lambda i: (i, 0)),
        out_shape=jax.ShapeDtypeStruct((n, 64), _F32),
    )(node_attr, W_node, b_node.reshape(1, -1))


def _edge_body(dij, et, hi, hj, w1, w2, wr, we4, bemb, x_out, rbf_out):
    d = dij[...] / _CUTOFF
    rbf = _rbf_from_d(d)
    oh = (et[...] == _iota4(et)).astype(_F32)
    acc = (_dot(hi[...], w1[...]) + _dot(hj[...], w2[...])
           + _dot(rbf, wr[...]) + _dot(oh, we4[...]) + bemb[...])
    x_out[...] = _swish(acc)
    rbf_out[...] = jnp.concatenate(
        [rbf, jnp.zeros_like(rbf[:, 0:2])], axis=1)


def _edge_embed(dij2, et2, hi, hj, w1, w2, wr, we4, bemb):
    e = dij2.shape[0]
    grid = e // _BE
    wspec = lambda a: pl.BlockSpec(a.shape, lambda i: (0, 0))
    return pl.pallas_call(
        _edge_body,
        grid=(grid,),
        in_specs=[
            pl.BlockSpec((_BE, 1), lambda i: (i, 0)),
            pl.BlockSpec((_BE, 1), lambda i: (i, 0)),
            pl.BlockSpec((_BE, 64), lambda i: (i, 0)),
            pl.BlockSpec((_BE, 64), lambda i: (i, 0)),
            wspec(w1), wspec(w2), wspec(wr), wspec(we4), wspec(bemb),
        ],
        out_specs=[
            pl.BlockSpec((_BE, 64), lambda i: (i, 0)),
            pl.BlockSpec((_BE, 8), lambda i: (i, 0)),
        ],
        out_shape=[
            jax.ShapeDtypeStruct((e, 64), _F32),
            jax.ShapeDtypeStruct((e, 8), _F32),
        ],
    )(dij2, et2, hi, hj, w1, w2, wr, we4, bemb)


def _int_pre_body(x, et, rbf8, we4, wji, bji, wkj, bkj, wrbf, wdown,
                  xji_out, xdown_out):
    oh = (et[...] == _iota4(et)).astype(_F32)
    m = x[...] + _dot(oh, we4[...])
    xji_out[...] = _swish(_dot(m, wji[...]) + bji[...])
    rbf_p = _dot(rbf8[:, 0:_NRAD], wrbf[...])
    xkj = _swish(_dot(m, wkj[...]) + bkj[...]) * rbf_p
    xdown_out[...] = _dot(xkj, wdown[...])


def _int_pre(x, et2, rbf8, we4, wji, bji, wkj, bkj, wrbf, wdown):
    e = x.shape[0]
    grid = e // _BE
    wspec = lambda a: pl.BlockSpec(a.shape, lambda i: (0, 0))
    return pl.pallas_call(
        _int_pre_body,
        grid=(grid,),
        in_specs=[
            pl.BlockSpec((_BE, 64), lambda i: (i, 0)),
            pl.BlockSpec((_BE, 1), lambda i: (i, 0)),
            pl.BlockSpec((_BE, 8), lambda i: (i, 0)),
            wspec(we4), wspec(wji), wspec(bji), wspec(wkj), wspec(bkj),
            wspec(wrbf), wspec(wdown),
        ],
        out_specs=[
            pl.BlockSpec((_BE, 64), lambda i: (i, 0)),
            pl.BlockSpec((_BE, 32), lambda i: (i, 0)),
        ],
        out_shape=[
            jax.ShapeDtypeStruct((e, 64), _F32),
            jax.ShapeDtypeStruct((e, 32), _F32),
        ],
    )(x, et2, rbf8, we4, wji, bji, wkj, bkj, wrbf, wdown)


def _int_post_body(xji, seg, xold, wup, wb0, bb0, wb1, bb1, wskip, bskip,
                   wa00, ba00, wa01, ba01, wa10, ba10, wa11, ba11, x_out):
    hh = xji[...] + _dot(seg[...], wup[...])
    h2 = _swish(_dot(hh, wb0[...]) + bb0[...])
    h2 = _swish(_dot(h2, wb1[...]) + bb1[...])
    hh = hh + h2
    hh = _swish(_dot(hh, wskip[...]) + bskip[...]) + xold[...]
    h2 = _swish(_dot(hh, wa00[...]) + ba00[...])
    h2 = _swish(_dot(h2, wa01[...]) + ba01[...])
    hh = hh + h2
    h2 = _swish(_dot(hh, wa10[...]) + ba10[...])
    h2 = _swish(_dot(h2, wa11[...]) + ba11[...])
    x_out[...] = hh + h2


def _int_post(xji, seg, xold, *ws):
    e = xji.shape[0]
    grid = e // _BE
    wspec = lambda a: pl.BlockSpec(a.shape, lambda i: (0, 0))
    return pl.pallas_call(
        _int_post_body,
        grid=(grid,),
        in_specs=[
            pl.BlockSpec((_BE, 64), lambda i: (i, 0)),
            pl.BlockSpec((_BE, 32), lambda i: (i, 0)),
            pl.BlockSpec((_BE, 64), lambda i: (i, 0)),
        ] + [wspec(w) for w in ws],
        out_specs=pl.BlockSpec((_BE, 64), lambda i: (i, 0)),
        out_shape=jax.ShapeDtypeStruct((e, 64), _F32),
    )(xji, seg, xold, *ws)


def _sbf_body(cang, dt, wsb0, wsb1, sp0_out, sp1_out):
    ncols = _NSPH * _NRAD
    k = jax.lax.broadcasted_iota(jnp.int32, (1, ncols), 1)
    lcol = k // _NRAD
    ncol = k - lcol * _NRAD + 1
    zs = np.float32(np.pi) * (ncol.astype(_F32)
                              + 0.5 * lcol.astype(_F32))
    d = dt[...] / _CUTOFF + 1e-9
    env = _envelope(d)
    x = zs * d
    sx = jnp.sin(x)
    cx = jnp.cos(x)
    j0 = sx / x
    j1 = sx / (x * x) - cx / x
    res = jnp.where(lcol == 0, j0, 0.0)
    res = jnp.where(lcol == 1, j1, res)
    jm2, jm1 = j0, j1
    for ll in range(2, _NSPH):
        jl = (2.0 * ll - 1.0) / x * jm1 - jm2
        res = jnp.where(lcol == ll, jl, res)
        jm2, jm1 = jm1, jl
    c = cang[...]
    ones42 = jnp.zeros_like(x) + 1.0
    p = jnp.where(lcol == 0, 1.0, 0.0)
    p = jnp.where(lcol == 1, c, p)
    pm2 = ones42
    pm1 = c * ones42
    for ll in range(2, _NSPH):
        pc = ((2.0 * ll - 1.0) * c * pm1 - (ll - 1.0) * pm2) / ll
        p = jnp.where(lcol == ll, pc, p)
        pm2, pm1 = pm1, pc
    sbf = env * res * p
    sp0_out[...] = _dot(sbf, wsb0[...])
    sp1_out[...] = _dot(sbf, wsb1[...])


def _sbf_project(cang2, dt2, wsb0, wsb1):
    t = cang2.shape[0]
    grid = t // _BT
    wspec = lambda a: pl.BlockSpec(a.shape, lambda i: (0, 0))
    return pl.pallas_call(
        _sbf_body,
        grid=(grid,),
        in_specs=[
            pl.BlockSpec((_BT, 1), lambda i: (i, 0)),
            pl.BlockSpec((_BT, 1), lambda i: (i, 0)),
            wspec(wsb0), wspec(wsb1),
        ],
        out_specs=[
            pl.BlockSpec((_BT, 32), lambda i: (i, 0)),
            pl.BlockSpec((_BT, 32), lambda i: (i, 0)),
        ],
        out_shape=[
            jax.ShapeDtypeStruct((t, 32), _F32),
            jax.ShapeDtypeStruct((t, 32), _F32),
        ],
    )(cang2, dt2, wsb0, wsb1)


def _out_body(t_in, wup, d0, b0, d1, b1, d2, b2, wf, p_out):
    t = _dot(t_in[...], wup[...])
    t = _swish(_dot(t, d0[...]) + b0[...])
    t = _swish(_dot(t, d1[...]) + b1[...])
    t = _swish(_dot(t, d2[...]) + b2[...])
    p_out[...] = _dot(t, wf[...])


def _out_block(t_nodes, wup, dw, db, wf_pad):
    n = t_nodes.shape[0]
    grid = n // _BN
    wspec = lambda a: pl.BlockSpec(a.shape, lambda i: (0, 0))
    args = [t_nodes, wup,
            dw[0], db[0].reshape(1, -1), dw[1], db[1].reshape(1, -1),
            dw[2], db[2].reshape(1, -1), wf_pad]
    return pl.pallas_call(
        _out_body,
        grid=(grid,),
        in_specs=[pl.BlockSpec((_BN, 64), lambda i: (i, 0))]
        + [wspec(a) for a in args[1:]],
        out_specs=pl.BlockSpec((_BN, 128), lambda i: (i, 0)),
        out_shape=jax.ShapeDtypeStruct((n, 128), _F32),
    )(*args)


_SC_NC = 2
_SC_NS = 16
_SC_NW = _SC_NC * _SC_NS
_SC_IW = 125
_SC_CH = 1000


def _sc_gmul_body(table_hbm, idx_hbm, sp_hbm, out_hbm, idx_v, rows_v, sp_v,
                  sem):
    t_total = out_hbm.shape[0]
    n_chunks = t_total // (_SC_NW * _SC_CH)
    wid = lax.axis_index("s") * _SC_NC + lax.axis_index("c")
    base_row = wid * (n_chunks * (_SC_CH // _SC_IW))

    def chunk_body(k, carry):
        row0 = base_row + k * (_SC_CH // _SC_IW)
        t0 = row0 * _SC_IW
        pltpu.sync_copy(idx_hbm.at[pl.ds(row0, _SC_CH // _SC_IW)], idx_v)
        copies = []
        for j in range(_SC_CH // _SC_IW):
            copies.append(pltpu.async_copy(
                table_hbm.at[idx_v.at[j]],
                rows_v.at[pl.ds(j * _SC_IW, _SC_IW)], sem))
        for cp in copies:
            cp.wait()
        pltpu.sync_copy(sp_hbm.at[pl.ds(t0 * 32, _SC_CH * 32)], sp_v)

        def mul_body(r, c2):
            rr = r * 4
            for u in range(4):
                a0 = rows_v[rr + u, pl.ds(0, 16)]
                a1 = rows_v[rr + u, pl.ds(16, 16)]
                b0 = sp_v[pl.ds((rr + u) * 32, 16)]
                b1 = sp_v[pl.ds((rr + u) * 32 + 16, 16)]
                rows_v[rr + u, pl.ds(0, 16)] = a0 * b0
                rows_v[rr + u, pl.ds(16, 16)] = a1 * b1
            return c2

        lax.fori_loop(0, _SC_CH // 4, mul_body, 0)
        pltpu.sync_copy(rows_v, out_hbm.at[pl.ds(t0, _SC_CH)])
        return carry

    lax.fori_loop(0, n_chunks, chunk_body, 0)


def _sc_gather_mul(table, idx_rows, sp_flat):
    t_total = idx_rows.shape[0] * idx_rows.shape[1]
    mesh = plsc.VectorSubcoreMesh(core_axis_name="c", subcore_axis_name="s",
                                  num_cores=_SC_NC, num_subcores=_SC_NS)
    f = pl.kernel(
        _sc_gmul_body,
        out_type=jax.ShapeDtypeStruct((t_total, 32), _F32),
        mesh=mesh,
        scratch_types=[
            pltpu.VMEM((_SC_CH // _SC_IW, _SC_IW), jnp.int32),
            pltpu.VMEM((_SC_CH, 32), _F32),
            pltpu.VMEM((_SC_CH * 32,), _F32),
            pltpu.SemaphoreType.DMA,
        ],
        compiler_params=pltpu.CompilerParams(use_tc_tiling_on_sc=False),
    )
    return f(table, idx_rows, sp_flat)


def _sc_gather2_body(table_hbm, idxa_hbm, idxb_hbm, outa_hbm, outb_hbm,
                     idx_v, rows_v, sem):
    b_total = outa_hbm.shape[0]
    d = outa_hbm.shape[1]
    n_chunks = b_total // (_SC_NW * _SC_CH)
    wid = lax.axis_index("s") * _SC_NC + lax.axis_index("c")
    rows_per_chunk = _SC_CH // _SC_IW
    base_row = wid * (n_chunks * rows_per_chunk)

    def chunk_body(k, carry):
        row0 = base_row + k * rows_per_chunk
        t0 = row0 * _SC_IW
        for idx_hbm, out_hbm in ((idxa_hbm, outa_hbm), (idxb_hbm, outb_hbm)):
            pltpu.sync_copy(idx_hbm.at[pl.ds(row0, rows_per_chunk)], idx_v)
            copies = []
            for j in range(rows_per_chunk):
                copies.append(pltpu.async_copy(
                    table_hbm.at[idx_v.at[j]],
                    rows_v.at[pl.ds(j * _SC_IW, _SC_IW)], sem))
            for cp in copies:
                cp.wait()
            pltpu.sync_copy(rows_v, out_hbm.at[pl.ds(t0, _SC_CH)])
        return carry

    lax.fori_loop(0, n_chunks, chunk_body, 0)


def _sc_gather2(table, idxa_rows, idxb_rows):
    b_total = idxa_rows.shape[0] * idxa_rows.shape[1]
    d = table.shape[1]
    mesh = plsc.VectorSubcoreMesh(core_axis_name="c", subcore_axis_name="s",
                                  num_cores=_SC_NC, num_subcores=_SC_NS)
    f = pl.kernel(
        _sc_gather2_body,
        out_type=[jax.ShapeDtypeStruct((b_total, d), _F32),
                  jax.ShapeDtypeStruct((b_total, d), _F32)],
        mesh=mesh,
        scratch_types=[
            pltpu.VMEM((_SC_CH // _SC_IW, _SC_IW), jnp.int32),
            pltpu.VMEM((_SC_CH, d), _F32),
            pltpu.SemaphoreType.DMA,
        ],
        compiler_params=pltpu.CompilerParams(use_tc_tiling_on_sc=False),
    )
    return f(table, idxa_rows, idxb_rows)


def kernel(node_attr, edge_type, Dij, Anglesijk, batch_seg, idnb_i, idnb_j,
           id_expand_kj, id_reduce_ji, emb_table, W_rbf_emb, W_node, b_node,
           W_emb, b_emb, int_W_edge, int_W_rbf1, int_W_rbf2, int_W_sbf1,
           int_W_sbf2, int_W_ji, int_b_ji, int_W_kj, int_b_kj, int_W_down,
           int_W_up, int_res_bef_W, int_res_bef_b, int_W_skip, int_b_skip,
           int_res_aft_W, int_res_aft_b, out_W_up, out_dense_W, out_dense_b,
           out_W_final):
    n = node_attr.shape[0]
    e = Dij.shape[0]
    nmol = 512

    dij2 = Dij.reshape(e, 1)
    et2 = edge_type.astype(jnp.int32).reshape(e, 1)
    cang2 = jnp.cos(Anglesijk).reshape(-1, 1)

    w1 = W_emb[0:64]
    w2 = W_emb[64:128]
    wr = W_rbf_emb @ W_emb[128:192]
    we4 = emb_table @ W_emb[192:256]
    bemb = b_emb.reshape(1, -1)

    h = _node_embed(node_attr, W_node, b_node)
    hi, hj = _sc_gather2(h,
                         idnb_i.astype(jnp.int32).reshape(-1, _SC_IW),
                         idnb_j.astype(jnp.int32).reshape(-1, _SC_IW))
    x, rbf8 = _edge_embed(dij2, et2, hi, hj, w1, w2, wr, we4, bemb)

    dt2 = jnp.take(Dij, id_reduce_ji, axis=0).reshape(-1, 1)
    idx_rows = id_expand_kj.astype(jnp.int32).reshape(-1, _SC_IW)
    wsb0 = int_W_sbf1[0] @ int_W_sbf2[0]
    wsb1 = int_W_sbf1[1] @ int_W_sbf2[1]
    sp = _sbf_project(cang2, dt2, wsb0, wsb1)

    wf_pad = [jnp.pad(out_W_final[i], ((0, 0), (0, 128 - out_W_final.shape[2])))
              for i in range(_NB + 1)]

    t0 = jax.ops.segment_sum(x, idnb_i, num_segments=n)
    P = _out_block(t0, out_W_up[0], out_dense_W[0], out_dense_b[0], wf_pad[0])

    for i in range(_NB):
        we4_i = emb_table @ int_W_edge[i]
        wrbf_i = int_W_rbf1[i] @ int_W_rbf2[i]
        xji, xdown = _int_pre(
            x, et2, rbf8, we4_i,
            int_W_ji[i], int_b_ji[i].reshape(1, -1),
            int_W_kj[i], int_b_kj[i].reshape(1, -1),
            wrbf_i, int_W_down[i])
        msg = _sc_gather_mul(xdown, idx_rows, sp[i].reshape(-1))
        seg = jax.ops.segment_sum(msg, id_reduce_ji, num_segments=e)
        x = _int_post(
            xji, seg, x, int_W_up[i],
            int_res_bef_W[i, 0, 0], int_res_bef_b[i, 0, 0].reshape(1, -1),
            int_res_bef_W[i, 0, 1], int_res_bef_b[i, 0, 1].reshape(1, -1),
            int_W_skip[i], int_b_skip[i].reshape(1, -1),
            int_res_aft_W[i, 0, 0], int_res_aft_b[i, 0, 0].reshape(1, -1),
            int_res_aft_W[i, 0, 1], int_res_aft_b[i, 0, 1].reshape(1, -1),
            int_res_aft_W[i, 1, 0], int_res_aft_b[i, 1, 0].reshape(1, -1),
            int_res_aft_W[i, 1, 1], int_res_aft_b[i, 1, 1].reshape(1, -1))
        ti = jax.ops.segment_sum(x, idnb_i, num_segments=n)
        P = P + _out_block(ti, out_W_up[i + 1], out_dense_W[i + 1],
                           out_dense_b[i + 1], wf_pad[i + 1])

    out = jax.ops.segment_sum(P, batch_seg, num_segments=nmol)
    return out[:, :12]


# speedup vs baseline: 1.6679x; 1.1087x over previous
import functools
import numpy as np
import jax
import jax.numpy as jnp
from jax import lax
from jax.experimental import pallas as pl
from jax.experimental.pallas import tpu as pltpu
from jax.experimental.pallas import tpu_sc as plsc

_CUTOFF = 5.0
_NRAD = 6
_NSPH = 7
_NB = 2
_NDO = 3

_BE = 1280
_BT = 9600
_BN = 2000

_F32 = jnp.float32


def _swish(x):
    return x * jax.nn.sigmoid(x)


def _envelope(d):
    a = -28.0
    b = 48.0
    c = -21.0
    d2 = d * d
    d4 = d2 * d2
    d5 = d4 * d
    env = 1.0 / d + a * d5 + b * d5 * d + c * d5 * d2
    return jnp.where(d < 1.0, env, 0.0)


def _rbf_from_d(d):
    k = jax.lax.broadcasted_iota(jnp.int32, (1, _NRAD), 1).astype(_F32)
    freq = (k + 1.0) * np.float32(np.pi)
    return _envelope(d) * jnp.sin(freq * d)


def _iota4(et):
    return jax.lax.broadcasted_iota(jnp.int32, (1, 4), 1)


def _dot(a, b):
    return jnp.dot(a, b, preferred_element_type=_F32)


def _node_body(na, w, b, h_out):
    h_out[...] = _dot(na[...], w[...]) + b[...]


def _node_embed(node_attr, W_node, b_node):
    n = node_attr.shape[0]
    grid = n // _BN
    return pl.pallas_call(
        _node_body,
        grid=(grid,),
        in_specs=[
            pl.BlockSpec((_BN, node_attr.shape[1]), lambda i: (i, 0)),
            pl.BlockSpec(W_node.shape, lambda i: (0, 0)),
            pl.BlockSpec((1, b_node.shape[0]), lambda i: (0, 0)),
        ],
        out_specs=pl.BlockSpec((_BN, 64), lambda i: (i, 0)),
        out_shape=jax.ShapeDtypeStruct((n, 64), _F32),
    )(node_attr, W_node, b_node.reshape(1, -1))


def _edge_body(dij, et, hi, hj, w1, w2, wr, we4, bemb, x_out, rbf_out):
    d = dij[...] / _CUTOFF
    rbf = _rbf_from_d(d)
    oh = (et[...] == _iota4(et)).astype(_F32)
    acc = (_dot(hi[...], w1[...]) + _dot(hj[...], w2[...])
           + _dot(rbf, wr[...]) + _dot(oh, we4[...]) + bemb[...])
    x_out[...] = _swish(acc)
    rbf_out[...] = jnp.concatenate(
        [rbf, jnp.zeros_like(rbf[:, 0:2])], axis=1)


def _edge_embed(dij2, et2, hi, hj, w1, w2, wr, we4, bemb):
    e = dij2.shape[0]
    grid = e // _BE
    wspec = lambda a: pl.BlockSpec(a.shape, lambda i: (0, 0))
    return pl.pallas_call(
        _edge_body,
        grid=(grid,),
        in_specs=[
            pl.BlockSpec((_BE, 1), lambda i: (i, 0)),
            pl.BlockSpec((_BE, 1), lambda i: (i, 0)),
            pl.BlockSpec((_BE, 64), lambda i: (i, 0)),
            pl.BlockSpec((_BE, 64), lambda i: (i, 0)),
            wspec(w1), wspec(w2), wspec(wr), wspec(we4), wspec(bemb),
        ],
        out_specs=[
            pl.BlockSpec((_BE, 64), lambda i: (i, 0)),
            pl.BlockSpec((_BE, 8), lambda i: (i, 0)),
        ],
        out_shape=[
            jax.ShapeDtypeStruct((e, 64), _F32),
            jax.ShapeDtypeStruct((e, 8), _F32),
        ],
    )(dij2, et2, hi, hj, w1, w2, wr, we4, bemb)


def _int_pre_body(x, et, rbf8, we4, wji, bji, wkj, bkj, wrbf, wdown,
                  xji_out, xdown_out):
    oh = (et[...] == _iota4(et)).astype(_F32)
    m = x[...] + _dot(oh, we4[...])
    xji_out[...] = _swish(_dot(m, wji[...]) + bji[...])
    rbf_p = _dot(rbf8[:, 0:_NRAD], wrbf[...])
    xkj = _swish(_dot(m, wkj[...]) + bkj[...]) * rbf_p
    xdown_out[...] = _dot(xkj, wdown[...])


def _int_pre(x, et2, rbf8, we4, wji, bji, wkj, bkj, wrbf, wdown):
    e = x.shape[0]
    grid = e // _BE
    wspec = lambda a: pl.BlockSpec(a.shape, lambda i: (0, 0))
    return pl.pallas_call(
        _int_pre_body,
        grid=(grid,),
        in_specs=[
            pl.BlockSpec((_BE, 64), lambda i: (i, 0)),
            pl.BlockSpec((_BE, 1), lambda i: (i, 0)),
            pl.BlockSpec((_BE, 8), lambda i: (i, 0)),
            wspec(we4), wspec(wji), wspec(bji), wspec(wkj), wspec(bkj),
            wspec(wrbf), wspec(wdown),
        ],
        out_specs=[
            pl.BlockSpec((_BE, 64), lambda i: (i, 0)),
            pl.BlockSpec((_BE, 32), lambda i: (i, 0)),
        ],
        out_shape=[
            jax.ShapeDtypeStruct((e, 64), _F32),
            jax.ShapeDtypeStruct((e, 32), _F32),
        ],
    )(x, et2, rbf8, we4, wji, bji, wkj, bkj, wrbf, wdown)


def _int_post_body(xji, seg, xold, wup, wb0, bb0, wb1, bb1, wskip, bskip,
                   wa00, ba00, wa01, ba01, wa10, ba10, wa11, ba11, x_out):
    hh = xji[...] + _dot(seg[...], wup[...])
    h2 = _swish(_dot(hh, wb0[...]) + bb0[...])
    h2 = _swish(_dot(h2, wb1[...]) + bb1[...])
    hh = hh + h2
    hh = _swish(_dot(hh, wskip[...]) + bskip[...]) + xold[...]
    h2 = _swish(_dot(hh, wa00[...]) + ba00[...])
    h2 = _swish(_dot(h2, wa01[...]) + ba01[...])
    hh = hh + h2
    h2 = _swish(_dot(hh, wa10[...]) + ba10[...])
    h2 = _swish(_dot(h2, wa11[...]) + ba11[...])
    x_out[...] = hh + h2


def _int_post(xji, seg, xold, *ws):
    e = xji.shape[0]
    grid = e // _BE
    wspec = lambda a: pl.BlockSpec(a.shape, lambda i: (0, 0))
    return pl.pallas_call(
        _int_post_body,
        grid=(grid,),
        in_specs=[
            pl.BlockSpec((_BE, 64), lambda i: (i, 0)),
            pl.BlockSpec((_BE, 32), lambda i: (i, 0)),
            pl.BlockSpec((_BE, 64), lambda i: (i, 0)),
        ] + [wspec(w) for w in ws],
        out_specs=pl.BlockSpec((_BE, 64), lambda i: (i, 0)),
        out_shape=jax.ShapeDtypeStruct((e, 64), _F32),
    )(xji, seg, xold, *ws)


_SBF_G = 3


def _sbf_body(cang, dt, wsb0, wsb1, sp0_out, sp1_out):
    ncols = _NSPH * _NRAD
    b0 = _BT // _SBF_G
    k = jax.lax.broadcasted_iota(jnp.int32, (1, _SBF_G * ncols), 1)
    k = k - (k // ncols) * ncols
    lcol = k // _NRAD
    ncol = k - lcol * _NRAD + 1
    zs = np.float32(np.pi) * (ncol.astype(_F32)
                              + 0.5 * lcol.astype(_F32))
    one_row = jnp.zeros((1, ncols), _F32) + 1.0

    def widen(col):
        parts = [col[g * b0:(g + 1) * b0, :] * one_row for g in range(_SBF_G)]
        return jnp.concatenate(parts, axis=1)

    d = widen(dt[...] / _CUTOFF + 1e-9)
    env = _envelope(d)
    x = zs * d
    sx = jnp.sin(x)
    cx = jnp.cos(x)
    j0 = sx / x
    j1 = sx / (x * x) - cx / x
    res = jnp.where(lcol == 0, j0, 0.0)
    res = jnp.where(lcol == 1, j1, res)
    jm2, jm1 = j0, j1
    for ll in range(2, _NSPH):
        jl = (2.0 * ll - 1.0) / x * jm1 - jm2
        res = jnp.where(lcol == ll, jl, res)
        jm2, jm1 = jm1, jl
    c = widen(cang[...])
    p = jnp.where(lcol == 0, 1.0, 0.0)
    p = jnp.where(lcol == 1, c, p)
    pm2 = jnp.zeros_like(x) + 1.0
    pm1 = c
    for ll in range(2, _NSPH):
        pc = ((2.0 * ll - 1.0) * c * pm1 - (ll - 1.0) * pm2) / ll
        p = jnp.where(lcol == ll, pc, p)
        pm2, pm1 = pm1, pc
    sbf = env * res * p
    for g in range(_SBF_G):
        blk = sbf[:, g * ncols:(g + 1) * ncols]
        sp0_out[g * b0:(g + 1) * b0, :] = _dot(blk, wsb0[...])
        sp1_out[g * b0:(g + 1) * b0, :] = _dot(blk, wsb1[...])


def _sbf_project(cang2, dt2, wsb0, wsb1):
    t = cang2.shape[0]
    grid = t // _BT
    wspec = lambda a: pl.BlockSpec(a.shape, lambda i: (0, 0))
    return pl.pallas_call(
        _sbf_body,
        grid=(grid,),
        in_specs=[
            pl.BlockSpec((_BT, 1), lambda i: (i, 0)),
            pl.BlockSpec((_BT, 1), lambda i: (i, 0)),
            wspec(wsb0), wspec(wsb1),
        ],
        out_specs=[
            pl.BlockSpec((_BT, 32), lambda i: (i, 0)),
            pl.BlockSpec((_BT, 32), lambda i: (i, 0)),
        ],
        out_shape=[
            jax.ShapeDtypeStruct((t, 32), _F32),
            jax.ShapeDtypeStruct((t, 32), _F32),
        ],
    )(cang2, dt2, wsb0, wsb1)


def _out_body(t_in, wup, d0, b0, d1, b1, d2, b2, wf, p_out):
    t = _dot(t_in[...], wup[...])
    t = _swish(_dot(t, d0[...]) + b0[...])
    t = _swish(_dot(t, d1[...]) + b1[...])
    t = _swish(_dot(t, d2[...]) + b2[...])
    p_out[...] = _dot(t, wf[...])


def _out_block(t_nodes, wup, dw, db, wf_pad):
    n = t_nodes.shape[0]
    grid = n // _BN
    wspec = lambda a: pl.BlockSpec(a.shape, lambda i: (0, 0))
    args = [t_nodes, wup,
            dw[0], db[0].reshape(1, -1), dw[1], db[1].reshape(1, -1),
            dw[2], db[2].reshape(1, -1), wf_pad]
    return pl.pallas_call(
        _out_body,
        grid=(grid,),
        in_specs=[pl.BlockSpec((_BN, 64), lambda i: (i, 0))]
        + [wspec(a) for a in args[1:]],
        out_specs=pl.BlockSpec((_BN, 128), lambda i: (i, 0)),
        out_shape=jax.ShapeDtypeStruct((n, 128), _F32),
    )(*args)


_SC_NC = 2
_SC_NS = 16
_SC_NW = _SC_NC * _SC_NS
_SC_IW = 125
_SC_CH = 1000


def _sc_gmul_body(table_hbm, idx_hbm, sp_hbm, out_hbm, idx_v, rows_v, sp_v,
                  sem):
    t_total = out_hbm.shape[0]
    n_chunks = t_total // (_SC_NW * _SC_CH)
    wid = lax.axis_index("s") * _SC_NC + lax.axis_index("c")
    base_row = wid * (n_chunks * (_SC_CH // _SC_IW))

    def chunk_body(k, carry):
        row0 = base_row + k * (_SC_CH // _SC_IW)
        t0 = row0 * _SC_IW
        pltpu.sync_copy(idx_hbm.at[pl.ds(row0, _SC_CH // _SC_IW)], idx_v)
        copies = []
        for j in range(_SC_CH // _SC_IW):
            copies.append(pltpu.async_copy(
                table_hbm.at[idx_v.at[j]],
                rows_v.at[pl.ds(j * _SC_IW, _SC_IW)], sem))
        for cp in copies:
            cp.wait()
        pltpu.sync_copy(sp_hbm.at[pl.ds(t0 * 32, _SC_CH * 32)], sp_v)

        def mul_body(r, c2):
            rr = r * 4
            for u in range(4):
                a0 = rows_v[rr + u, pl.ds(0, 16)]
                a1 = rows_v[rr + u, pl.ds(16, 16)]
                b0 = sp_v[pl.ds((rr + u) * 32, 16)]
                b1 = sp_v[pl.ds((rr + u) * 32 + 16, 16)]
                rows_v[rr + u, pl.ds(0, 16)] = a0 * b0
                rows_v[rr + u, pl.ds(16, 16)] = a1 * b1
            return c2

        lax.fori_loop(0, _SC_CH // 4, mul_body, 0)
        pltpu.sync_copy(rows_v, out_hbm.at[pl.ds(t0, _SC_CH)])
        return carry

    lax.fori_loop(0, n_chunks, chunk_body, 0)


def _sc_gather_mul(table, idx_rows, sp_flat):
    t_total = idx_rows.shape[0] * idx_rows.shape[1]
    mesh = plsc.VectorSubcoreMesh(core_axis_name="c", subcore_axis_name="s",
                                  num_cores=_SC_NC, num_subcores=_SC_NS)
    f = pl.kernel(
        _sc_gmul_body,
        out_type=jax.ShapeDtypeStruct((t_total, 32), _F32),
        mesh=mesh,
        scratch_types=[
            pltpu.VMEM((_SC_CH // _SC_IW, _SC_IW), jnp.int32),
            pltpu.VMEM((_SC_CH, 32), _F32),
            pltpu.VMEM((_SC_CH * 32,), _F32),
            pltpu.SemaphoreType.DMA,
        ],
        compiler_params=pltpu.CompilerParams(use_tc_tiling_on_sc=False),
    )
    return f(table, idx_rows, sp_flat)


def _sc_gather2_body(table_hbm, idxa_hbm, idxb_hbm, outa_hbm, outb_hbm,
                     idx_v, rows_v, sem):
    b_total = outa_hbm.shape[0]
    d = outa_hbm.shape[1]
    n_chunks = b_total // (_SC_NW * _SC_CH)
    wid = lax.axis_index("s") * _SC_NC + lax.axis_index("c")
    rows_per_chunk = _SC_CH // _SC_IW
    base_row = wid * (n_chunks * rows_per_chunk)

    def chunk_body(k, carry):
        row0 = base_row + k * rows_per_chunk
        t0 = row0 * _SC_IW
        for idx_hbm, out_hbm in ((idxa_hbm, outa_hbm), (idxb_hbm, outb_hbm)):
            pltpu.sync_copy(idx_hbm.at[pl.ds(row0, rows_per_chunk)], idx_v)
            copies = []
            for j in range(rows_per_chunk):
                copies.append(pltpu.async_copy(
                    table_hbm.at[idx_v.at[j]],
                    rows_v.at[pl.ds(j * _SC_IW, _SC_IW)], sem))
            for cp in copies:
                cp.wait()
            pltpu.sync_copy(rows_v, out_hbm.at[pl.ds(t0, _SC_CH)])
        return carry

    lax.fori_loop(0, n_chunks, chunk_body, 0)


def _sc_gather2(table, idxa_rows, idxb_rows):
    b_total = idxa_rows.shape[0] * idxa_rows.shape[1]
    d = table.shape[1]
    mesh = plsc.VectorSubcoreMesh(core_axis_name="c", subcore_axis_name="s",
                                  num_cores=_SC_NC, num_subcores=_SC_NS)
    f = pl.kernel(
        _sc_gather2_body,
        out_type=[jax.ShapeDtypeStruct((b_total, d), _F32),
                  jax.ShapeDtypeStruct((b_total, d), _F32)],
        mesh=mesh,
        scratch_types=[
            pltpu.VMEM((_SC_CH // _SC_IW, _SC_IW), jnp.int32),
            pltpu.VMEM((_SC_CH, d), _F32),
            pltpu.SemaphoreType.DMA,
        ],
        compiler_params=pltpu.CompilerParams(use_tc_tiling_on_sc=False),
    )
    return f(table, idxa_rows, idxb_rows)


def kernel(node_attr, edge_type, Dij, Anglesijk, batch_seg, idnb_i, idnb_j,
           id_expand_kj, id_reduce_ji, emb_table, W_rbf_emb, W_node, b_node,
           W_emb, b_emb, int_W_edge, int_W_rbf1, int_W_rbf2, int_W_sbf1,
           int_W_sbf2, int_W_ji, int_b_ji, int_W_kj, int_b_kj, int_W_down,
           int_W_up, int_res_bef_W, int_res_bef_b, int_W_skip, int_b_skip,
           int_res_aft_W, int_res_aft_b, out_W_up, out_dense_W, out_dense_b,
           out_W_final):
    n = node_attr.shape[0]
    e = Dij.shape[0]
    nmol = 512

    dij2 = Dij.reshape(e, 1)
    et2 = edge_type.astype(jnp.int32).reshape(e, 1)
    t_len = Anglesijk.shape[0]
    t_pad = ((t_len + _BT - 1) // _BT) * _BT - t_len
    cang2 = jnp.pad(jnp.cos(Anglesijk), (0, t_pad)).reshape(-1, 1)

    w1 = W_emb[0:64]
    w2 = W_emb[64:128]
    wr = W_rbf_emb @ W_emb[128:192]
    we4 = emb_table @ W_emb[192:256]
    bemb = b_emb.reshape(1, -1)

    h = _node_embed(node_attr, W_node, b_node)
    hi, hj = _sc_gather2(h,
                         idnb_i.astype(jnp.int32).reshape(-1, _SC_IW),
                         idnb_j.astype(jnp.int32).reshape(-1, _SC_IW))
    x, rbf8 = _edge_embed(dij2, et2, hi, hj, w1, w2, wr, we4, bemb)

    dt2 = jnp.pad(jnp.take(Dij, id_reduce_ji, axis=0), (0, t_pad),
                  constant_values=_CUTOFF).reshape(-1, 1)
    idx_rows = id_expand_kj.astype(jnp.int32).reshape(-1, _SC_IW)
    wsb0 = int_W_sbf1[0] @ int_W_sbf2[0]
    wsb1 = int_W_sbf1[1] @ int_W_sbf2[1]
    sp = _sbf_project(cang2, dt2, wsb0, wsb1)

    wf_pad = [jnp.pad(out_W_final[i], ((0, 0), (0, 128 - out_W_final.shape[2])))
              for i in range(_NB + 1)]

    t0 = jax.ops.segment_sum(x, idnb_i, num_segments=n)
    P = _out_block(t0, out_W_up[0], out_dense_W[0], out_dense_b[0], wf_pad[0])

    for i in range(_NB):
        we4_i = emb_table @ int_W_edge[i]
        wrbf_i = int_W_rbf1[i] @ int_W_rbf2[i]
        xji, xdown = _int_pre(
            x, et2, rbf8, we4_i,
            int_W_ji[i], int_b_ji[i].reshape(1, -1),
            int_W_kj[i], int_b_kj[i].reshape(1, -1),
            wrbf_i, int_W_down[i])
        msg = _sc_gather_mul(xdown, idx_rows, sp[i].reshape(-1))
        seg = jax.ops.segment_sum(msg, id_reduce_ji, num_segments=e)
        x = _int_post(
            xji, seg, x, int_W_up[i],
            int_res_bef_W[i, 0, 0], int_res_bef_b[i, 0, 0].reshape(1, -1),
            int_res_bef_W[i, 0, 1], int_res_bef_b[i, 0, 1].reshape(1, -1),
            int_W_skip[i], int_b_skip[i].reshape(1, -1),
            int_res_aft_W[i, 0, 0], int_res_aft_b[i, 0, 0].reshape(1, -1),
            int_res_aft_W[i, 0, 1], int_res_aft_b[i, 0, 1].reshape(1, -1),
            int_res_aft_W[i, 1, 0], int_res_aft_b[i, 1, 0].reshape(1, -1),
            int_res_aft_W[i, 1, 1], int_res_aft_b[i, 1, 1].reshape(1, -1))
        ti = jax.ops.segment_sum(x, idnb_i, num_segments=n)
        P = P + _out_block(ti, out_W_up[i + 1], out_dense_W[i + 1],
                           out_dense_b[i + 1], wf_pad[i + 1])

    out = jax.ops.segment_sum(P, batch_seg, num_segments=nmol)
    return out[:, :12]


# speedup vs baseline: 1.6703x; 1.0014x over previous
import functools
import numpy as np
import jax
import jax.numpy as jnp
from jax import lax
from jax.experimental import pallas as pl
from jax.experimental.pallas import tpu as pltpu
from jax.experimental.pallas import tpu_sc as plsc

_CUTOFF = 5.0
_NRAD = 6
_NSPH = 7
_NB = 2
_NDO = 3

_BE = 1280
_BT = 9600
_BN = 2000

_F32 = jnp.float32


def _swish(x):
    return x * jax.nn.sigmoid(x)


def _envelope(d):
    a = -28.0
    b = 48.0
    c = -21.0
    d2 = d * d
    d4 = d2 * d2
    d5 = d4 * d
    env = 1.0 / d + a * d5 + b * d5 * d + c * d5 * d2
    return jnp.where(d < 1.0, env, 0.0)


def _rbf_from_d(d):
    k = jax.lax.broadcasted_iota(jnp.int32, (1, _NRAD), 1).astype(_F32)
    freq = (k + 1.0) * np.float32(np.pi)
    return _envelope(d) * jnp.sin(freq * d)


def _iota4(et):
    return jax.lax.broadcasted_iota(jnp.int32, (1, 4), 1)


def _dot(a, b):
    return jnp.dot(a, b, preferred_element_type=_F32)


def _node_body(na, w, b, h_out):
    h_out[...] = _dot(na[...], w[...]) + b[...]


def _node_embed(node_attr, W_node, b_node):
    n = node_attr.shape[0]
    grid = n // _BN
    return pl.pallas_call(
        _node_body,
        grid=(grid,),
        in_specs=[
            pl.BlockSpec((_BN, node_attr.shape[1]), lambda i: (i, 0)),
            pl.BlockSpec(W_node.shape, lambda i: (0, 0)),
            pl.BlockSpec((1, b_node.shape[0]), lambda i: (0, 0)),
        ],
        out_specs=pl.BlockSpec((_BN, 64), lambda i: (i, 0)),
        out_shape=jax.ShapeDtypeStruct((n, 64), _F32),
    )(node_attr, W_node, b_node.reshape(1, -1))


def _edge_body(dij, et, hi, hj, w1, w2, wr, we4, bemb, x_out, rbf_out):
    d = dij[...] / _CUTOFF
    rbf = _rbf_from_d(d)
    oh = (et[...] == _iota4(et)).astype(_F32)
    acc = (_dot(hi[...], w1[...]) + _dot(hj[...], w2[...])
           + _dot(rbf, wr[...]) + _dot(oh, we4[...]) + bemb[...])
    x_out[...] = _swish(acc)
    rbf_out[...] = jnp.concatenate(
        [rbf, jnp.zeros_like(rbf[:, 0:2])], axis=1)


def _edge_embed(dij2, et2, hi, hj, w1, w2, wr, we4, bemb):
    e = dij2.shape[0]
    grid = e // _BE
    wspec = lambda a: pl.BlockSpec(a.shape, lambda i: (0, 0))
    return pl.pallas_call(
        _edge_body,
        grid=(grid,),
        in_specs=[
            pl.BlockSpec((_BE, 1), lambda i: (i, 0)),
            pl.BlockSpec((_BE, 1), lambda i: (i, 0)),
            pl.BlockSpec((_BE, 64), lambda i: (i, 0)),
            pl.BlockSpec((_BE, 64), lambda i: (i, 0)),
            wspec(w1), wspec(w2), wspec(wr), wspec(we4), wspec(bemb),
        ],
        out_specs=[
            pl.BlockSpec((_BE, 64), lambda i: (i, 0)),
            pl.BlockSpec((_BE, 8), lambda i: (i, 0)),
        ],
        out_shape=[
            jax.ShapeDtypeStruct((e, 64), _F32),
            jax.ShapeDtypeStruct((e, 8), _F32),
        ],
    )(dij2, et2, hi, hj, w1, w2, wr, we4, bemb)


def _int_pre_body(x, et, rbf8, we4, wji, bji, wkj, bkj, wrbf, wdown,
                  xji_out, xdown_out):
    oh = (et[...] == _iota4(et)).astype(_F32)
    m = x[...] + _dot(oh, we4[...])
    xji_out[...] = _swish(_dot(m, wji[...]) + bji[...])
    rbf_p = _dot(rbf8[:, 0:_NRAD], wrbf[...])
    xkj = _swish(_dot(m, wkj[...]) + bkj[...]) * rbf_p
    xdown_out[...] = _dot(xkj, wdown[...])


def _int_pre(x, et2, rbf8, we4, wji, bji, wkj, bkj, wrbf, wdown):
    e = x.shape[0]
    grid = e // _BE
    wspec = lambda a: pl.BlockSpec(a.shape, lambda i: (0, 0))
    return pl.pallas_call(
        _int_pre_body,
        grid=(grid,),
        in_specs=[
            pl.BlockSpec((_BE, 64), lambda i: (i, 0)),
            pl.BlockSpec((_BE, 1), lambda i: (i, 0)),
            pl.BlockSpec((_BE, 8), lambda i: (i, 0)),
            wspec(we4), wspec(wji), wspec(bji), wspec(wkj), wspec(bkj),
            wspec(wrbf), wspec(wdown),
        ],
        out_specs=[
            pl.BlockSpec((_BE, 64), lambda i: (i, 0)),
            pl.BlockSpec((_BE, 32), lambda i: (i, 0)),
        ],
        out_shape=[
            jax.ShapeDtypeStruct((e, 64), _F32),
            jax.ShapeDtypeStruct((e, 32), _F32),
        ],
    )(x, et2, rbf8, we4, wji, bji, wkj, bkj, wrbf, wdown)


def _int_post_body(xji, seg, xold, wup, wb0, bb0, wb1, bb1, wskip, bskip,
                   wa00, ba00, wa01, ba01, wa10, ba10, wa11, ba11, x_out):
    hh = xji[...] + _dot(seg[...], wup[...])
    h2 = _swish(_dot(hh, wb0[...]) + bb0[...])
    h2 = _swish(_dot(h2, wb1[...]) + bb1[...])
    hh = hh + h2
    hh = _swish(_dot(hh, wskip[...]) + bskip[...]) + xold[...]
    h2 = _swish(_dot(hh, wa00[...]) + ba00[...])
    h2 = _swish(_dot(h2, wa01[...]) + ba01[...])
    hh = hh + h2
    h2 = _swish(_dot(hh, wa10[...]) + ba10[...])
    h2 = _swish(_dot(h2, wa11[...]) + ba11[...])
    x_out[...] = hh + h2


def _int_post(xji, seg, xold, *ws):
    e = xji.shape[0]
    grid = e // _BE
    wspec = lambda a: pl.BlockSpec(a.shape, lambda i: (0, 0))
    return pl.pallas_call(
        _int_post_body,
        grid=(grid,),
        in_specs=[
            pl.BlockSpec((_BE, 64), lambda i: (i, 0)),
            pl.BlockSpec((_BE, 32), lambda i: (i, 0)),
            pl.BlockSpec((_BE, 64), lambda i: (i, 0)),
        ] + [wspec(w) for w in ws],
        out_specs=pl.BlockSpec((_BE, 64), lambda i: (i, 0)),
        out_shape=jax.ShapeDtypeStruct((e, 64), _F32),
    )(xji, seg, xold, *ws)


_SBF_G = 3


def _sbf_body(cang, dt, wsb0, wsb1, sp0_out, sp1_out):
    ncols = _NSPH * _NRAD
    b0 = _BT // _SBF_G
    k = jax.lax.broadcasted_iota(jnp.int32, (1, _SBF_G * ncols), 1)
    k = k - (k // ncols) * ncols
    lcol = k // _NRAD
    ncol = k - lcol * _NRAD + 1
    zs = np.float32(np.pi) * (ncol.astype(_F32)
                              + 0.5 * lcol.astype(_F32))
    one_row = jnp.zeros((1, ncols), _F32) + 1.0

    def widen(col):
        parts = [col[g * b0:(g + 1) * b0, :] * one_row for g in range(_SBF_G)]
        return jnp.concatenate(parts, axis=1)

    d = widen(dt[...] / _CUTOFF + 1e-9)
    env = _envelope(d)
    x = zs * d
    sx = jnp.sin(x)
    cx = jnp.cos(x)
    j0 = sx / x
    j1 = sx / (x * x) - cx / x
    res = jnp.where(lcol == 0, j0, 0.0)
    res = jnp.where(lcol == 1, j1, res)
    jm2, jm1 = j0, j1
    for ll in range(2, _NSPH):
        jl = (2.0 * ll - 1.0) / x * jm1 - jm2
        res = jnp.where(lcol == ll, jl, res)
        jm2, jm1 = jm1, jl
    c = widen(cang[...])
    p = jnp.where(lcol == 0, 1.0, 0.0)
    p = jnp.where(lcol == 1, c, p)
    pm2 = jnp.zeros_like(x) + 1.0
    pm1 = c
    for ll in range(2, _NSPH):
        pc = ((2.0 * ll - 1.0) * c * pm1 - (ll - 1.0) * pm2) / ll
        p = jnp.where(lcol == ll, pc, p)
        pm2, pm1 = pm1, pc
    sbf = env * res * p
    for g in range(_SBF_G):
        blk = sbf[:, g * ncols:(g + 1) * ncols]
        sp0_out[g * b0:(g + 1) * b0, :] = _dot(blk, wsb0[...])
        sp1_out[g * b0:(g + 1) * b0, :] = _dot(blk, wsb1[...])


def _sbf_project(cang2, dt2, wsb0, wsb1):
    t = cang2.shape[0]
    grid = t // _BT
    wspec = lambda a: pl.BlockSpec(a.shape, lambda i: (0, 0))
    return pl.pallas_call(
        _sbf_body,
        grid=(grid,),
        in_specs=[
            pl.BlockSpec((_BT, 1), lambda i: (i, 0)),
            pl.BlockSpec((_BT, 1), lambda i: (i, 0)),
            wspec(wsb0), wspec(wsb1),
        ],
        out_specs=[
            pl.BlockSpec((_BT, 32), lambda i: (i, 0)),
            pl.BlockSpec((_BT, 32), lambda i: (i, 0)),
        ],
        out_shape=[
            jax.ShapeDtypeStruct((t, 32), _F32),
            jax.ShapeDtypeStruct((t, 32), _F32),
        ],
    )(cang2, dt2, wsb0, wsb1)


def _out_body(t_in, wup, d0, b0, d1, b1, d2, b2, wf, p_out):
    t = _dot(t_in[...], wup[...])
    t = _swish(_dot(t, d0[...]) + b0[...])
    t = _swish(_dot(t, d1[...]) + b1[...])
    t = _swish(_dot(t, d2[...]) + b2[...])
    p_out[...] = _dot(t, wf[...])


def _out_block(t_nodes, wup, dw, db, wf_pad):
    n = t_nodes.shape[0]
    grid = n // _BN
    wspec = lambda a: pl.BlockSpec(a.shape, lambda i: (0, 0))
    args = [t_nodes, wup,
            dw[0], db[0].reshape(1, -1), dw[1], db[1].reshape(1, -1),
            dw[2], db[2].reshape(1, -1), wf_pad]
    return pl.pallas_call(
        _out_body,
        grid=(grid,),
        in_specs=[pl.BlockSpec((_BN, 64), lambda i: (i, 0))]
        + [wspec(a) for a in args[1:]],
        out_specs=pl.BlockSpec((_BN, 128), lambda i: (i, 0)),
        out_shape=jax.ShapeDtypeStruct((n, 128), _F32),
    )(*args)


_SC_NC = 2
_SC_NS = 16
_SC_NW = _SC_NC * _SC_NS
_SC_IW = 125
_SC_CH = 1000


def _sc_gmul_body(table_hbm, idx_hbm, sp_hbm, out_hbm, idx_v, rows_v, sp_v,
                  sem):
    t_total = out_hbm.shape[0]
    n_chunks = t_total // (_SC_NW * _SC_CH)
    wid = lax.axis_index("s") * _SC_NC + lax.axis_index("c")
    base_row = wid * (n_chunks * (_SC_CH // _SC_IW))

    def chunk_body(k, carry):
        row0 = base_row + k * (_SC_CH // _SC_IW)
        t0 = row0 * _SC_IW
        pltpu.sync_copy(idx_hbm.at[pl.ds(row0, _SC_CH // _SC_IW)], idx_v)
        copies = [pltpu.async_copy(
            sp_hbm.at[pl.ds(t0 * 32, _SC_CH * 32)], sp_v, sem)]
        for j in range(_SC_CH // _SC_IW):
            copies.append(pltpu.async_copy(
                table_hbm.at[idx_v.at[j]],
                rows_v.at[pl.ds(j * _SC_IW, _SC_IW)], sem))
        for cp in copies:
            cp.wait()

        def mul_body(r, c2):
            rr = r * 4
            for u in range(4):
                a0 = rows_v[rr + u, pl.ds(0, 16)]
                a1 = rows_v[rr + u, pl.ds(16, 16)]
                b0 = sp_v[pl.ds((rr + u) * 32, 16)]
                b1 = sp_v[pl.ds((rr + u) * 32 + 16, 16)]
                rows_v[rr + u, pl.ds(0, 16)] = a0 * b0
                rows_v[rr + u, pl.ds(16, 16)] = a1 * b1
            return c2

        lax.fori_loop(0, _SC_CH // 4, mul_body, 0)
        pltpu.sync_copy(rows_v, out_hbm.at[pl.ds(t0, _SC_CH)])
        return carry

    lax.fori_loop(0, n_chunks, chunk_body, 0)


def _sc_gather_mul(table, idx_rows, sp_flat):
    t_total = idx_rows.shape[0] * idx_rows.shape[1]
    mesh = plsc.VectorSubcoreMesh(core_axis_name="c", subcore_axis_name="s",
                                  num_cores=_SC_NC, num_subcores=_SC_NS)
    f = pl.kernel(
        _sc_gmul_body,
        out_type=jax.ShapeDtypeStruct((t_total, 32), _F32),
        mesh=mesh,
        scratch_types=[
            pltpu.VMEM((_SC_CH // _SC_IW, _SC_IW), jnp.int32),
            pltpu.VMEM((_SC_CH, 32), _F32),
            pltpu.VMEM((_SC_CH * 32,), _F32),
            pltpu.SemaphoreType.DMA,
        ],
        compiler_params=pltpu.CompilerParams(use_tc_tiling_on_sc=False),
    )
    return f(table, idx_rows, sp_flat)


_SC_CH2 = 500


def _sc_gather2_body(table_hbm, idxa_hbm, idxb_hbm, outa_hbm, outb_hbm,
                     idx_v, rowsa_v, rowsb_v, sem):
    b_total = outa_hbm.shape[0]
    n_chunks = b_total // (_SC_NW * _SC_CH2)
    wid = lax.axis_index("s") * _SC_NC + lax.axis_index("c")
    rpc = _SC_CH2 // _SC_IW
    base_row = wid * (n_chunks * rpc)

    def chunk_body(k, carry):
        row0 = base_row + k * rpc
        t0 = row0 * _SC_IW
        pltpu.sync_copy(idxa_hbm.at[pl.ds(row0, rpc)], idx_v.at[pl.ds(0, rpc)])
        pltpu.sync_copy(idxb_hbm.at[pl.ds(row0, rpc)],
                        idx_v.at[pl.ds(rpc, rpc)])
        copies = []
        for j in range(rpc):
            copies.append(pltpu.async_copy(
                table_hbm.at[idx_v.at[j]],
                rowsa_v.at[pl.ds(j * _SC_IW, _SC_IW)], sem))
            copies.append(pltpu.async_copy(
                table_hbm.at[idx_v.at[rpc + j]],
                rowsb_v.at[pl.ds(j * _SC_IW, _SC_IW)], sem))
        for cp in copies:
            cp.wait()
        pltpu.sync_copy(rowsa_v, outa_hbm.at[pl.ds(t0, _SC_CH2)])
        pltpu.sync_copy(rowsb_v, outb_hbm.at[pl.ds(t0, _SC_CH2)])
        return carry

    lax.fori_loop(0, n_chunks, chunk_body, 0)


def _sc_gather2(table, idxa_rows, idxb_rows):
    b_total = idxa_rows.shape[0] * idxa_rows.shape[1]
    d = table.shape[1]
    mesh = plsc.VectorSubcoreMesh(core_axis_name="c", subcore_axis_name="s",
                                  num_cores=_SC_NC, num_subcores=_SC_NS)
    f = pl.kernel(
        _sc_gather2_body,
        out_type=[jax.ShapeDtypeStruct((b_total, d), _F32),
                  jax.ShapeDtypeStruct((b_total, d), _F32)],
        mesh=mesh,
        scratch_types=[
            pltpu.VMEM((2 * (_SC_CH2 // _SC_IW), _SC_IW), jnp.int32),
            pltpu.VMEM((_SC_CH2, d), _F32),
            pltpu.VMEM((_SC_CH2, d), _F32),
            pltpu.SemaphoreType.DMA,
        ],
        compiler_params=pltpu.CompilerParams(use_tc_tiling_on_sc=False),
    )
    return f(table, idxa_rows, idxb_rows)


def kernel(node_attr, edge_type, Dij, Anglesijk, batch_seg, idnb_i, idnb_j,
           id_expand_kj, id_reduce_ji, emb_table, W_rbf_emb, W_node, b_node,
           W_emb, b_emb, int_W_edge, int_W_rbf1, int_W_rbf2, int_W_sbf1,
           int_W_sbf2, int_W_ji, int_b_ji, int_W_kj, int_b_kj, int_W_down,
           int_W_up, int_res_bef_W, int_res_bef_b, int_W_skip, int_b_skip,
           int_res_aft_W, int_res_aft_b, out_W_up, out_dense_W, out_dense_b,
           out_W_final):
    n = node_attr.shape[0]
    e = Dij.shape[0]
    nmol = 512

    dij2 = Dij.reshape(e, 1)
    et2 = edge_type.astype(jnp.int32).reshape(e, 1)
    t_len = Anglesijk.shape[0]
    t_pad = ((t_len + _BT - 1) // _BT) * _BT - t_len
    cang2 = jnp.pad(jnp.cos(Anglesijk), (0, t_pad)).reshape(-1, 1)

    w1 = W_emb[0:64]
    w2 = W_emb[64:128]
    wr = W_rbf_emb @ W_emb[128:192]
    we4 = emb_table @ W_emb[192:256]
    bemb = b_emb.reshape(1, -1)

    h = _node_embed(node_attr, W_node, b_node)
    hi, hj = _sc_gather2(h,
                         idnb_i.astype(jnp.int32).reshape(-1, _SC_IW),
                         idnb_j.astype(jnp.int32).reshape(-1, _SC_IW))
    x, rbf8 = _edge_embed(dij2, et2, hi, hj, w1, w2, wr, we4, bemb)

    dt2 = jnp.pad(jnp.take(Dij, id_reduce_ji, axis=0), (0, t_pad),
                  constant_values=_CUTOFF).reshape(-1, 1)
    idx_rows = id_expand_kj.astype(jnp.int32).reshape(-1, _SC_IW)
    wsb0 = int_W_sbf1[0] @ int_W_sbf2[0]
    wsb1 = int_W_sbf1[1] @ int_W_sbf2[1]
    sp = _sbf_project(cang2, dt2, wsb0, wsb1)

    wf_pad = [jnp.pad(out_W_final[i], ((0, 0), (0, 128 - out_W_final.shape[2])))
              for i in range(_NB + 1)]

    t0 = jax.ops.segment_sum(x, idnb_i, num_segments=n)
    P = _out_block(t0, out_W_up[0], out_dense_W[0], out_dense_b[0], wf_pad[0])

    for i in range(_NB):
        we4_i = emb_table @ int_W_edge[i]
        wrbf_i = int_W_rbf1[i] @ int_W_rbf2[i]
        xji, xdown = _int_pre(
            x, et2, rbf8, we4_i,
            int_W_ji[i], int_b_ji[i].reshape(1, -1),
            int_W_kj[i], int_b_kj[i].reshape(1, -1),
            wrbf_i, int_W_down[i])
        msg = _sc_gather_mul(xdown, idx_rows, sp[i].reshape(-1))
        seg = jax.ops.segment_sum(msg, id_reduce_ji, num_segments=e)
        x = _int_post(
            xji, seg, x, int_W_up[i],
            int_res_bef_W[i, 0, 0], int_res_bef_b[i, 0, 0].reshape(1, -1),
            int_res_bef_W[i, 0, 1], int_res_bef_b[i, 0, 1].reshape(1, -1),
            int_W_skip[i], int_b_skip[i].reshape(1, -1),
            int_res_aft_W[i, 0, 0], int_res_aft_b[i, 0, 0].reshape(1, -1),
            int_res_aft_W[i, 0, 1], int_res_aft_b[i, 0, 1].reshape(1, -1),
            int_res_aft_W[i, 1, 0], int_res_aft_b[i, 1, 0].reshape(1, -1),
            int_res_aft_W[i, 1, 1], int_res_aft_b[i, 1, 1].reshape(1, -1))
        ti = jax.ops.segment_sum(x, idnb_i, num_segments=n)
        P = P + _out_block(ti, out_W_up[i + 1], out_dense_W[i + 1],
                           out_dense_b[i + 1], wf_pad[i + 1])

    out = jax.ops.segment_sum(P, batch_seg, num_segments=nmol)
    return out[:, :12]


# speedup vs baseline: 1.7334x; 1.0378x over previous
import functools
import numpy as np
import jax
import jax.numpy as jnp
from jax import lax
from jax.experimental import pallas as pl
from jax.experimental.pallas import tpu as pltpu
from jax.experimental.pallas import tpu_sc as plsc

_CUTOFF = 5.0
_NRAD = 6
_NSPH = 7
_NB = 2
_NDO = 3

_BE = 1280
_BT = 9600
_BN = 2000

_F32 = jnp.float32


def _swish(x):
    return x * jax.nn.sigmoid(x)


def _envelope(d):
    a = -28.0
    b = 48.0
    c = -21.0
    d2 = d * d
    d4 = d2 * d2
    d5 = d4 * d
    env = 1.0 / d + a * d5 + b * d5 * d + c * d5 * d2
    return jnp.where(d < 1.0, env, 0.0)


def _rbf_from_d(d):
    k = jax.lax.broadcasted_iota(jnp.int32, (1, _NRAD), 1).astype(_F32)
    freq = (k + 1.0) * np.float32(np.pi)
    return _envelope(d) * jnp.sin(freq * d)


def _iota4(et):
    return jax.lax.broadcasted_iota(jnp.int32, (1, 4), 1)


def _dot(a, b):
    return jnp.dot(a, b, preferred_element_type=_F32)


def _node_body(na, w, b, h_out):
    h_out[...] = _dot(na[...], w[...]) + b[...]


def _node_embed(node_attr, W_node, b_node):
    n = node_attr.shape[0]
    grid = n // _BN
    return pl.pallas_call(
        _node_body,
        grid=(grid,),
        in_specs=[
            pl.BlockSpec((_BN, node_attr.shape[1]), lambda i: (i, 0)),
            pl.BlockSpec(W_node.shape, lambda i: (0, 0)),
            pl.BlockSpec((1, b_node.shape[0]), lambda i: (0, 0)),
        ],
        out_specs=pl.BlockSpec((_BN, 64), lambda i: (i, 0)),
        out_shape=jax.ShapeDtypeStruct((n, 64), _F32),
    )(node_attr, W_node, b_node.reshape(1, -1))


def _edge_body(de, hi, hj, w1, w2, wr, we4, bemb, x_out, rbf_out):
    d = de[:, 0:1] / _CUTOFF
    rbf = _rbf_from_d(d)
    oh = (de[:, 1:2] == _iota4(de).astype(_F32)).astype(_F32)
    acc = (_dot(hi[...], w1[...]) + _dot(hj[...], w2[...])
           + _dot(rbf, wr[...]) + _dot(oh, we4[...]) + bemb[...])
    x_out[...] = _swish(acc)
    rbf_out[...] = jnp.concatenate(
        [rbf, jnp.zeros_like(rbf[:, 0:2])], axis=1)


def _edge_embed(de2, hi, hj, w1, w2, wr, we4, bemb):
    e = de2.shape[0]
    grid = e // _BE
    wspec = lambda a: pl.BlockSpec(a.shape, lambda i: (0, 0))
    return pl.pallas_call(
        _edge_body,
        grid=(grid,),
        in_specs=[
            pl.BlockSpec((_BE, 2), lambda i: (i, 0)),
            pl.BlockSpec((_BE, 64), lambda i: (i, 0)),
            pl.BlockSpec((_BE, 64), lambda i: (i, 0)),
            wspec(w1), wspec(w2), wspec(wr), wspec(we4), wspec(bemb),
        ],
        out_specs=[
            pl.BlockSpec((_BE, 64), lambda i: (i, 0)),
            pl.BlockSpec((_BE, 8), lambda i: (i, 0)),
        ],
        out_shape=[
            jax.ShapeDtypeStruct((e, 64), _F32),
            jax.ShapeDtypeStruct((e, 8), _F32),
        ],
    )(de2, hi, hj, w1, w2, wr, we4, bemb)


def _int_pre_body(x, de, rbf8, we4, wji, bji, wkj, bkj, wrbf, wdown,
                  xji_out, xdown_out):
    oh = (de[:, 1:2] == _iota4(de).astype(_F32)).astype(_F32)
    m = x[...] + _dot(oh, we4[...])
    xji_out[...] = _swish(_dot(m, wji[...]) + bji[...])
    rbf_p = _dot(rbf8[:, 0:_NRAD], wrbf[...])
    xkj = _swish(_dot(m, wkj[...]) + bkj[...]) * rbf_p
    xdown_out[...] = _dot(xkj, wdown[...])


def _int_pre(x, de2, rbf8, we4, wji, bji, wkj, bkj, wrbf, wdown):
    e = x.shape[0]
    grid = e // _BE
    wspec = lambda a: pl.BlockSpec(a.shape, lambda i: (0, 0))
    return pl.pallas_call(
        _int_pre_body,
        grid=(grid,),
        in_specs=[
            pl.BlockSpec((_BE, 64), lambda i: (i, 0)),
            pl.BlockSpec((_BE, 2), lambda i: (i, 0)),
            pl.BlockSpec((_BE, 8), lambda i: (i, 0)),
            wspec(we4), wspec(wji), wspec(bji), wspec(wkj), wspec(bkj),
            wspec(wrbf), wspec(wdown),
        ],
        out_specs=[
            pl.BlockSpec((_BE, 64), lambda i: (i, 0)),
            pl.BlockSpec((_BE, 32), lambda i: (i, 0)),
        ],
        out_shape=[
            jax.ShapeDtypeStruct((e, 64), _F32),
            jax.ShapeDtypeStruct((e, 32), _F32),
        ],
    )(x, de2, rbf8, we4, wji, bji, wkj, bkj, wrbf, wdown)


def _int_post_body(xji, seg, xold, wup, wb0, bb0, wb1, bb1, wskip, bskip,
                   wa00, ba00, wa01, ba01, wa10, ba10, wa11, ba11, x_out):
    hh = xji[...] + _dot(seg[...], wup[...])
    h2 = _swish(_dot(hh, wb0[...]) + bb0[...])
    h2 = _swish(_dot(h2, wb1[...]) + bb1[...])
    hh = hh + h2
    hh = _swish(_dot(hh, wskip[...]) + bskip[...]) + xold[...]
    h2 = _swish(_dot(hh, wa00[...]) + ba00[...])
    h2 = _swish(_dot(h2, wa01[...]) + ba01[...])
    hh = hh + h2
    h2 = _swish(_dot(hh, wa10[...]) + ba10[...])
    h2 = _swish(_dot(h2, wa11[...]) + ba11[...])
    x_out[...] = hh + h2


def _int_post(xji, seg, xold, *ws):
    e = xji.shape[0]
    grid = e // _BE
    wspec = lambda a: pl.BlockSpec(a.shape, lambda i: (0, 0))
    return pl.pallas_call(
        _int_post_body,
        grid=(grid,),
        in_specs=[
            pl.BlockSpec((_BE, 64), lambda i: (i, 0)),
            pl.BlockSpec((_BE, 32), lambda i: (i, 0)),
            pl.BlockSpec((_BE, 64), lambda i: (i, 0)),
        ] + [wspec(w) for w in ws],
        out_specs=pl.BlockSpec((_BE, 64), lambda i: (i, 0)),
        out_shape=jax.ShapeDtypeStruct((e, 64), _F32),
    )(xji, seg, xold, *ws)


_SBF_G = 3


def _sbf_body(ct, wsb0, wsb1, sp0_out, sp1_out):
    ncols = _NSPH * _NRAD
    b0 = _BT // _SBF_G
    k = jax.lax.broadcasted_iota(jnp.int32, (1, _SBF_G * ncols), 1)
    k = k - (k // ncols) * ncols
    lcol = k // _NRAD
    ncol = k - lcol * _NRAD + 1
    zs = np.float32(np.pi) * (ncol.astype(_F32)
                              + 0.5 * lcol.astype(_F32))
    one_row = jnp.zeros((1, ncols), _F32) + 1.0

    def widen(col):
        parts = [col[g * b0:(g + 1) * b0, :] * one_row for g in range(_SBF_G)]
        return jnp.concatenate(parts, axis=1)

    d = widen(ct[:, 0:1] / _CUTOFF + 1e-9)
    env = _envelope(d)
    x = zs * d
    sx = jnp.sin(x)
    cx = jnp.cos(x)
    j0 = sx / x
    j1 = sx / (x * x) - cx / x
    res = jnp.where(lcol == 0, j0, 0.0)
    res = jnp.where(lcol == 1, j1, res)
    jm2, jm1 = j0, j1
    for ll in range(2, _NSPH):
        jl = (2.0 * ll - 1.0) / x * jm1 - jm2
        res = jnp.where(lcol == ll, jl, res)
        jm2, jm1 = jm1, jl
    c = widen(ct[:, 1:2])
    p = jnp.where(lcol == 0, 1.0, 0.0)
    p = jnp.where(lcol == 1, c, p)
    pm2 = jnp.zeros_like(x) + 1.0
    pm1 = c
    for ll in range(2, _NSPH):
        pc = ((2.0 * ll - 1.0) * c * pm1 - (ll - 1.0) * pm2) / ll
        p = jnp.where(lcol == ll, pc, p)
        pm2, pm1 = pm1, pc
    sbf = env * res * p
    for g in range(_SBF_G):
        blk = sbf[:, g * ncols:(g + 1) * ncols]
        sp0_out[g * b0:(g + 1) * b0, :] = _dot(blk, wsb0[...])
        sp1_out[g * b0:(g + 1) * b0, :] = _dot(blk, wsb1[...])


def _sbf_project(ct2, wsb0, wsb1):
    t = ct2.shape[0]
    grid = t // _BT
    wspec = lambda a: pl.BlockSpec(a.shape, lambda i: (0, 0))
    return pl.pallas_call(
        _sbf_body,
        grid=(grid,),
        in_specs=[
            pl.BlockSpec((_BT, 2), lambda i: (i, 0)),
            wspec(wsb0), wspec(wsb1),
        ],
        out_specs=[
            pl.BlockSpec((_BT, 32), lambda i: (i, 0)),
            pl.BlockSpec((_BT, 32), lambda i: (i, 0)),
        ],
        out_shape=[
            jax.ShapeDtypeStruct((t, 32), _F32),
            jax.ShapeDtypeStruct((t, 32), _F32),
        ],
    )(ct2, wsb0, wsb1)


def _out_body(t_in, wup, d0, b0, d1, b1, d2, b2, wf, p_out):
    t = _dot(t_in[...], wup[...])
    t = _swish(_dot(t, d0[...]) + b0[...])
    t = _swish(_dot(t, d1[...]) + b1[...])
    t = _swish(_dot(t, d2[...]) + b2[...])
    p_out[...] = _dot(t, wf[...])


def _out_block(t_nodes, wup, dw, db, wf_pad):
    n = t_nodes.shape[0]
    grid = n // _BN
    wspec = lambda a: pl.BlockSpec(a.shape, lambda i: (0, 0))
    args = [t_nodes, wup,
            dw[0], db[0].reshape(1, -1), dw[1], db[1].reshape(1, -1),
            dw[2], db[2].reshape(1, -1), wf_pad]
    return pl.pallas_call(
        _out_body,
        grid=(grid,),
        in_specs=[pl.BlockSpec((_BN, 64), lambda i: (i, 0))]
        + [wspec(a) for a in args[1:]],
        out_specs=pl.BlockSpec((_BN, 128), lambda i: (i, 0)),
        out_shape=jax.ShapeDtypeStruct((n, 128), _F32),
    )(*args)


_SC_NC = 2
_SC_NS = 16
_SC_NW = _SC_NC * _SC_NS
_SC_IW = 125
_SC_CH = 1000


def _sc_gmul_body(table_hbm, idx_hbm, sp_hbm, out_hbm, idx_v, rows_v, sp_v,
                  sem):
    t_total = out_hbm.shape[0]
    n_chunks = t_total // (_SC_NW * _SC_CH)
    wid = lax.axis_index("s") * _SC_NC + lax.axis_index("c")
    base_row = wid * (n_chunks * (_SC_CH // _SC_IW))

    def chunk_body(k, carry):
        row0 = base_row + k * (_SC_CH // _SC_IW)
        t0 = row0 * _SC_IW
        pltpu.sync_copy(idx_hbm.at[pl.ds(row0, _SC_CH // _SC_IW)], idx_v)
        copies = [pltpu.async_copy(
            sp_hbm.at[pl.ds(t0 * 32, _SC_CH * 32)], sp_v, sem)]
        for j in range(_SC_CH // _SC_IW):
            copies.append(pltpu.async_copy(
                table_hbm.at[idx_v.at[j]],
                rows_v.at[pl.ds(j * _SC_IW, _SC_IW)], sem))
        for cp in copies:
            cp.wait()

        def mul_body(r, c2):
            rr = r * 4
            for u in range(4):
                a0 = rows_v[rr + u, pl.ds(0, 16)]
                a1 = rows_v[rr + u, pl.ds(16, 16)]
                b0 = sp_v[pl.ds((rr + u) * 32, 16)]
                b1 = sp_v[pl.ds((rr + u) * 32 + 16, 16)]
                rows_v[rr + u, pl.ds(0, 16)] = a0 * b0
                rows_v[rr + u, pl.ds(16, 16)] = a1 * b1
            return c2

        lax.fori_loop(0, _SC_CH // 4, mul_body, 0)
        pltpu.sync_copy(rows_v, out_hbm.at[pl.ds(t0, _SC_CH)])
        return carry

    lax.fori_loop(0, n_chunks, chunk_body, 0)


def _sc_gather_mul(table, idx_rows, sp_flat):
    t_total = idx_rows.shape[0] * idx_rows.shape[1]
    mesh = plsc.VectorSubcoreMesh(core_axis_name="c", subcore_axis_name="s",
                                  num_cores=_SC_NC, num_subcores=_SC_NS)
    f = pl.kernel(
        _sc_gmul_body,
        out_type=jax.ShapeDtypeStruct((t_total, 32), _F32),
        mesh=mesh,
        scratch_types=[
            pltpu.VMEM((_SC_CH // _SC_IW, _SC_IW), jnp.int32),
            pltpu.VMEM((_SC_CH, 32), _F32),
            pltpu.VMEM((_SC_CH * 32,), _F32),
            pltpu.SemaphoreType.DMA,
        ],
        compiler_params=pltpu.CompilerParams(use_tc_tiling_on_sc=False),
    )
    return f(table, idx_rows, sp_flat)


_SC_CH2 = 500


def _sc_gather2_body(table_hbm, idxa_hbm, idxb_hbm, outa_hbm, outb_hbm,
                     idx_v, rowsa_v, rowsb_v, sem):
    b_total = outa_hbm.shape[0]
    n_chunks = b_total // (_SC_NW * _SC_CH2)
    wid = lax.axis_index("s") * _SC_NC + lax.axis_index("c")
    rpc = _SC_CH2 // _SC_IW
    base_row = wid * (n_chunks * rpc)

    def chunk_body(k, carry):
        row0 = base_row + k * rpc
        t0 = row0 * _SC_IW
        pltpu.sync_copy(idxa_hbm.at[pl.ds(row0, rpc)], idx_v.at[pl.ds(0, rpc)])
        pltpu.sync_copy(idxb_hbm.at[pl.ds(row0, rpc)],
                        idx_v.at[pl.ds(rpc, rpc)])
        copies = []
        for j in range(rpc):
            copies.append(pltpu.async_copy(
                table_hbm.at[idx_v.at[j]],
                rowsa_v.at[pl.ds(j * _SC_IW, _SC_IW)], sem))
            copies.append(pltpu.async_copy(
                table_hbm.at[idx_v.at[rpc + j]],
                rowsb_v.at[pl.ds(j * _SC_IW, _SC_IW)], sem))
        for cp in copies:
            cp.wait()
        pltpu.sync_copy(rowsa_v, outa_hbm.at[pl.ds(t0, _SC_CH2)])
        pltpu.sync_copy(rowsb_v, outb_hbm.at[pl.ds(t0, _SC_CH2)])
        return carry

    lax.fori_loop(0, n_chunks, chunk_body, 0)


def _sc_gather2(table, idxa_rows, idxb_rows):
    b_total = idxa_rows.shape[0] * idxa_rows.shape[1]
    d = table.shape[1]
    mesh = plsc.VectorSubcoreMesh(core_axis_name="c", subcore_axis_name="s",
                                  num_cores=_SC_NC, num_subcores=_SC_NS)
    f = pl.kernel(
        _sc_gather2_body,
        out_type=[jax.ShapeDtypeStruct((b_total, d), _F32),
                  jax.ShapeDtypeStruct((b_total, d), _F32)],
        mesh=mesh,
        scratch_types=[
            pltpu.VMEM((2 * (_SC_CH2 // _SC_IW), _SC_IW), jnp.int32),
            pltpu.VMEM((_SC_CH2, d), _F32),
            pltpu.VMEM((_SC_CH2, d), _F32),
            pltpu.SemaphoreType.DMA,
        ],
        compiler_params=pltpu.CompilerParams(use_tc_tiling_on_sc=False),
    )
    return f(table, idxa_rows, idxb_rows)


def kernel(node_attr, edge_type, Dij, Anglesijk, batch_seg, idnb_i, idnb_j,
           id_expand_kj, id_reduce_ji, emb_table, W_rbf_emb, W_node, b_node,
           W_emb, b_emb, int_W_edge, int_W_rbf1, int_W_rbf2, int_W_sbf1,
           int_W_sbf2, int_W_ji, int_b_ji, int_W_kj, int_b_kj, int_W_down,
           int_W_up, int_res_bef_W, int_res_bef_b, int_W_skip, int_b_skip,
           int_res_aft_W, int_res_aft_b, out_W_up, out_dense_W, out_dense_b,
           out_W_final):
    n = node_attr.shape[0]
    e = Dij.shape[0]
    nmol = 512

    de2 = jnp.concatenate(
        [Dij.reshape(e, 1), edge_type.astype(_F32).reshape(e, 1)], axis=1)
    t_len = Anglesijk.shape[0]
    t_pad = ((t_len + _BT - 1) // _BT) * _BT - t_len
    cang_p = jnp.pad(jnp.cos(Anglesijk), (0, t_pad))

    w1 = W_emb[0:64]
    w2 = W_emb[64:128]
    wr = W_rbf_emb @ W_emb[128:192]
    we4 = emb_table @ W_emb[192:256]
    bemb = b_emb.reshape(1, -1)

    h = _node_embed(node_attr, W_node, b_node)
    hi, hj = _sc_gather2(h,
                         idnb_i.astype(jnp.int32).reshape(-1, _SC_IW),
                         idnb_j.astype(jnp.int32).reshape(-1, _SC_IW))
    x, rbf8 = _edge_embed(de2, hi, hj, w1, w2, wr, we4, bemb)

    dt_p = jnp.pad(jnp.take(Dij, id_reduce_ji, axis=0), (0, t_pad),
                   constant_values=_CUTOFF)
    ct2 = jnp.concatenate(
        [dt_p.reshape(-1, 1), cang_p.reshape(-1, 1)], axis=1)
    idx_rows = id_expand_kj.astype(jnp.int32).reshape(-1, _SC_IW)
    wsb0 = int_W_sbf1[0] @ int_W_sbf2[0]
    wsb1 = int_W_sbf1[1] @ int_W_sbf2[1]
    sp = _sbf_project(ct2, wsb0, wsb1)

    wf_pad = [jnp.pad(out_W_final[i], ((0, 0), (0, 128 - out_W_final.shape[2])))
              for i in range(_NB + 1)]

    t0 = jax.ops.segment_sum(x, idnb_i, num_segments=n)
    P = _out_block(t0, out_W_up[0], out_dense_W[0], out_dense_b[0], wf_pad[0])

    for i in range(_NB):
        we4_i = emb_table @ int_W_edge[i]
        wrbf_i = int_W_rbf1[i] @ int_W_rbf2[i]
        xji, xdown = _int_pre(
            x, de2, rbf8, we4_i,
            int_W_ji[i], int_b_ji[i].reshape(1, -1),
            int_W_kj[i], int_b_kj[i].reshape(1, -1),
            wrbf_i, int_W_down[i])
        msg = _sc_gather_mul(xdown, idx_rows, sp[i].reshape(-1))
        seg = jax.ops.segment_sum(msg, id_reduce_ji, num_segments=e)
        x = _int_post(
            xji, seg, x, int_W_up[i],
            int_res_bef_W[i, 0, 0], int_res_bef_b[i, 0, 0].reshape(1, -1),
            int_res_bef_W[i, 0, 1], int_res_bef_b[i, 0, 1].reshape(1, -1),
            int_W_skip[i], int_b_skip[i].reshape(1, -1),
            int_res_aft_W[i, 0, 0], int_res_aft_b[i, 0, 0].reshape(1, -1),
            int_res_aft_W[i, 0, 1], int_res_aft_b[i, 0, 1].reshape(1, -1),
            int_res_aft_W[i, 1, 0], int_res_aft_b[i, 1, 0].reshape(1, -1),
            int_res_aft_W[i, 1, 1], int_res_aft_b[i, 1, 1].reshape(1, -1))
        ti = jax.ops.segment_sum(x, idnb_i, num_segments=n)
        P = P + _out_block(ti, out_W_up[i + 1], out_dense_W[i + 1],
                           out_dense_b[i + 1], wf_pad[i + 1])

    out = jax.ops.segment_sum(P, batch_seg, num_segments=nmol)
    return out[:, :12]


# speedup vs baseline: 1.9094x; 1.1015x over previous
import functools
import numpy as np
import jax
import jax.numpy as jnp
from jax import lax
from jax.experimental import pallas as pl
from jax.experimental.pallas import tpu as pltpu
from jax.experimental.pallas import tpu_sc as plsc

_CUTOFF = 5.0
_NRAD = 6
_NSPH = 7
_NB = 2
_NDO = 3

_BE = 1280
_BT = 9600
_BN = 2000

_F32 = jnp.float32


def _swish(x):
    return x * jax.nn.sigmoid(x)


def _envelope(d):
    a = -28.0
    b = 48.0
    c = -21.0
    d2 = d * d
    d4 = d2 * d2
    d5 = d4 * d
    env = 1.0 / d + a * d5 + b * d5 * d + c * d5 * d2
    return jnp.where(d < 1.0, env, 0.0)


def _rbf_from_d(d):
    k = jax.lax.broadcasted_iota(jnp.int32, (1, _NRAD), 1).astype(_F32)
    freq = (k + 1.0) * np.float32(np.pi)
    return _envelope(d) * jnp.sin(freq * d)


def _iota4(et):
    return jax.lax.broadcasted_iota(jnp.int32, (1, 4), 1)


def _dot(a, b):
    return jnp.dot(a, b, preferred_element_type=_F32)


def _node_body(na, w, b, h_out):
    h_out[...] = _dot(na[...], w[...]) + b[...]


def _node_embed(node_attr, W_node, b_node):
    n = node_attr.shape[0]
    grid = n // _BN
    return pl.pallas_call(
        _node_body,
        grid=(grid,),
        in_specs=[
            pl.BlockSpec((_BN, node_attr.shape[1]), lambda i: (i, 0)),
            pl.BlockSpec(W_node.shape, lambda i: (0, 0)),
            pl.BlockSpec((1, b_node.shape[0]), lambda i: (0, 0)),
        ],
        out_specs=pl.BlockSpec((_BN, 64), lambda i: (i, 0)),
        out_shape=jax.ShapeDtypeStruct((n, 64), _F32),
    )(node_attr, W_node, b_node.reshape(1, -1))


def _edge_body(de, hi, hj, w1, w2, wr, we4, bemb, x_out, rbf_out):
    d = de[:, 0:1] / _CUTOFF
    rbf = _rbf_from_d(d)
    oh = (de[:, 1:2] == _iota4(de).astype(_F32)).astype(_F32)
    acc = (_dot(hi[...], w1[...]) + _dot(hj[...], w2[...])
           + _dot(rbf, wr[...]) + _dot(oh, we4[...]) + bemb[...])
    x_out[...] = _swish(acc)
    rbf_out[...] = jnp.concatenate(
        [rbf, jnp.zeros_like(rbf[:, 0:2])], axis=1)


def _edge_embed(de2, hi, hj, w1, w2, wr, we4, bemb):
    e = de2.shape[0]
    grid = e // _BE
    wspec = lambda a: pl.BlockSpec(a.shape, lambda i: (0, 0))
    return pl.pallas_call(
        _edge_body,
        grid=(grid,),
        in_specs=[
            pl.BlockSpec((_BE, 2), lambda i: (i, 0)),
            pl.BlockSpec((_BE, 64), lambda i: (i, 0)),
            pl.BlockSpec((_BE, 64), lambda i: (i, 0)),
            wspec(w1), wspec(w2), wspec(wr), wspec(we4), wspec(bemb),
        ],
        out_specs=[
            pl.BlockSpec((_BE, 64), lambda i: (i, 0)),
            pl.BlockSpec((_BE, 8), lambda i: (i, 0)),
        ],
        out_shape=[
            jax.ShapeDtypeStruct((e, 64), _F32),
            jax.ShapeDtypeStruct((e, 8), _F32),
        ],
    )(de2, hi, hj, w1, w2, wr, we4, bemb)


def _int_pre_body(x, de, rbf8, we4, wji, bji, wkj, bkj, wrbf, wdown,
                  xji_out, xdown_out):
    oh = (de[:, 1:2] == _iota4(de).astype(_F32)).astype(_F32)
    m = x[...] + _dot(oh, we4[...])
    xji_out[...] = _swish(_dot(m, wji[...]) + bji[...])
    rbf_p = _dot(rbf8[:, 0:_NRAD], wrbf[...])
    xkj = _swish(_dot(m, wkj[...]) + bkj[...]) * rbf_p
    xdown_out[...] = _dot(xkj, wdown[...])


def _int_pre(x, de2, rbf8, we4, wji, bji, wkj, bkj, wrbf, wdown):
    e = x.shape[0]
    grid = e // _BE
    wspec = lambda a: pl.BlockSpec(a.shape, lambda i: (0, 0))
    return pl.pallas_call(
        _int_pre_body,
        grid=(grid,),
        in_specs=[
            pl.BlockSpec((_BE, 64), lambda i: (i, 0)),
            pl.BlockSpec((_BE, 2), lambda i: (i, 0)),
            pl.BlockSpec((_BE, 8), lambda i: (i, 0)),
            wspec(we4), wspec(wji), wspec(bji), wspec(wkj), wspec(bkj),
            wspec(wrbf), wspec(wdown),
        ],
        out_specs=[
            pl.BlockSpec((_BE, 64), lambda i: (i, 0)),
            pl.BlockSpec((_BE, 32), lambda i: (i, 0)),
        ],
        out_shape=[
            jax.ShapeDtypeStruct((e, 64), _F32),
            jax.ShapeDtypeStruct((e, 32), _F32),
        ],
    )(x, de2, rbf8, we4, wji, bji, wkj, bkj, wrbf, wdown)


def _int_post_body(xji, seg, xold, wup, wb0, bb0, wb1, bb1, wskip, bskip,
                   wa00, ba00, wa01, ba01, wa10, ba10, wa11, ba11, x_out):
    hh = xji[...] + _dot(seg[...], wup[...])
    h2 = _swish(_dot(hh, wb0[...]) + bb0[...])
    h2 = _swish(_dot(h2, wb1[...]) + bb1[...])
    hh = hh + h2
    hh = _swish(_dot(hh, wskip[...]) + bskip[...]) + xold[...]
    h2 = _swish(_dot(hh, wa00[...]) + ba00[...])
    h2 = _swish(_dot(h2, wa01[...]) + ba01[...])
    hh = hh + h2
    h2 = _swish(_dot(hh, wa10[...]) + ba10[...])
    h2 = _swish(_dot(h2, wa11[...]) + ba11[...])
    x_out[...] = hh + h2


def _int_post(xji, seg, xold, *ws):
    e = xji.shape[0]
    grid = e // _BE
    wspec = lambda a: pl.BlockSpec(a.shape, lambda i: (0, 0))
    return pl.pallas_call(
        _int_post_body,
        grid=(grid,),
        in_specs=[
            pl.BlockSpec((_BE, 64), lambda i: (i, 0)),
            pl.BlockSpec((_BE, 32), lambda i: (i, 0)),
            pl.BlockSpec((_BE, 64), lambda i: (i, 0)),
        ] + [wspec(w) for w in ws],
        out_specs=pl.BlockSpec((_BE, 64), lambda i: (i, 0)),
        out_shape=jax.ShapeDtypeStruct((e, 64), _F32),
    )(xji, seg, xold, *ws)


_SBF_G = 3


def _sbf_body(ct, wsb0, wsb1, sp0_out, sp1_out):
    ncols = _NSPH * _NRAD
    b0 = _BT // _SBF_G
    k = jax.lax.broadcasted_iota(jnp.int32, (1, _SBF_G * ncols), 1)
    k = k - (k // ncols) * ncols
    lcol = k // _NRAD
    ncol = k - lcol * _NRAD + 1
    zs = np.float32(np.pi) * (ncol.astype(_F32)
                              + 0.5 * lcol.astype(_F32))
    one_row = jnp.zeros((1, ncols), _F32) + 1.0

    def widen(col):
        parts = [col[g * b0:(g + 1) * b0, :] * one_row for g in range(_SBF_G)]
        return jnp.concatenate(parts, axis=1)

    d = widen(ct[:, 0:1] / _CUTOFF + 1e-9)
    env = _envelope(d)
    x = zs * d
    sx = jnp.sin(x)
    cx = jnp.cos(x)
    j0 = sx / x
    j1 = sx / (x * x) - cx / x
    res = jnp.where(lcol == 0, j0, 0.0)
    res = jnp.where(lcol == 1, j1, res)
    jm2, jm1 = j0, j1
    for ll in range(2, _NSPH):
        jl = (2.0 * ll - 1.0) / x * jm1 - jm2
        res = jnp.where(lcol == ll, jl, res)
        jm2, jm1 = jm1, jl
    c = widen(ct[:, 1:2])
    p = jnp.where(lcol == 0, 1.0, 0.0)
    p = jnp.where(lcol == 1, c, p)
    pm2 = jnp.zeros_like(x) + 1.0
    pm1 = c
    for ll in range(2, _NSPH):
        pc = ((2.0 * ll - 1.0) * c * pm1 - (ll - 1.0) * pm2) / ll
        p = jnp.where(lcol == ll, pc, p)
        pm2, pm1 = pm1, pc
    sbf = env * res * p
    for g in range(_SBF_G):
        blk = sbf[:, g * ncols:(g + 1) * ncols]
        sp0_out[g * b0:(g + 1) * b0, :] = _dot(blk, wsb0[...])
        sp1_out[g * b0:(g + 1) * b0, :] = _dot(blk, wsb1[...])


def _sbf_project(ct2, wsb0, wsb1):
    t = ct2.shape[0]
    grid = t // _BT
    wspec = lambda a: pl.BlockSpec(a.shape, lambda i: (0, 0))
    return pl.pallas_call(
        _sbf_body,
        grid=(grid,),
        in_specs=[
            pl.BlockSpec((_BT, 2), lambda i: (i, 0)),
            wspec(wsb0), wspec(wsb1),
        ],
        out_specs=[
            pl.BlockSpec((_BT, 32), lambda i: (i, 0)),
            pl.BlockSpec((_BT, 32), lambda i: (i, 0)),
        ],
        out_shape=[
            jax.ShapeDtypeStruct((t, 32), _F32),
            jax.ShapeDtypeStruct((t, 32), _F32),
        ],
    )(ct2, wsb0, wsb1)


def _out_body(t_in, wup, d0, b0, d1, b1, d2, b2, wf, p_out):
    t = _dot(t_in[...], wup[...])
    t = _swish(_dot(t, d0[...]) + b0[...])
    t = _swish(_dot(t, d1[...]) + b1[...])
    t = _swish(_dot(t, d2[...]) + b2[...])
    p_out[...] = _dot(t, wf[...])


def _out_block(t_nodes, wup, dw, db, wf_pad):
    n = t_nodes.shape[0]
    grid = n // _BN
    wspec = lambda a: pl.BlockSpec(a.shape, lambda i: (0, 0))
    args = [t_nodes, wup,
            dw[0], db[0].reshape(1, -1), dw[1], db[1].reshape(1, -1),
            dw[2], db[2].reshape(1, -1), wf_pad]
    return pl.pallas_call(
        _out_body,
        grid=(grid,),
        in_specs=[pl.BlockSpec((_BN, 64), lambda i: (i, 0))]
        + [wspec(a) for a in args[1:]],
        out_specs=pl.BlockSpec((_BN, 128), lambda i: (i, 0)),
        out_shape=jax.ShapeDtypeStruct((n, 128), _F32),
    )(*args)


_SC_NC = 2
_SC_NS = 16
_SC_NW = _SC_NC * _SC_NS
_SC_IW = 125
_SC_CH = 1000


def _sc_gmul_body(table_hbm, idx_hbm, sp_hbm, out_hbm, idx_v, rows_v, sp_v,
                  sem):
    t_total = out_hbm.shape[0]
    n_chunks = t_total // (_SC_NW * _SC_CH)
    wid = lax.axis_index("s") * _SC_NC + lax.axis_index("c")
    base_row = wid * (n_chunks * (_SC_CH // _SC_IW))

    def chunk_body(k, carry):
        row0 = base_row + k * (_SC_CH // _SC_IW)
        t0 = row0 * _SC_IW
        pltpu.sync_copy(idx_hbm.at[pl.ds(row0, _SC_CH // _SC_IW)], idx_v)
        copies = [pltpu.async_copy(
            sp_hbm.at[pl.ds(t0 * 32, _SC_CH * 32)], sp_v, sem)]
        for j in range(_SC_CH // _SC_IW):
            copies.append(pltpu.async_copy(
                table_hbm.at[idx_v.at[j]],
                rows_v.at[pl.ds(j * _SC_IW, _SC_IW)], sem))
        for cp in copies:
            cp.wait()

        def mul_body(r, c2):
            rr = r * 4
            for u in range(4):
                a0 = rows_v[rr + u, pl.ds(0, 16)]
                a1 = rows_v[rr + u, pl.ds(16, 16)]
                b0 = sp_v[pl.ds((rr + u) * 32, 16)]
                b1 = sp_v[pl.ds((rr + u) * 32 + 16, 16)]
                rows_v[rr + u, pl.ds(0, 16)] = a0 * b0
                rows_v[rr + u, pl.ds(16, 16)] = a1 * b1
            return c2

        lax.fori_loop(0, _SC_CH // 4, mul_body, 0)
        pltpu.sync_copy(rows_v, out_hbm.at[pl.ds(t0, _SC_CH)])
        return carry

    lax.fori_loop(0, n_chunks, chunk_body, 0)


def _sc_gather_mul(table, idx_rows, sp_flat):
    t_total = idx_rows.shape[0] * idx_rows.shape[1]
    mesh = plsc.VectorSubcoreMesh(core_axis_name="c", subcore_axis_name="s",
                                  num_cores=_SC_NC, num_subcores=_SC_NS)
    f = pl.kernel(
        _sc_gmul_body,
        out_type=jax.ShapeDtypeStruct((t_total, 32), _F32),
        mesh=mesh,
        scratch_types=[
            pltpu.VMEM((_SC_CH // _SC_IW, _SC_IW), jnp.int32),
            pltpu.VMEM((_SC_CH, 32), _F32),
            pltpu.VMEM((_SC_CH * 32,), _F32),
            pltpu.SemaphoreType.DMA,
        ],
        compiler_params=pltpu.CompilerParams(use_tc_tiling_on_sc=False),
    )
    return f(table, idx_rows, sp_flat)


_SC_CH2 = 500


def _sc_gather2_body(table_hbm, idxa_hbm, idxb_hbm, outa_hbm, outb_hbm,
                     idx_v, rowsa_v, rowsb_v, sem):
    b_total = outa_hbm.shape[0]
    n_chunks = b_total // (_SC_NW * _SC_CH2)
    wid = lax.axis_index("s") * _SC_NC + lax.axis_index("c")
    rpc = _SC_CH2 // _SC_IW
    base_row = wid * (n_chunks * rpc)

    def chunk_body(k, carry):
        row0 = base_row + k * rpc
        t0 = row0 * _SC_IW
        pltpu.sync_copy(idxa_hbm.at[pl.ds(row0, rpc)], idx_v.at[pl.ds(0, rpc)])
        pltpu.sync_copy(idxb_hbm.at[pl.ds(row0, rpc)],
                        idx_v.at[pl.ds(rpc, rpc)])
        copies = []
        for j in range(rpc):
            copies.append(pltpu.async_copy(
                table_hbm.at[idx_v.at[j]],
                rowsa_v.at[pl.ds(j * _SC_IW, _SC_IW)], sem))
            copies.append(pltpu.async_copy(
                table_hbm.at[idx_v.at[rpc + j]],
                rowsb_v.at[pl.ds(j * _SC_IW, _SC_IW)], sem))
        for cp in copies:
            cp.wait()
        pltpu.sync_copy(rowsa_v, outa_hbm.at[pl.ds(t0, _SC_CH2)])
        pltpu.sync_copy(rowsb_v, outb_hbm.at[pl.ds(t0, _SC_CH2)])
        return carry

    lax.fori_loop(0, n_chunks, chunk_body, 0)


def _sc_gather2(table, idxa_rows, idxb_rows):
    b_total = idxa_rows.shape[0] * idxa_rows.shape[1]
    d = table.shape[1]
    mesh = plsc.VectorSubcoreMesh(core_axis_name="c", subcore_axis_name="s",
                                  num_cores=_SC_NC, num_subcores=_SC_NS)
    f = pl.kernel(
        _sc_gather2_body,
        out_type=[jax.ShapeDtypeStruct((b_total, d), _F32),
                  jax.ShapeDtypeStruct((b_total, d), _F32)],
        mesh=mesh,
        scratch_types=[
            pltpu.VMEM((2 * (_SC_CH2 // _SC_IW), _SC_IW), jnp.int32),
            pltpu.VMEM((_SC_CH2, d), _F32),
            pltpu.VMEM((_SC_CH2, d), _F32),
            pltpu.SemaphoreType.DMA,
        ],
        compiler_params=pltpu.CompilerParams(use_tc_tiling_on_sc=False),
    )
    return f(table, idxa_rows, idxb_rows)


def kernel(node_attr, edge_type, Dij, Anglesijk, batch_seg, idnb_i, idnb_j,
           id_expand_kj, id_reduce_ji, emb_table, W_rbf_emb, W_node, b_node,
           W_emb, b_emb, int_W_edge, int_W_rbf1, int_W_rbf2, int_W_sbf1,
           int_W_sbf2, int_W_ji, int_b_ji, int_W_kj, int_b_kj, int_W_down,
           int_W_up, int_res_bef_W, int_res_bef_b, int_W_skip, int_b_skip,
           int_res_aft_W, int_res_aft_b, out_W_up, out_dense_W, out_dense_b,
           out_W_final):
    n = node_attr.shape[0]
    e = Dij.shape[0]
    nmol = 512

    de2 = jnp.concatenate(
        [Dij.reshape(e, 1), edge_type.astype(_F32).reshape(e, 1)], axis=1)
    t_len = Anglesijk.shape[0]
    t_pad = ((t_len + _BT - 1) // _BT) * _BT - t_len
    cang_p = jnp.pad(jnp.cos(Anglesijk), (0, t_pad))

    w1 = W_emb[0:64]
    w2 = W_emb[64:128]
    wr = W_rbf_emb @ W_emb[128:192]
    we4 = emb_table @ W_emb[192:256]
    bemb = b_emb.reshape(1, -1)

    h = _node_embed(node_attr, W_node, b_node)
    hi, hj = _sc_gather2(h,
                         idnb_i.astype(jnp.int32).reshape(-1, _SC_IW),
                         idnb_j.astype(jnp.int32).reshape(-1, _SC_IW))
    x, rbf8 = _edge_embed(de2, hi, hj, w1, w2, wr, we4, bemb)

    dt_p = jnp.pad(jnp.take(Dij, id_reduce_ji, axis=0), (0, t_pad),
                   constant_values=_CUTOFF)
    ct2 = jnp.concatenate(
        [dt_p.reshape(-1, 1), cang_p.reshape(-1, 1)], axis=1)
    idx_rows = id_expand_kj.astype(jnp.int32).reshape(-1, _SC_IW)
    wsb0 = int_W_sbf1[0] @ int_W_sbf2[0]
    wsb1 = int_W_sbf1[1] @ int_W_sbf2[1]
    sp = _sbf_project(ct2, wsb0, wsb1)

    wf_pad = [jnp.pad(out_W_final[i], ((0, 0), (0, 128 - out_W_final.shape[2])))
              for i in range(_NB + 1)]

    t0 = jax.ops.segment_sum(x, idnb_i, num_segments=n)
    P = _out_block(t0, out_W_up[0], out_dense_W[0], out_dense_b[0], wf_pad[0])

    for i in range(_NB):
        we4_i = emb_table @ int_W_edge[i]
        wrbf_i = int_W_rbf1[i] @ int_W_rbf2[i]
        xji, xdown = _int_pre(
            x, de2, rbf8, we4_i,
            int_W_ji[i], int_b_ji[i].reshape(1, -1),
            int_W_kj[i], int_b_kj[i].reshape(1, -1),
            wrbf_i, int_W_down[i])
        msg = _sc_gather_mul(xdown, idx_rows, sp[i].reshape(-1))
        seg = jax.ops.segment_sum(msg, id_reduce_ji, num_segments=e,
                                  indices_are_sorted=True)
        x = _int_post(
            xji, seg, x, int_W_up[i],
            int_res_bef_W[i, 0, 0], int_res_bef_b[i, 0, 0].reshape(1, -1),
            int_res_bef_W[i, 0, 1], int_res_bef_b[i, 0, 1].reshape(1, -1),
            int_W_skip[i], int_b_skip[i].reshape(1, -1),
            int_res_aft_W[i, 0, 0], int_res_aft_b[i, 0, 0].reshape(1, -1),
            int_res_aft_W[i, 0, 1], int_res_aft_b[i, 0, 1].reshape(1, -1),
            int_res_aft_W[i, 1, 0], int_res_aft_b[i, 1, 0].reshape(1, -1),
            int_res_aft_W[i, 1, 1], int_res_aft_b[i, 1, 1].reshape(1, -1))
        ti = jax.ops.segment_sum(x, idnb_i, num_segments=n)
        P = P + _out_block(ti, out_W_up[i + 1], out_dense_W[i + 1],
                           out_dense_b[i + 1], wf_pad[i + 1])

    out = jax.ops.segment_sum(P, batch_seg, num_segments=nmol,
                              indices_are_sorted=True)
    return out[:, :12]


# speedup vs baseline: 2.1931x; 1.1486x over previous
import functools
import numpy as np
import jax
import jax.numpy as jnp
from jax import lax
from jax.experimental import pallas as pl
from jax.experimental.pallas import tpu as pltpu
from jax.experimental.pallas import tpu_sc as plsc

_CUTOFF = 5.0
_NRAD = 6
_NSPH = 7
_NB = 2
_NDO = 3

_BE = 1280
_BT = 9600
_BN = 2000

_F32 = jnp.float32


def _swish(x):
    return x * jax.nn.sigmoid(x)


def _envelope(d):
    a = -28.0
    b = 48.0
    c = -21.0
    d2 = d * d
    d4 = d2 * d2
    d5 = d4 * d
    env = 1.0 / d + a * d5 + b * d5 * d + c * d5 * d2
    return jnp.where(d < 1.0, env, 0.0)


def _rbf_from_d(d):
    k = jax.lax.broadcasted_iota(jnp.int32, (1, _NRAD), 1).astype(_F32)
    freq = (k + 1.0) * np.float32(np.pi)
    return _envelope(d) * jnp.sin(freq * d)


def _iota4(et):
    return jax.lax.broadcasted_iota(jnp.int32, (1, 4), 1)


def _dot(a, b):
    return jnp.dot(a, b, preferred_element_type=_F32)


def _node_body(na, w, b, h_out):
    h_out[...] = _dot(na[...], w[...]) + b[...]


def _node_embed(node_attr, W_node, b_node):
    n = node_attr.shape[0]
    grid = n // _BN
    return pl.pallas_call(
        _node_body,
        grid=(grid,),
        in_specs=[
            pl.BlockSpec((_BN, node_attr.shape[1]), lambda i: (i, 0)),
            pl.BlockSpec(W_node.shape, lambda i: (0, 0)),
            pl.BlockSpec((1, b_node.shape[0]), lambda i: (0, 0)),
        ],
        out_specs=pl.BlockSpec((_BN, 64), lambda i: (i, 0)),
        out_shape=jax.ShapeDtypeStruct((n, 64), _F32),
    )(node_attr, W_node, b_node.reshape(1, -1))


def _edge_body(de, hi, hj, w1, w2, wr, we4, bemb, x_out, rbf_out):
    d = de[:, 0:1] / _CUTOFF
    rbf = _rbf_from_d(d)
    oh = (de[:, 1:2] == _iota4(de).astype(_F32)).astype(_F32)
    acc = (_dot(hi[...], w1[...]) + _dot(hj[...], w2[...])
           + _dot(rbf, wr[...]) + _dot(oh, we4[...]) + bemb[...])
    x_out[...] = _swish(acc)
    rbf_out[...] = jnp.concatenate(
        [rbf, jnp.zeros_like(rbf[:, 0:2])], axis=1)


def _edge_embed(de2, hi, hj, w1, w2, wr, we4, bemb):
    e = de2.shape[0]
    grid = e // _BE
    wspec = lambda a: pl.BlockSpec(a.shape, lambda i: (0, 0))
    return pl.pallas_call(
        _edge_body,
        grid=(grid,),
        in_specs=[
            pl.BlockSpec((_BE, 2), lambda i: (i, 0)),
            pl.BlockSpec((_BE, 64), lambda i: (i, 0)),
            pl.BlockSpec((_BE, 64), lambda i: (i, 0)),
            wspec(w1), wspec(w2), wspec(wr), wspec(we4), wspec(bemb),
        ],
        out_specs=[
            pl.BlockSpec((_BE, 64), lambda i: (i, 0)),
            pl.BlockSpec((_BE, 8), lambda i: (i, 0)),
        ],
        out_shape=[
            jax.ShapeDtypeStruct((e, 64), _F32),
            jax.ShapeDtypeStruct((e, 8), _F32),
        ],
    )(de2, hi, hj, w1, w2, wr, we4, bemb)


def _int_pre_body(x, de, rbf8, we4, wji, bji, wkj, bkj, wrbf, wdown,
                  xji_out, xdown_out):
    oh = (de[:, 1:2] == _iota4(de).astype(_F32)).astype(_F32)
    m = x[...] + _dot(oh, we4[...])
    xji_out[...] = _swish(_dot(m, wji[...]) + bji[...])
    rbf_p = _dot(rbf8[:, 0:_NRAD], wrbf[...])
    xkj = _swish(_dot(m, wkj[...]) + bkj[...]) * rbf_p
    xdown_out[...] = _dot(xkj, wdown[...])


def _int_pre(x, de2, rbf8, we4, wji, bji, wkj, bkj, wrbf, wdown):
    e = x.shape[0]
    grid = e // _BE
    wspec = lambda a: pl.BlockSpec(a.shape, lambda i: (0, 0))
    return pl.pallas_call(
        _int_pre_body,
        grid=(grid,),
        in_specs=[
            pl.BlockSpec((_BE, 64), lambda i: (i, 0)),
            pl.BlockSpec((_BE, 2), lambda i: (i, 0)),
            pl.BlockSpec((_BE, 8), lambda i: (i, 0)),
            wspec(we4), wspec(wji), wspec(bji), wspec(wkj), wspec(bkj),
            wspec(wrbf), wspec(wdown),
        ],
        out_specs=[
            pl.BlockSpec((_BE, 64), lambda i: (i, 0)),
            pl.BlockSpec((_BE, 32), lambda i: (i, 0)),
        ],
        out_shape=[
            jax.ShapeDtypeStruct((e, 64), _F32),
            jax.ShapeDtypeStruct((e, 32), _F32),
        ],
    )(x, de2, rbf8, we4, wji, bji, wkj, bkj, wrbf, wdown)


def _int_post_body(xji, seg, xold, wup, wb0, bb0, wb1, bb1, wskip, bskip,
                   wa00, ba00, wa01, ba01, wa10, ba10, wa11, ba11, x_out):
    hh = xji[...] + _dot(seg[...], wup[...])
    h2 = _swish(_dot(hh, wb0[...]) + bb0[...])
    h2 = _swish(_dot(h2, wb1[...]) + bb1[...])
    hh = hh + h2
    hh = _swish(_dot(hh, wskip[...]) + bskip[...]) + xold[...]
    h2 = _swish(_dot(hh, wa00[...]) + ba00[...])
    h2 = _swish(_dot(h2, wa01[...]) + ba01[...])
    hh = hh + h2
    h2 = _swish(_dot(hh, wa10[...]) + ba10[...])
    h2 = _swish(_dot(h2, wa11[...]) + ba11[...])
    x_out[...] = hh + h2


def _int_post(xji, seg, xold, *ws):
    e = xji.shape[0]
    grid = e // _BE
    wspec = lambda a: pl.BlockSpec(a.shape, lambda i: (0, 0))
    return pl.pallas_call(
        _int_post_body,
        grid=(grid,),
        in_specs=[
            pl.BlockSpec((_BE, 64), lambda i: (i, 0)),
            pl.BlockSpec((_BE, 32), lambda i: (i, 0)),
            pl.BlockSpec((_BE, 64), lambda i: (i, 0)),
        ] + [wspec(w) for w in ws],
        out_specs=pl.BlockSpec((_BE, 64), lambda i: (i, 0)),
        out_shape=jax.ShapeDtypeStruct((e, 64), _F32),
    )(xji, seg, xold, *ws)


_SBF_G = 3


def _sbf_body(ct, wsb0, wsb1, sp0_out, sp1_out):
    ncols = _NSPH * _NRAD
    b0 = _BT // _SBF_G
    k = jax.lax.broadcasted_iota(jnp.int32, (1, _SBF_G * ncols), 1)
    k = k - (k // ncols) * ncols
    lcol = k // _NRAD
    ncol = k - lcol * _NRAD + 1
    zs = np.float32(np.pi) * (ncol.astype(_F32)
                              + 0.5 * lcol.astype(_F32))
    one_row = jnp.zeros((1, ncols), _F32) + 1.0

    def widen(col):
        parts = [col[g * b0:(g + 1) * b0, :] * one_row for g in range(_SBF_G)]
        return jnp.concatenate(parts, axis=1)

    d = widen(ct[:, 0:1] / _CUTOFF + 1e-9)
    env = _envelope(d)
    x = zs * d
    sx = jnp.sin(x)
    cx = jnp.cos(x)
    j0 = sx / x
    j1 = sx / (x * x) - cx / x
    res = jnp.where(lcol == 0, j0, 0.0)
    res = jnp.where(lcol == 1, j1, res)
    jm2, jm1 = j0, j1
    for ll in range(2, _NSPH):
        jl = (2.0 * ll - 1.0) / x * jm1 - jm2
        res = jnp.where(lcol == ll, jl, res)
        jm2, jm1 = jm1, jl
    c = widen(ct[:, 1:2])
    p = jnp.where(lcol == 0, 1.0, 0.0)
    p = jnp.where(lcol == 1, c, p)
    pm2 = jnp.zeros_like(x) + 1.0
    pm1 = c
    for ll in range(2, _NSPH):
        pc = ((2.0 * ll - 1.0) * c * pm1 - (ll - 1.0) * pm2) / ll
        p = jnp.where(lcol == ll, pc, p)
        pm2, pm1 = pm1, pc
    sbf = env * res * p
    for g in range(_SBF_G):
        blk = sbf[:, g * ncols:(g + 1) * ncols]
        sp0_out[g * b0:(g + 1) * b0, :] = _dot(blk, wsb0[...])
        sp1_out[g * b0:(g + 1) * b0, :] = _dot(blk, wsb1[...])


def _sbf_project(ct2, wsb0, wsb1):
    t = ct2.shape[0]
    grid = t // _BT
    wspec = lambda a: pl.BlockSpec(a.shape, lambda i: (0, 0))
    return pl.pallas_call(
        _sbf_body,
        grid=(grid,),
        in_specs=[
            pl.BlockSpec((_BT, 2), lambda i: (i, 0)),
            wspec(wsb0), wspec(wsb1),
        ],
        out_specs=[
            pl.BlockSpec((_BT, 32), lambda i: (i, 0)),
            pl.BlockSpec((_BT, 32), lambda i: (i, 0)),
        ],
        out_shape=[
            jax.ShapeDtypeStruct((t, 32), _F32),
            jax.ShapeDtypeStruct((t, 32), _F32),
        ],
    )(ct2, wsb0, wsb1)


def _out_body(t_in, wup, d0, b0, d1, b1, d2, b2, wf, p_out):
    t = _dot(t_in[...], wup[...])
    t = _swish(_dot(t, d0[...]) + b0[...])
    t = _swish(_dot(t, d1[...]) + b1[...])
    t = _swish(_dot(t, d2[...]) + b2[...])
    p_out[...] = _dot(t, wf[...])


def _out_block(t_nodes, wup, dw, db, wf_pad):
    n = t_nodes.shape[0]
    grid = n // _BN
    wspec = lambda a: pl.BlockSpec(a.shape, lambda i: (0, 0))
    args = [t_nodes, wup,
            dw[0], db[0].reshape(1, -1), dw[1], db[1].reshape(1, -1),
            dw[2], db[2].reshape(1, -1), wf_pad]
    return pl.pallas_call(
        _out_body,
        grid=(grid,),
        in_specs=[pl.BlockSpec((_BN, 64), lambda i: (i, 0))]
        + [wspec(a) for a in args[1:]],
        out_specs=pl.BlockSpec((_BN, 128), lambda i: (i, 0)),
        out_shape=jax.ShapeDtypeStruct((n, 128), _F32),
    )(*args)


_SC_NC = 2
_SC_NS = 16
_SC_NW = _SC_NC * _SC_NS
_SC_IW = 125
_SC_CH = 1000


def _sc_gmul_body(table_hbm, idx_hbm, sp_hbm, out_hbm, idx_v, rows_v, sp_v,
                  sem):
    t_total = out_hbm.shape[0]
    n_chunks = t_total // (_SC_NW * _SC_CH)
    wid = lax.axis_index("s") * _SC_NC + lax.axis_index("c")
    base_row = wid * (n_chunks * (_SC_CH // _SC_IW))

    def chunk_body(k, carry):
        row0 = base_row + k * (_SC_CH // _SC_IW)
        t0 = row0 * _SC_IW
        pltpu.sync_copy(idx_hbm.at[pl.ds(row0, _SC_CH // _SC_IW)], idx_v)
        copies = [pltpu.async_copy(
            sp_hbm.at[pl.ds(t0 * 32, _SC_CH * 32)], sp_v, sem)]
        for j in range(_SC_CH // _SC_IW):
            copies.append(pltpu.async_copy(
                table_hbm.at[idx_v.at[j]],
                rows_v.at[pl.ds(j * _SC_IW, _SC_IW)], sem))
        for cp in copies:
            cp.wait()

        def mul_body(r, c2):
            rr = r * 4
            for u in range(4):
                a0 = rows_v[rr + u, pl.ds(0, 16)]
                a1 = rows_v[rr + u, pl.ds(16, 16)]
                b0 = sp_v[pl.ds((rr + u) * 32, 16)]
                b1 = sp_v[pl.ds((rr + u) * 32 + 16, 16)]
                rows_v[rr + u, pl.ds(0, 16)] = a0 * b0
                rows_v[rr + u, pl.ds(16, 16)] = a1 * b1
            return c2

        lax.fori_loop(0, _SC_CH // 4, mul_body, 0)
        pltpu.sync_copy(rows_v, out_hbm.at[pl.ds(t0, _SC_CH)])
        return carry

    lax.fori_loop(0, n_chunks, chunk_body, 0)


def _sc_gather_mul(table, idx_rows, sp_flat):
    t_total = idx_rows.shape[0] * idx_rows.shape[1]
    mesh = plsc.VectorSubcoreMesh(core_axis_name="c", subcore_axis_name="s",
                                  num_cores=_SC_NC, num_subcores=_SC_NS)
    f = pl.kernel(
        _sc_gmul_body,
        out_type=jax.ShapeDtypeStruct((t_total, 32), _F32),
        mesh=mesh,
        scratch_types=[
            pltpu.VMEM((_SC_CH // _SC_IW, _SC_IW), jnp.int32),
            pltpu.VMEM((_SC_CH, 32), _F32),
            pltpu.VMEM((_SC_CH * 32,), _F32),
            pltpu.SemaphoreType.DMA,
        ],
        compiler_params=pltpu.CompilerParams(use_tc_tiling_on_sc=False),
    )
    return f(table, idx_rows, sp_flat)


_SC_CH2 = 500


def _sc_gather2_body(table_hbm, idxa_hbm, idxb_hbm, outa_hbm, outb_hbm,
                     idx_v, rowsa_v, rowsb_v, sem):
    b_total = outa_hbm.shape[0]
    n_chunks = b_total // (_SC_NW * _SC_CH2)
    wid = lax.axis_index("s") * _SC_NC + lax.axis_index("c")
    rpc = _SC_CH2 // _SC_IW
    base_row = wid * (n_chunks * rpc)

    def chunk_body(k, carry):
        row0 = base_row + k * rpc
        t0 = row0 * _SC_IW
        pltpu.sync_copy(idxa_hbm.at[pl.ds(row0, rpc)], idx_v.at[pl.ds(0, rpc)])
        pltpu.sync_copy(idxb_hbm.at[pl.ds(row0, rpc)],
                        idx_v.at[pl.ds(rpc, rpc)])
        copies = []
        for j in range(rpc):
            copies.append(pltpu.async_copy(
                table_hbm.at[idx_v.at[j]],
                rowsa_v.at[pl.ds(j * _SC_IW, _SC_IW)], sem))
            copies.append(pltpu.async_copy(
                table_hbm.at[idx_v.at[rpc + j]],
                rowsb_v.at[pl.ds(j * _SC_IW, _SC_IW)], sem))
        for cp in copies:
            cp.wait()
        pltpu.sync_copy(rowsa_v, outa_hbm.at[pl.ds(t0, _SC_CH2)])
        pltpu.sync_copy(rowsb_v, outb_hbm.at[pl.ds(t0, _SC_CH2)])
        return carry

    lax.fori_loop(0, n_chunks, chunk_body, 0)


def _sc_gather2(table, idxa_rows, idxb_rows):
    b_total = idxa_rows.shape[0] * idxa_rows.shape[1]
    d = table.shape[1]
    mesh = plsc.VectorSubcoreMesh(core_axis_name="c", subcore_axis_name="s",
                                  num_cores=_SC_NC, num_subcores=_SC_NS)
    f = pl.kernel(
        _sc_gather2_body,
        out_type=[jax.ShapeDtypeStruct((b_total, d), _F32),
                  jax.ShapeDtypeStruct((b_total, d), _F32)],
        mesh=mesh,
        scratch_types=[
            pltpu.VMEM((2 * (_SC_CH2 // _SC_IW), _SC_IW), jnp.int32),
            pltpu.VMEM((_SC_CH2, d), _F32),
            pltpu.VMEM((_SC_CH2, d), _F32),
            pltpu.SemaphoreType.DMA,
        ],
        compiler_params=pltpu.CompilerParams(use_tc_tiling_on_sc=False),
    )
    return f(table, idxa_rows, idxb_rows)


def kernel(node_attr, edge_type, Dij, Anglesijk, batch_seg, idnb_i, idnb_j,
           id_expand_kj, id_reduce_ji, emb_table, W_rbf_emb, W_node, b_node,
           W_emb, b_emb, int_W_edge, int_W_rbf1, int_W_rbf2, int_W_sbf1,
           int_W_sbf2, int_W_ji, int_b_ji, int_W_kj, int_b_kj, int_W_down,
           int_W_up, int_res_bef_W, int_res_bef_b, int_W_skip, int_b_skip,
           int_res_aft_W, int_res_aft_b, out_W_up, out_dense_W, out_dense_b,
           out_W_final):
    n = node_attr.shape[0]
    e = Dij.shape[0]
    nmol = 512

    de2 = jnp.concatenate(
        [Dij.reshape(e, 1), edge_type.astype(_F32).reshape(e, 1)], axis=1)
    t_len = Anglesijk.shape[0]
    t_pad = ((t_len + _BT - 1) // _BT) * _BT - t_len
    cang_p = jnp.pad(jnp.cos(Anglesijk), (0, t_pad))

    w1 = W_emb[0:64]
    w2 = W_emb[64:128]
    wr = W_rbf_emb @ W_emb[128:192]
    we4 = emb_table @ W_emb[192:256]
    bemb = b_emb.reshape(1, -1)

    h = _node_embed(node_attr, W_node, b_node)
    hi, hj = _sc_gather2(h,
                         idnb_i.astype(jnp.int32).reshape(-1, _SC_IW),
                         idnb_j.astype(jnp.int32).reshape(-1, _SC_IW))
    x, rbf8 = _edge_embed(de2, hi, hj, w1, w2, wr, we4, bemb)

    dt_p = jnp.pad(
        Dij.at[id_reduce_ji].get(indices_are_sorted=True,
                                 mode="promise_in_bounds"),
        (0, t_pad), constant_values=_CUTOFF)
    ct2 = jnp.concatenate(
        [dt_p.reshape(-1, 1), cang_p.reshape(-1, 1)], axis=1)
    idx_rows = id_expand_kj.astype(jnp.int32).reshape(-1, _SC_IW)
    wsb0 = int_W_sbf1[0] @ int_W_sbf2[0]
    wsb1 = int_W_sbf1[1] @ int_W_sbf2[1]
    sp = _sbf_project(ct2, wsb0, wsb1)

    wf_pad = [jnp.pad(out_W_final[i], ((0, 0), (0, 128 - out_W_final.shape[2])))
              for i in range(_NB + 1)]

    t0 = jax.ops.segment_sum(x, idnb_i, num_segments=n)
    P = _out_block(t0, out_W_up[0], out_dense_W[0], out_dense_b[0], wf_pad[0])

    for i in range(_NB):
        we4_i = emb_table @ int_W_edge[i]
        wrbf_i = int_W_rbf1[i] @ int_W_rbf2[i]
        xji, xdown = _int_pre(
            x, de2, rbf8, we4_i,
            int_W_ji[i], int_b_ji[i].reshape(1, -1),
            int_W_kj[i], int_b_kj[i].reshape(1, -1),
            wrbf_i, int_W_down[i])
        msg = _sc_gather_mul(xdown, idx_rows, sp[i].reshape(-1))
        seg = jax.ops.segment_sum(msg, id_reduce_ji, num_segments=e,
                                  indices_are_sorted=True)
        x = _int_post(
            xji, seg, x, int_W_up[i],
            int_res_bef_W[i, 0, 0], int_res_bef_b[i, 0, 0].reshape(1, -1),
            int_res_bef_W[i, 0, 1], int_res_bef_b[i, 0, 1].reshape(1, -1),
            int_W_skip[i], int_b_skip[i].reshape(1, -1),
            int_res_aft_W[i, 0, 0], int_res_aft_b[i, 0, 0].reshape(1, -1),
            int_res_aft_W[i, 0, 1], int_res_aft_b[i, 0, 1].reshape(1, -1),
            int_res_aft_W[i, 1, 0], int_res_aft_b[i, 1, 0].reshape(1, -1),
            int_res_aft_W[i, 1, 1], int_res_aft_b[i, 1, 1].reshape(1, -1))
        ti = jax.ops.segment_sum(x, idnb_i, num_segments=n)
        P = P + _out_block(ti, out_W_up[i + 1], out_dense_W[i + 1],
                           out_dense_b[i + 1], wf_pad[i + 1])

    out = jax.ops.segment_sum(P, batch_seg, num_segments=nmol,
                              indices_are_sorted=True)
    return out[:, :12]


# speedup vs baseline: 2.1958x; 1.0013x over previous
import functools
import numpy as np
import jax
import jax.numpy as jnp
from jax import lax
from jax.experimental import pallas as pl
from jax.experimental.pallas import tpu as pltpu
from jax.experimental.pallas import tpu_sc as plsc

_CUTOFF = 5.0
_NRAD = 6
_NSPH = 7
_NB = 2
_NDO = 3

_BE = 1280
_BT = 9600
_BN = 2000

_F32 = jnp.float32


def _swish(x):
    return x * jax.nn.sigmoid(x)


def _envelope(d):
    a = -28.0
    b = 48.0
    c = -21.0
    d2 = d * d
    d4 = d2 * d2
    d5 = d4 * d
    env = 1.0 / d + a * d5 + b * d5 * d + c * d5 * d2
    return jnp.where(d < 1.0, env, 0.0)


def _rbf_from_d(d):
    k = jax.lax.broadcasted_iota(jnp.int32, (1, _NRAD), 1).astype(_F32)
    freq = (k + 1.0) * np.float32(np.pi)
    return _envelope(d) * jnp.sin(freq * d)


def _iota4(et):
    return jax.lax.broadcasted_iota(jnp.int32, (1, 4), 1)


def _dot(a, b):
    return jnp.dot(a, b, preferred_element_type=_F32)


def _node_body(na, w, b, h_out):
    h_out[...] = _dot(na[...], w[...]) + b[...]


def _node_embed(node_attr, W_node, b_node):
    n = node_attr.shape[0]
    grid = n // _BN
    return pl.pallas_call(
        _node_body,
        grid=(grid,),
        in_specs=[
            pl.BlockSpec((_BN, node_attr.shape[1]), lambda i: (i, 0)),
            pl.BlockSpec(W_node.shape, lambda i: (0, 0)),
            pl.BlockSpec((1, b_node.shape[0]), lambda i: (0, 0)),
        ],
        out_specs=pl.BlockSpec((_BN, 64), lambda i: (i, 0)),
        out_shape=jax.ShapeDtypeStruct((n, 64), _F32),
    )(node_attr, W_node, b_node.reshape(1, -1))


def _edge_body(de, hi, hj, w1, w2, wr, we4, bemb, x_out, rbf_out):
    d = de[:, 0:1] / _CUTOFF
    rbf = _rbf_from_d(d)
    oh = (de[:, 1:2] == _iota4(de).astype(_F32)).astype(_F32)
    acc = (_dot(hi[...], w1[...]) + _dot(hj[...], w2[...])
           + _dot(rbf, wr[...]) + _dot(oh, we4[...]) + bemb[...])
    x_out[...] = _swish(acc)
    rbf_out[...] = jnp.concatenate(
        [rbf, jnp.zeros_like(rbf[:, 0:2])], axis=1)


def _edge_embed(de2, hi, hj, w1, w2, wr, we4, bemb):
    e = de2.shape[0]
    grid = e // _BE
    wspec = lambda a: pl.BlockSpec(a.shape, lambda i: (0, 0))
    return pl.pallas_call(
        _edge_body,
        grid=(grid,),
        in_specs=[
            pl.BlockSpec((_BE, 2), lambda i: (i, 0)),
            pl.BlockSpec((_BE, 64), lambda i: (i, 0)),
            pl.BlockSpec((_BE, 64), lambda i: (i, 0)),
            wspec(w1), wspec(w2), wspec(wr), wspec(we4), wspec(bemb),
        ],
        out_specs=[
            pl.BlockSpec((_BE, 64), lambda i: (i, 0)),
            pl.BlockSpec((_BE, 8), lambda i: (i, 0)),
        ],
        out_shape=[
            jax.ShapeDtypeStruct((e, 64), _F32),
            jax.ShapeDtypeStruct((e, 8), _F32),
        ],
    )(de2, hi, hj, w1, w2, wr, we4, bemb)


def _int_pre_body(x, de, rbf8, we4, wji, bji, wkj, bkj, wrbf, wdown,
                  xji_out, xdown_out):
    oh = (de[:, 1:2] == _iota4(de).astype(_F32)).astype(_F32)
    m = x[...] + _dot(oh, we4[...])
    xji_out[...] = _swish(_dot(m, wji[...]) + bji[...])
    rbf_p = _dot(rbf8[:, 0:_NRAD], wrbf[...])
    xkj = _swish(_dot(m, wkj[...]) + bkj[...]) * rbf_p
    xdown_out[...] = _dot(xkj, wdown[...])


def _int_pre(x, de2, rbf8, we4, wji, bji, wkj, bkj, wrbf, wdown):
    e = x.shape[0]
    grid = e // _BE
    wspec = lambda a: pl.BlockSpec(a.shape, lambda i: (0, 0))
    return pl.pallas_call(
        _int_pre_body,
        grid=(grid,),
        in_specs=[
            pl.BlockSpec((_BE, 64), lambda i: (i, 0)),
            pl.BlockSpec((_BE, 2), lambda i: (i, 0)),
            pl.BlockSpec((_BE, 8), lambda i: (i, 0)),
            wspec(we4), wspec(wji), wspec(bji), wspec(wkj), wspec(bkj),
            wspec(wrbf), wspec(wdown),
        ],
        out_specs=[
            pl.BlockSpec((_BE, 64), lambda i: (i, 0)),
            pl.BlockSpec((_BE, 32), lambda i: (i, 0)),
        ],
        out_shape=[
            jax.ShapeDtypeStruct((e, 64), _F32),
            jax.ShapeDtypeStruct((e, 32), _F32),
        ],
    )(x, de2, rbf8, we4, wji, bji, wkj, bkj, wrbf, wdown)


def _int_post_body(xji, seg, xold, wup, wb0, bb0, wb1, bb1, wskip, bskip,
                   wa00, ba00, wa01, ba01, wa10, ba10, wa11, ba11, x_out):
    hh = xji[...] + _dot(seg[...], wup[...])
    h2 = _swish(_dot(hh, wb0[...]) + bb0[...])
    h2 = _swish(_dot(h2, wb1[...]) + bb1[...])
    hh = hh + h2
    hh = _swish(_dot(hh, wskip[...]) + bskip[...]) + xold[...]
    h2 = _swish(_dot(hh, wa00[...]) + ba00[...])
    h2 = _swish(_dot(h2, wa01[...]) + ba01[...])
    hh = hh + h2
    h2 = _swish(_dot(hh, wa10[...]) + ba10[...])
    h2 = _swish(_dot(h2, wa11[...]) + ba11[...])
    x_out[...] = hh + h2


def _int_post(xji, seg, xold, *ws):
    e = xji.shape[0]
    grid = e // _BE
    wspec = lambda a: pl.BlockSpec(a.shape, lambda i: (0, 0))
    return pl.pallas_call(
        _int_post_body,
        grid=(grid,),
        in_specs=[
            pl.BlockSpec((_BE, 64), lambda i: (i, 0)),
            pl.BlockSpec((_BE, 32), lambda i: (i, 0)),
            pl.BlockSpec((_BE, 64), lambda i: (i, 0)),
        ] + [wspec(w) for w in ws],
        out_specs=pl.BlockSpec((_BE, 64), lambda i: (i, 0)),
        out_shape=jax.ShapeDtypeStruct((e, 64), _F32),
    )(xji, seg, xold, *ws)


_SBF_G = 3


def _sbf_body(ct, wsb0, wsb1, sp0_out, sp1_out):
    ncols = _NSPH * _NRAD
    b0 = _BT // _SBF_G
    k = jax.lax.broadcasted_iota(jnp.int32, (1, _SBF_G * ncols), 1)
    k = k - (k // ncols) * ncols
    lcol = k // _NRAD
    ncol = k - lcol * _NRAD + 1
    zs = np.float32(np.pi) * (ncol.astype(_F32)
                              + 0.5 * lcol.astype(_F32))
    one_row = jnp.zeros((1, ncols), _F32) + 1.0

    def widen(col):
        parts = [col[g * b0:(g + 1) * b0, :] * one_row for g in range(_SBF_G)]
        return jnp.concatenate(parts, axis=1)

    d = widen(ct[:, 0:1] / _CUTOFF + 1e-9)
    env = _envelope(d)
    x = zs * d
    sx = jnp.sin(x)
    cx = jnp.cos(x)
    j0 = sx / x
    j1 = sx / (x * x) - cx / x
    res = jnp.where(lcol == 0, j0, 0.0)
    res = jnp.where(lcol == 1, j1, res)
    jm2, jm1 = j0, j1
    for ll in range(2, _NSPH):
        jl = (2.0 * ll - 1.0) / x * jm1 - jm2
        res = jnp.where(lcol == ll, jl, res)
        jm2, jm1 = jm1, jl
    c = widen(ct[:, 1:2])
    p = jnp.where(lcol == 0, 1.0, 0.0)
    p = jnp.where(lcol == 1, c, p)
    pm2 = jnp.zeros_like(x) + 1.0
    pm1 = c
    for ll in range(2, _NSPH):
        pc = ((2.0 * ll - 1.0) * c * pm1 - (ll - 1.0) * pm2) / ll
        p = jnp.where(lcol == ll, pc, p)
        pm2, pm1 = pm1, pc
    sbf = env * res * p
    for g in range(_SBF_G):
        blk = sbf[:, g * ncols:(g + 1) * ncols]
        sp0_out[g * b0:(g + 1) * b0, :] = _dot(blk, wsb0[...])
        sp1_out[g * b0:(g + 1) * b0, :] = _dot(blk, wsb1[...])


def _sbf_project(ct2, wsb0, wsb1):
    t = ct2.shape[0]
    grid = t // _BT
    wspec = lambda a: pl.BlockSpec(a.shape, lambda i: (0, 0))
    return pl.pallas_call(
        _sbf_body,
        grid=(grid,),
        in_specs=[
            pl.BlockSpec((_BT, 2), lambda i: (i, 0)),
            wspec(wsb0), wspec(wsb1),
        ],
        out_specs=[
            pl.BlockSpec((_BT, 32), lambda i: (i, 0)),
            pl.BlockSpec((_BT, 32), lambda i: (i, 0)),
        ],
        out_shape=[
            jax.ShapeDtypeStruct((t, 32), _F32),
            jax.ShapeDtypeStruct((t, 32), _F32),
        ],
    )(ct2, wsb0, wsb1)


def _out_body(t_in, wup, d0, b0, d1, b1, d2, b2, wf, p_out):
    t = _dot(t_in[...], wup[...])
    t = _swish(_dot(t, d0[...]) + b0[...])
    t = _swish(_dot(t, d1[...]) + b1[...])
    t = _swish(_dot(t, d2[...]) + b2[...])
    p_out[...] = _dot(t, wf[...])


def _out_block(t_nodes, wup, dw, db, wf_pad):
    n = t_nodes.shape[0]
    grid = n // _BN
    wspec = lambda a: pl.BlockSpec(a.shape, lambda i: (0, 0))
    args = [t_nodes, wup,
            dw[0], db[0].reshape(1, -1), dw[1], db[1].reshape(1, -1),
            dw[2], db[2].reshape(1, -1), wf_pad]
    return pl.pallas_call(
        _out_body,
        grid=(grid,),
        in_specs=[pl.BlockSpec((_BN, 64), lambda i: (i, 0))]
        + [wspec(a) for a in args[1:]],
        out_specs=pl.BlockSpec((_BN, 128), lambda i: (i, 0)),
        out_shape=jax.ShapeDtypeStruct((n, 128), _F32),
    )(*args)


_SC_NC = 2
_SC_NS = 16
_SC_NW = _SC_NC * _SC_NS
_SC_IW = 125
_SC_CH = 1250


def _sc_gmul_body(table_hbm, idx_hbm, sp_hbm, out_hbm, idx_v, rows_v, sp_v,
                  sem):
    t_total = out_hbm.shape[0]
    n_chunks = t_total // (_SC_NW * _SC_CH)
    wid = lax.axis_index("s") * _SC_NC + lax.axis_index("c")
    base_row = wid * (n_chunks * (_SC_CH // _SC_IW))

    def chunk_body(k, carry):
        row0 = base_row + k * (_SC_CH // _SC_IW)
        t0 = row0 * _SC_IW
        pltpu.sync_copy(idx_hbm.at[pl.ds(row0, _SC_CH // _SC_IW)], idx_v)
        copies = [pltpu.async_copy(
            sp_hbm.at[pl.ds(t0 * 32, _SC_CH * 32)], sp_v, sem)]
        for j in range(_SC_CH // _SC_IW):
            copies.append(pltpu.async_copy(
                table_hbm.at[idx_v.at[j]],
                rows_v.at[pl.ds(j * _SC_IW, _SC_IW)], sem))
        for cp in copies:
            cp.wait()

        def mul_body(r, c2):
            rr = r * 2
            for u in range(2):
                a0 = rows_v[rr + u, pl.ds(0, 16)]
                a1 = rows_v[rr + u, pl.ds(16, 16)]
                b0 = sp_v[pl.ds((rr + u) * 32, 16)]
                b1 = sp_v[pl.ds((rr + u) * 32 + 16, 16)]
                rows_v[rr + u, pl.ds(0, 16)] = a0 * b0
                rows_v[rr + u, pl.ds(16, 16)] = a1 * b1
            return c2

        lax.fori_loop(0, _SC_CH // 2, mul_body, 0)
        pltpu.sync_copy(rows_v, out_hbm.at[pl.ds(t0, _SC_CH)])
        return carry

    lax.fori_loop(0, n_chunks, chunk_body, 0)


def _sc_gather_mul(table, idx_rows, sp_flat):
    t_total = idx_rows.shape[0] * idx_rows.shape[1]
    mesh = plsc.VectorSubcoreMesh(core_axis_name="c", subcore_axis_name="s",
                                  num_cores=_SC_NC, num_subcores=_SC_NS)
    f = pl.kernel(
        _sc_gmul_body,
        out_type=jax.ShapeDtypeStruct((t_total, 32), _F32),
        mesh=mesh,
        scratch_types=[
            pltpu.VMEM((_SC_CH // _SC_IW, _SC_IW), jnp.int32),
            pltpu.VMEM((_SC_CH, 32), _F32),
            pltpu.VMEM((_SC_CH * 32,), _F32),
            pltpu.SemaphoreType.DMA,
        ],
        compiler_params=pltpu.CompilerParams(use_tc_tiling_on_sc=False),
    )
    return f(table, idx_rows, sp_flat)


_SC_CH2 = 500


def _sc_gather2_body(table_hbm, idxa_hbm, idxb_hbm, outa_hbm, outb_hbm,
                     idx_v, rowsa_v, rowsb_v, sem):
    b_total = outa_hbm.shape[0]
    n_chunks = b_total // (_SC_NW * _SC_CH2)
    wid = lax.axis_index("s") * _SC_NC + lax.axis_index("c")
    rpc = _SC_CH2 // _SC_IW
    base_row = wid * (n_chunks * rpc)

    def chunk_body(k, carry):
        row0 = base_row + k * rpc
        t0 = row0 * _SC_IW
        pltpu.sync_copy(idxa_hbm.at[pl.ds(row0, rpc)], idx_v.at[pl.ds(0, rpc)])
        pltpu.sync_copy(idxb_hbm.at[pl.ds(row0, rpc)],
                        idx_v.at[pl.ds(rpc, rpc)])
        copies = []
        for j in range(rpc):
            copies.append(pltpu.async_copy(
                table_hbm.at[idx_v.at[j]],
                rowsa_v.at[pl.ds(j * _SC_IW, _SC_IW)], sem))
            copies.append(pltpu.async_copy(
                table_hbm.at[idx_v.at[rpc + j]],
                rowsb_v.at[pl.ds(j * _SC_IW, _SC_IW)], sem))
        for cp in copies:
            cp.wait()
        pltpu.sync_copy(rowsa_v, outa_hbm.at[pl.ds(t0, _SC_CH2)])
        pltpu.sync_copy(rowsb_v, outb_hbm.at[pl.ds(t0, _SC_CH2)])
        return carry

    lax.fori_loop(0, n_chunks, chunk_body, 0)


def _sc_gather2(table, idxa_rows, idxb_rows):
    b_total = idxa_rows.shape[0] * idxa_rows.shape[1]
    d = table.shape[1]
    mesh = plsc.VectorSubcoreMesh(core_axis_name="c", subcore_axis_name="s",
                                  num_cores=_SC_NC, num_subcores=_SC_NS)
    f = pl.kernel(
        _sc_gather2_body,
        out_type=[jax.ShapeDtypeStruct((b_total, d), _F32),
                  jax.ShapeDtypeStruct((b_total, d), _F32)],
        mesh=mesh,
        scratch_types=[
            pltpu.VMEM((2 * (_SC_CH2 // _SC_IW), _SC_IW), jnp.int32),
            pltpu.VMEM((_SC_CH2, d), _F32),
            pltpu.VMEM((_SC_CH2, d), _F32),
            pltpu.SemaphoreType.DMA,
        ],
        compiler_params=pltpu.CompilerParams(use_tc_tiling_on_sc=False),
    )
    return f(table, idxa_rows, idxb_rows)


def kernel(node_attr, edge_type, Dij, Anglesijk, batch_seg, idnb_i, idnb_j,
           id_expand_kj, id_reduce_ji, emb_table, W_rbf_emb, W_node, b_node,
           W_emb, b_emb, int_W_edge, int_W_rbf1, int_W_rbf2, int_W_sbf1,
           int_W_sbf2, int_W_ji, int_b_ji, int_W_kj, int_b_kj, int_W_down,
           int_W_up, int_res_bef_W, int_res_bef_b, int_W_skip, int_b_skip,
           int_res_aft_W, int_res_aft_b, out_W_up, out_dense_W, out_dense_b,
           out_W_final):
    n = node_attr.shape[0]
    e = Dij.shape[0]
    nmol = 512

    de2 = jnp.concatenate(
        [Dij.reshape(e, 1), edge_type.astype(_F32).reshape(e, 1)], axis=1)
    t_len = Anglesijk.shape[0]
    t_pad = ((t_len + _BT - 1) // _BT) * _BT - t_len
    cang_p = jnp.pad(jnp.cos(Anglesijk), (0, t_pad))

    w1 = W_emb[0:64]
    w2 = W_emb[64:128]
    wr = W_rbf_emb @ W_emb[128:192]
    we4 = emb_table @ W_emb[192:256]
    bemb = b_emb.reshape(1, -1)

    h = _node_embed(node_attr, W_node, b_node)
    hi, hj = _sc_gather2(h,
                         idnb_i.astype(jnp.int32).reshape(-1, _SC_IW),
                         idnb_j.astype(jnp.int32).reshape(-1, _SC_IW))
    x, rbf8 = _edge_embed(de2, hi, hj, w1, w2, wr, we4, bemb)

    dt_p = jnp.pad(
        Dij.at[id_reduce_ji].get(indices_are_sorted=True,
                                 mode="promise_in_bounds"),
        (0, t_pad), constant_values=_CUTOFF)
    ct2 = jnp.concatenate(
        [dt_p.reshape(-1, 1), cang_p.reshape(-1, 1)], axis=1)
    idx_rows = id_expand_kj.astype(jnp.int32).reshape(-1, _SC_IW)
    wsb0 = int_W_sbf1[0] @ int_W_sbf2[0]
    wsb1 = int_W_sbf1[1] @ int_W_sbf2[1]
    sp = _sbf_project(ct2, wsb0, wsb1)

    wf_pad = [jnp.pad(out_W_final[i], ((0, 0), (0, 128 - out_W_final.shape[2])))
              for i in range(_NB + 1)]

    t0 = jax.ops.segment_sum(x, idnb_i, num_segments=n)
    P = _out_block(t0, out_W_up[0], out_dense_W[0], out_dense_b[0], wf_pad[0])

    for i in range(_NB):
        we4_i = emb_table @ int_W_edge[i]
        wrbf_i = int_W_rbf1[i] @ int_W_rbf2[i]
        xji, xdown = _int_pre(
            x, de2, rbf8, we4_i,
            int_W_ji[i], int_b_ji[i].reshape(1, -1),
            int_W_kj[i], int_b_kj[i].reshape(1, -1),
            wrbf_i, int_W_down[i])
        msg = _sc_gather_mul(xdown, idx_rows, sp[i].reshape(-1))
        seg = jax.ops.segment_sum(msg, id_reduce_ji, num_segments=e,
                                  indices_are_sorted=True)
        x = _int_post(
            xji, seg, x, int_W_up[i],
            int_res_bef_W[i, 0, 0], int_res_bef_b[i, 0, 0].reshape(1, -1),
            int_res_bef_W[i, 0, 1], int_res_bef_b[i, 0, 1].reshape(1, -1),
            int_W_skip[i], int_b_skip[i].reshape(1, -1),
            int_res_aft_W[i, 0, 0], int_res_aft_b[i, 0, 0].reshape(1, -1),
            int_res_aft_W[i, 0, 1], int_res_aft_b[i, 0, 1].reshape(1, -1),
            int_res_aft_W[i, 1, 0], int_res_aft_b[i, 1, 0].reshape(1, -1),
            int_res_aft_W[i, 1, 1], int_res_aft_b[i, 1, 1].reshape(1, -1))
        ti = jax.ops.segment_sum(x, idnb_i, num_segments=n)
        P = P + _out_block(ti, out_W_up[i + 1], out_dense_W[i + 1],
                           out_dense_b[i + 1], wf_pad[i + 1])

    out = jax.ops.segment_sum(P, batch_seg, num_segments=nmol,
                              indices_are_sorted=True)
    return out[:, :12]


# speedup vs baseline: 2.1987x; 1.0013x over previous
import functools
import numpy as np
import jax
import jax.numpy as jnp
from jax import lax
from jax.experimental import pallas as pl
from jax.experimental.pallas import tpu as pltpu
from jax.experimental.pallas import tpu_sc as plsc

_CUTOFF = 5.0
_NRAD = 6
_NSPH = 7
_NB = 2
_NDO = 3

_BE = 1280
_BT = 9600
_BN = 2000

_F32 = jnp.float32


def _swish(x):
    return x * jax.nn.sigmoid(x)


def _envelope(d):
    a = -28.0
    b = 48.0
    c = -21.0
    d2 = d * d
    d4 = d2 * d2
    d5 = d4 * d
    env = 1.0 / d + a * d5 + b * d5 * d + c * d5 * d2
    return jnp.where(d < 1.0, env, 0.0)


def _rbf_from_d(d):
    k = jax.lax.broadcasted_iota(jnp.int32, (1, _NRAD), 1).astype(_F32)
    freq = (k + 1.0) * np.float32(np.pi)
    return _envelope(d) * jnp.sin(freq * d)


def _iota4(et):
    return jax.lax.broadcasted_iota(jnp.int32, (1, 4), 1)


def _dot(a, b):
    return jnp.dot(a, b, preferred_element_type=_F32)


def _node_body(na, w, b, h_out):
    h_out[...] = _dot(na[...], w[...]) + b[...]


def _node_embed(node_attr, W_node, b_node):
    n = node_attr.shape[0]
    grid = n // _BN
    return pl.pallas_call(
        _node_body,
        grid=(grid,),
        in_specs=[
            pl.BlockSpec((_BN, node_attr.shape[1]), lambda i: (i, 0)),
            pl.BlockSpec(W_node.shape, lambda i: (0, 0)),
            pl.BlockSpec((1, b_node.shape[0]), lambda i: (0, 0)),
        ],
        out_specs=pl.BlockSpec((_BN, 64), lambda i: (i, 0)),
        out_shape=jax.ShapeDtypeStruct((n, 64), _F32),
    )(node_attr, W_node, b_node.reshape(1, -1))


def _edge_body(de, hi, hj, w1, w2, wr, we4, bemb, x_out, rbf_out):
    d = de[:, 0:1] / _CUTOFF
    rbf = _rbf_from_d(d)
    oh = (de[:, 1:2] == _iota4(de).astype(_F32)).astype(_F32)
    acc = (_dot(hi[...], w1[...]) + _dot(hj[...], w2[...])
           + _dot(rbf, wr[...]) + _dot(oh, we4[...]) + bemb[...])
    x_out[...] = _swish(acc)
    rbf_out[...] = jnp.concatenate(
        [rbf, jnp.zeros_like(rbf[:, 0:2])], axis=1)


def _edge_embed(de2, hi, hj, w1, w2, wr, we4, bemb):
    e = de2.shape[0]
    grid = e // _BE
    wspec = lambda a: pl.BlockSpec(a.shape, lambda i: (0, 0))
    return pl.pallas_call(
        _edge_body,
        grid=(grid,),
        in_specs=[
            pl.BlockSpec((_BE, 2), lambda i: (i, 0)),
            pl.BlockSpec((_BE, 64), lambda i: (i, 0)),
            pl.BlockSpec((_BE, 64), lambda i: (i, 0)),
            wspec(w1), wspec(w2), wspec(wr), wspec(we4), wspec(bemb),
        ],
        out_specs=[
            pl.BlockSpec((_BE, 64), lambda i: (i, 0)),
            pl.BlockSpec((_BE, 8), lambda i: (i, 0)),
        ],
        out_shape=[
            jax.ShapeDtypeStruct((e, 64), _F32),
            jax.ShapeDtypeStruct((e, 8), _F32),
        ],
    )(de2, hi, hj, w1, w2, wr, we4, bemb)


def _int_pre_body(x, de, rbf8, we4, wji, bji, wkj, bkj, wrbf, wdown,
                  xji_out, xdown_out):
    oh = (de[:, 1:2] == _iota4(de).astype(_F32)).astype(_F32)
    m = x[...] + _dot(oh, we4[...])
    xji_out[...] = _swish(_dot(m, wji[...]) + bji[...])
    rbf_p = _dot(rbf8[:, 0:_NRAD], wrbf[...])
    xkj = _swish(_dot(m, wkj[...]) + bkj[...]) * rbf_p
    xdown_out[...] = _dot(xkj, wdown[...])


def _int_pre(x, de2, rbf8, we4, wji, bji, wkj, bkj, wrbf, wdown):
    e = x.shape[0]
    grid = e // _BE
    wspec = lambda a: pl.BlockSpec(a.shape, lambda i: (0, 0))
    return pl.pallas_call(
        _int_pre_body,
        grid=(grid,),
        in_specs=[
            pl.BlockSpec((_BE, 64), lambda i: (i, 0)),
            pl.BlockSpec((_BE, 2), lambda i: (i, 0)),
            pl.BlockSpec((_BE, 8), lambda i: (i, 0)),
            wspec(we4), wspec(wji), wspec(bji), wspec(wkj), wspec(bkj),
            wspec(wrbf), wspec(wdown),
        ],
        out_specs=[
            pl.BlockSpec((_BE, 64), lambda i: (i, 0)),
            pl.BlockSpec((_BE, 32), lambda i: (i, 0)),
        ],
        out_shape=[
            jax.ShapeDtypeStruct((e, 64), _F32),
            jax.ShapeDtypeStruct((e, 32), _F32),
        ],
    )(x, de2, rbf8, we4, wji, bji, wkj, bkj, wrbf, wdown)


def _int_post_body(xji, seg, xold, wup, wb0, bb0, wb1, bb1, wskip, bskip,
                   wa00, ba00, wa01, ba01, wa10, ba10, wa11, ba11, x_out):
    hh = xji[...] + _dot(seg[...], wup[...])
    h2 = _swish(_dot(hh, wb0[...]) + bb0[...])
    h2 = _swish(_dot(h2, wb1[...]) + bb1[...])
    hh = hh + h2
    hh = _swish(_dot(hh, wskip[...]) + bskip[...]) + xold[...]
    h2 = _swish(_dot(hh, wa00[...]) + ba00[...])
    h2 = _swish(_dot(h2, wa01[...]) + ba01[...])
    hh = hh + h2
    h2 = _swish(_dot(hh, wa10[...]) + ba10[...])
    h2 = _swish(_dot(h2, wa11[...]) + ba11[...])
    x_out[...] = hh + h2


def _int_post(xji, seg, xold, *ws):
    e = xji.shape[0]
    grid = e // _BE
    wspec = lambda a: pl.BlockSpec(a.shape, lambda i: (0, 0))
    return pl.pallas_call(
        _int_post_body,
        grid=(grid,),
        in_specs=[
            pl.BlockSpec((_BE, 64), lambda i: (i, 0)),
            pl.BlockSpec((_BE, 32), lambda i: (i, 0)),
            pl.BlockSpec((_BE, 64), lambda i: (i, 0)),
        ] + [wspec(w) for w in ws],
        out_specs=pl.BlockSpec((_BE, 64), lambda i: (i, 0)),
        out_shape=jax.ShapeDtypeStruct((e, 64), _F32),
    )(xji, seg, xold, *ws)


_SBF_G = 3


def _sbf_body(ct, wsb0, wsb1, sp0_out, sp1_out):
    ncols = _NSPH * _NRAD
    b0 = _BT // _SBF_G
    k = jax.lax.broadcasted_iota(jnp.int32, (1, _SBF_G * ncols), 1)
    k = k - (k // ncols) * ncols
    lcol = k // _NRAD
    ncol = k - lcol * _NRAD + 1
    zs = np.float32(np.pi) * (ncol.astype(_F32)
                              + 0.5 * lcol.astype(_F32))
    one_row = jnp.zeros((1, ncols), _F32) + 1.0

    def widen(col):
        parts = [col[g * b0:(g + 1) * b0, :] * one_row for g in range(_SBF_G)]
        return jnp.concatenate(parts, axis=1)

    d = widen(ct[:, 0:1] / _CUTOFF + 1e-9)
    env = _envelope(d)
    x = zs * d
    sx = jnp.sin(x)
    cx = jnp.cos(x)
    j0 = sx / x
    j1 = sx / (x * x) - cx / x
    res = jnp.where(lcol == 0, j0, 0.0)
    res = jnp.where(lcol == 1, j1, res)
    jm2, jm1 = j0, j1
    for ll in range(2, _NSPH):
        jl = (2.0 * ll - 1.0) / x * jm1 - jm2
        res = jnp.where(lcol == ll, jl, res)
        jm2, jm1 = jm1, jl
    c = widen(ct[:, 1:2])
    p = jnp.where(lcol == 0, 1.0, 0.0)
    p = jnp.where(lcol == 1, c, p)
    pm2 = jnp.zeros_like(x) + 1.0
    pm1 = c
    for ll in range(2, _NSPH):
        pc = ((2.0 * ll - 1.0) * c * pm1 - (ll - 1.0) * pm2) / ll
        p = jnp.where(lcol == ll, pc, p)
        pm2, pm1 = pm1, pc
    sbf = env * res * p
    for g in range(_SBF_G):
        blk = sbf[:, g * ncols:(g + 1) * ncols]
        sp0_out[g * b0:(g + 1) * b0, :] = _dot(blk, wsb0[...])
        sp1_out[g * b0:(g + 1) * b0, :] = _dot(blk, wsb1[...])


def _sbf_project(ct2, wsb0, wsb1):
    t = ct2.shape[0]
    grid = t // _BT
    wspec = lambda a: pl.BlockSpec(a.shape, lambda i: (0, 0))
    return pl.pallas_call(
        _sbf_body,
        grid=(grid,),
        in_specs=[
            pl.BlockSpec((_BT, 2), lambda i: (i, 0)),
            wspec(wsb0), wspec(wsb1),
        ],
        out_specs=[
            pl.BlockSpec((_BT, 32), lambda i: (i, 0)),
            pl.BlockSpec((_BT, 32), lambda i: (i, 0)),
        ],
        out_shape=[
            jax.ShapeDtypeStruct((t, 32), _F32),
            jax.ShapeDtypeStruct((t, 32), _F32),
        ],
    )(ct2, wsb0, wsb1)


def _out_body(t_in, wup, d0, b0, d1, b1, d2, b2, wf, p_out):
    t = _dot(t_in[...], wup[...])
    t = _swish(_dot(t, d0[...]) + b0[...])
    t = _swish(_dot(t, d1[...]) + b1[...])
    t = _swish(_dot(t, d2[...]) + b2[...])
    p_out[...] = _dot(t, wf[...])


def _out_block(t_nodes, wup, dw, db, wf_pad):
    n = t_nodes.shape[0]
    grid = n // _BN
    wspec = lambda a: pl.BlockSpec(a.shape, lambda i: (0, 0))
    args = [t_nodes, wup,
            dw[0], db[0].reshape(1, -1), dw[1], db[1].reshape(1, -1),
            dw[2], db[2].reshape(1, -1), wf_pad]
    return pl.pallas_call(
        _out_body,
        grid=(grid,),
        in_specs=[pl.BlockSpec((_BN, 64), lambda i: (i, 0))]
        + [wspec(a) for a in args[1:]],
        out_specs=pl.BlockSpec((_BN, 32), lambda i: (i, 0)),
        out_shape=jax.ShapeDtypeStruct((n, 32), _F32),
    )(*args)


_SC_NC = 2
_SC_NS = 16
_SC_NW = _SC_NC * _SC_NS
_SC_IW = 125
_SC_CH = 1250


def _sc_gmul_body(table_hbm, idx_hbm, sp_hbm, out_hbm, idx_v, rows_v, sp_v,
                  sem):
    t_total = out_hbm.shape[0]
    n_chunks = t_total // (_SC_NW * _SC_CH)
    wid = lax.axis_index("s") * _SC_NC + lax.axis_index("c")
    base_row = wid * (n_chunks * (_SC_CH // _SC_IW))

    def chunk_body(k, carry):
        row0 = base_row + k * (_SC_CH // _SC_IW)
        t0 = row0 * _SC_IW
        pltpu.sync_copy(idx_hbm.at[pl.ds(row0, _SC_CH // _SC_IW)], idx_v)
        copies = [pltpu.async_copy(
            sp_hbm.at[pl.ds(t0 * 32, _SC_CH * 32)], sp_v, sem)]
        for j in range(_SC_CH // _SC_IW):
            copies.append(pltpu.async_copy(
                table_hbm.at[idx_v.at[j]],
                rows_v.at[pl.ds(j * _SC_IW, _SC_IW)], sem))
        for cp in copies:
            cp.wait()

        def mul_body(r, c2):
            rr = r * 2
            for u in range(2):
                a0 = rows_v[rr + u, pl.ds(0, 16)]
                a1 = rows_v[rr + u, pl.ds(16, 16)]
                b0 = sp_v[pl.ds((rr + u) * 32, 16)]
                b1 = sp_v[pl.ds((rr + u) * 32 + 16, 16)]
                rows_v[rr + u, pl.ds(0, 16)] = a0 * b0
                rows_v[rr + u, pl.ds(16, 16)] = a1 * b1
            return c2

        lax.fori_loop(0, _SC_CH // 2, mul_body, 0)
        pltpu.sync_copy(rows_v, out_hbm.at[pl.ds(t0, _SC_CH)])
        return carry

    lax.fori_loop(0, n_chunks, chunk_body, 0)


def _sc_gather_mul(table, idx_rows, sp_flat):
    t_total = idx_rows.shape[0] * idx_rows.shape[1]
    mesh = plsc.VectorSubcoreMesh(core_axis_name="c", subcore_axis_name="s",
                                  num_cores=_SC_NC, num_subcores=_SC_NS)
    f = pl.kernel(
        _sc_gmul_body,
        out_type=jax.ShapeDtypeStruct((t_total, 32), _F32),
        mesh=mesh,
        scratch_types=[
            pltpu.VMEM((_SC_CH // _SC_IW, _SC_IW), jnp.int32),
            pltpu.VMEM((_SC_CH, 32), _F32),
            pltpu.VMEM((_SC_CH * 32,), _F32),
            pltpu.SemaphoreType.DMA,
        ],
        compiler_params=pltpu.CompilerParams(use_tc_tiling_on_sc=False),
    )
    return f(table, idx_rows, sp_flat)


_SC_CH2 = 500


def _sc_gather2_body(table_hbm, idxa_hbm, idxb_hbm, outa_hbm, outb_hbm,
                     idx_v, rowsa_v, rowsb_v, sem):
    b_total = outa_hbm.shape[0]
    n_chunks = b_total // (_SC_NW * _SC_CH2)
    wid = lax.axis_index("s") * _SC_NC + lax.axis_index("c")
    rpc = _SC_CH2 // _SC_IW
    base_row = wid * (n_chunks * rpc)

    def chunk_body(k, carry):
        row0 = base_row + k * rpc
        t0 = row0 * _SC_IW
        pltpu.sync_copy(idxa_hbm.at[pl.ds(row0, rpc)], idx_v.at[pl.ds(0, rpc)])
        pltpu.sync_copy(idxb_hbm.at[pl.ds(row0, rpc)],
                        idx_v.at[pl.ds(rpc, rpc)])
        copies = []
        for j in range(rpc):
            copies.append(pltpu.async_copy(
                table_hbm.at[idx_v.at[j]],
                rowsa_v.at[pl.ds(j * _SC_IW, _SC_IW)], sem))
            copies.append(pltpu.async_copy(
                table_hbm.at[idx_v.at[rpc + j]],
                rowsb_v.at[pl.ds(j * _SC_IW, _SC_IW)], sem))
        for cp in copies:
            cp.wait()
        pltpu.sync_copy(rowsa_v, outa_hbm.at[pl.ds(t0, _SC_CH2)])
        pltpu.sync_copy(rowsb_v, outb_hbm.at[pl.ds(t0, _SC_CH2)])
        return carry

    lax.fori_loop(0, n_chunks, chunk_body, 0)


def _sc_gather2(table, idxa_rows, idxb_rows):
    b_total = idxa_rows.shape[0] * idxa_rows.shape[1]
    d = table.shape[1]
    mesh = plsc.VectorSubcoreMesh(core_axis_name="c", subcore_axis_name="s",
                                  num_cores=_SC_NC, num_subcores=_SC_NS)
    f = pl.kernel(
        _sc_gather2_body,
        out_type=[jax.ShapeDtypeStruct((b_total, d), _F32),
                  jax.ShapeDtypeStruct((b_total, d), _F32)],
        mesh=mesh,
        scratch_types=[
            pltpu.VMEM((2 * (_SC_CH2 // _SC_IW), _SC_IW), jnp.int32),
            pltpu.VMEM((_SC_CH2, d), _F32),
            pltpu.VMEM((_SC_CH2, d), _F32),
            pltpu.SemaphoreType.DMA,
        ],
        compiler_params=pltpu.CompilerParams(use_tc_tiling_on_sc=False),
    )
    return f(table, idxa_rows, idxb_rows)


def kernel(node_attr, edge_type, Dij, Anglesijk, batch_seg, idnb_i, idnb_j,
           id_expand_kj, id_reduce_ji, emb_table, W_rbf_emb, W_node, b_node,
           W_emb, b_emb, int_W_edge, int_W_rbf1, int_W_rbf2, int_W_sbf1,
           int_W_sbf2, int_W_ji, int_b_ji, int_W_kj, int_b_kj, int_W_down,
           int_W_up, int_res_bef_W, int_res_bef_b, int_W_skip, int_b_skip,
           int_res_aft_W, int_res_aft_b, out_W_up, out_dense_W, out_dense_b,
           out_W_final):
    n = node_attr.shape[0]
    e = Dij.shape[0]
    nmol = 512

    de2 = jnp.concatenate(
        [Dij.reshape(e, 1), edge_type.astype(_F32).reshape(e, 1)], axis=1)
    t_len = Anglesijk.shape[0]
    t_pad = ((t_len + _BT - 1) // _BT) * _BT - t_len
    cang_p = jnp.pad(jnp.cos(Anglesijk), (0, t_pad))

    w1 = W_emb[0:64]
    w2 = W_emb[64:128]
    wr = W_rbf_emb @ W_emb[128:192]
    we4 = emb_table @ W_emb[192:256]
    bemb = b_emb.reshape(1, -1)

    h = _node_embed(node_attr, W_node, b_node)
    hi, hj = _sc_gather2(h,
                         idnb_i.astype(jnp.int32).reshape(-1, _SC_IW),
                         idnb_j.astype(jnp.int32).reshape(-1, _SC_IW))
    x, rbf8 = _edge_embed(de2, hi, hj, w1, w2, wr, we4, bemb)

    dt_p = jnp.pad(
        Dij.at[id_reduce_ji].get(indices_are_sorted=True,
                                 mode="promise_in_bounds"),
        (0, t_pad), constant_values=_CUTOFF)
    ct2 = jnp.concatenate(
        [dt_p.reshape(-1, 1), cang_p.reshape(-1, 1)], axis=1)
    idx_rows = id_expand_kj.astype(jnp.int32).reshape(-1, _SC_IW)
    wsb0 = int_W_sbf1[0] @ int_W_sbf2[0]
    wsb1 = int_W_sbf1[1] @ int_W_sbf2[1]
    sp = _sbf_project(ct2, wsb0, wsb1)

    wf_pad = [jnp.pad(out_W_final[i], ((0, 0), (0, 32 - out_W_final.shape[2])))
              for i in range(_NB + 1)]

    t0 = jax.ops.segment_sum(x, idnb_i, num_segments=n)
    P = _out_block(t0, out_W_up[0], out_dense_W[0], out_dense_b[0], wf_pad[0])

    for i in range(_NB):
        we4_i = emb_table @ int_W_edge[i]
        wrbf_i = int_W_rbf1[i] @ int_W_rbf2[i]
        xji, xdown = _int_pre(
            x, de2, rbf8, we4_i,
            int_W_ji[i], int_b_ji[i].reshape(1, -1),
            int_W_kj[i], int_b_kj[i].reshape(1, -1),
            wrbf_i, int_W_down[i])
        msg = _sc_gather_mul(xdown, idx_rows, sp[i].reshape(-1))
        seg = jax.ops.segment_sum(msg, id_reduce_ji, num_segments=e,
                                  indices_are_sorted=True)
        x = _int_post(
            xji, seg, x, int_W_up[i],
            int_res_bef_W[i, 0, 0], int_res_bef_b[i, 0, 0].reshape(1, -1),
            int_res_bef_W[i, 0, 1], int_res_bef_b[i, 0, 1].reshape(1, -1),
            int_W_skip[i], int_b_skip[i].reshape(1, -1),
            int_res_aft_W[i, 0, 0], int_res_aft_b[i, 0, 0].reshape(1, -1),
            int_res_aft_W[i, 0, 1], int_res_aft_b[i, 0, 1].reshape(1, -1),
            int_res_aft_W[i, 1, 0], int_res_aft_b[i, 1, 0].reshape(1, -1),
            int_res_aft_W[i, 1, 1], int_res_aft_b[i, 1, 1].reshape(1, -1))
        ti = jax.ops.segment_sum(x, idnb_i, num_segments=n)
        P = P + _out_block(ti, out_W_up[i + 1], out_dense_W[i + 1],
                           out_dense_b[i + 1], wf_pad[i + 1])

    out = jax.ops.segment_sum(P, batch_seg, num_segments=nmol,
                              indices_are_sorted=True)
    return out[:, :12]
